# baseline (speedup 1.0000x reference)
import functools

import jax
import jax.numpy as jnp
from jax import lax
from jax.experimental import pallas as pl
from jax.experimental.pallas import tpu as pltpu
from jax.experimental.pallas import tpu_sc as plsc

F = 128
NPT = 128
K = 128
NW = 32
EBLK = 1152
NBLK = 2000
BN_EPS = 1e-5
ETA_EPS = 1e-6


def _sigmoid16(x):
    return 1.0 / (1.0 + jnp.exp(-x))


def _sc_layer(bh1, bh2, vv, c2p, b3e, src_p, dst_p, estart_p, n_ranges):
    Ep = b3e.shape[0]
    Npad = n_ranges * NPT
    ESZ = estart_p.shape[0]
    mesh = plsc.VectorSubcoreMesh(core_axis_name="c", subcore_axis_name="s",
                                  num_cores=2, num_subcores=16)

    @functools.partial(
        pl.kernel,
        out_type=(
            jax.ShapeDtypeStruct((Ep, F), jnp.float32),
            jax.ShapeDtypeStruct((Npad, F), jnp.float32),
            jax.ShapeDtypeStruct((Npad, F), jnp.float32),
        ),
        mesh=mesh,
        scratch_types=[
            pltpu.VMEM((K, F), jnp.float32),
            pltpu.VMEM((K, F), jnp.float32),
            pltpu.VMEM((K, F), jnp.float32),
            pltpu.VMEM((NPT, F), jnp.float32),
            pltpu.VMEM((NPT, F), jnp.float32),
            pltpu.VMEM((NPT, F), jnp.float32),
            pltpu.VMEM((K,), jnp.int32),
            pltpu.VMEM((K,), jnp.int32),
            pltpu.VMEM((K + 16,), jnp.int32),
            pltpu.VMEM((ESZ,), jnp.int32),
            pltpu.SemaphoreType.DMA,
        ],
    )
    def k(bh1_h, bh2_h, vv_h, c2p_h, b3e_h, src_h, dst_h, est_h,
          hat_h, hagg_h, pagg_h,
          A, B, C, ss, hacc, pacc, sidx, didx, dsm, estv, sem):
        w = lax.axis_index("s") * 2 + lax.axis_index("c")
        pltpu.sync_copy(est_h, estv)

        n_mine = (n_ranges - 1 - w) // NW + 1

        def per_range(ri, _):
            r = w + ri * NW
            base = r * NPT
            e0 = estv[pl.ds(r, 16)][0]
            e1 = estv[pl.ds(r + 1, 16)][0]
            c0 = (e0 // 8) * 8
            nch = jnp.maximum(e1 - c0 + K - 1, 0) // K

            def zero_rows(n, _):
                zz = jnp.zeros((16,), jnp.float32)
                for j in range(F // 16):
                    sl = pl.ds(j * 16, 16)
                    ss[n, sl] = zz
                    hacc[n, sl] = zz
                    pacc[n, sl] = zz
                return 0

            lax.fori_loop(0, NPT, zero_rows, 0)

            def phase_a(ci, _):
                c = c0 + ci * K
                pltpu.sync_copy(src_h.at[pl.ds(c, K)], sidx)
                pltpu.sync_copy(dst_h.at[pl.ds(c, K)], didx)
                pltpu.sync_copy(dst_h.at[pl.ds(c, K)], dsm.at[pl.ds(0, K)])
                pltpu.sync_copy(b3e_h.at[pl.ds(c, K)], A)
                pltpu.async_copy(bh1_h.at[didx], B, sem).wait()
                pltpu.async_copy(bh2_h.at[sidx], C, sem).wait()

                def bulk(el, _):
                    for j in range(F // 16):
                        sl = pl.ds(j * 16, 16)
                        A[el, sl] = A[el, sl] + B[el, sl] + C[el, sl]
                    return 0

                lax.fori_loop(0, K, bulk, 0)
                pltpu.sync_copy(A, hat_h.at[pl.ds(c, K)])

                def acc_ss(e, _):
                    el = e - c
                    dl = dsm[pl.ds(el, 16)][0] - base
                    for j in range(F // 16):
                        sl = pl.ds(j * 16, 16)
                        ss[dl, sl] = ss[dl, sl] + _sigmoid16(A[el, sl])
                    return 0

                lax.fori_loop(jnp.maximum(e0, c), jnp.minimum(e1, c + K),
                              acc_ss, 0)
                return 0

            lax.fori_loop(0, nch, phase_a, 0)

            def invert(n, _):
                for j in range(F // 16):
                    sl = pl.ds(j * 16, 16)
                    ss[n, sl] = 1.0 / (ss[n, sl] + ETA_EPS)
                return 0

            lax.fori_loop(0, NPT, invert, 0)

            def phase_b(ci, _):
                c = c0 + ci * K
                pltpu.sync_copy(src_h.at[pl.ds(c, K)], sidx)
                pltpu.sync_copy(dst_h.at[pl.ds(c, K)], dsm.at[pl.ds(0, K)])
                pltpu.sync_copy(hat_h.at[pl.ds(c, K)], A)
                pltpu.async_copy(vv_h.at[sidx], B, sem).wait()
                pltpu.async_copy(c2p_h.at[sidx], C, sem).wait()

                def acc_hp(e, _):
                    el = e - c
                    dl = dsm[pl.ds(el, 16)][0] - base
                    for j in range(F // 16):
                        sl = pl.ds(j * 16, 16)
                        eta = _sigmoid16(A[el, sl]) * ss[dl, sl]
                        hacc[dl, sl] = hacc[dl, sl] + eta * B[el, sl]
                        pacc[dl, sl] = pacc[dl, sl] + eta * C[el, sl]
                    return 0

                lax.fori_loop(jnp.maximum(e0, c), jnp.minimum(e1, c + K),
                              acc_hp, 0)
                return 0

            lax.fori_loop(0, nch, phase_b, 0)

            pltpu.sync_copy(hacc, hagg_h.at[pl.ds(base, NPT)])
            pltpu.sync_copy(pacc, pagg_h.at[pl.ds(base, NPT)])
            return 0

        lax.fori_loop(0, n_mine, per_range, 0)

    return k(bh1, bh2, vv, c2p, b3e, src_p, dst_p, estart_p)


def _full(shape):
    return pl.BlockSpec(shape, lambda i: (0, 0))


def _enc_node(nf, pos, wn, bn_, gn, bbn, wp, bp):
    n = nf.shape[0]
    grid = n // NBLK

    def body(nf_r, pos_r, wn_r, bn_r, gn_r, bbn_r, wp_r, bp_r, h_r, pe_r):
        x = jnp.dot(nf_r[...], wn_r[...], preferred_element_type=jnp.float32) + bn_r[...]
        m = jnp.mean(x, axis=-1, keepdims=True)
        v = jnp.mean((x - m) ** 2, axis=-1, keepdims=True)
        h_r[...] = (x - m) / jnp.sqrt(v + BN_EPS) * gn_r[...] + bbn_r[...]
        pe_r[...] = jnp.dot(pos_r[...], wp_r[...], preferred_element_type=jnp.float32) + bp_r[...]

    return pl.pallas_call(
        body,
        grid=(grid,),
        in_specs=[
            pl.BlockSpec((NBLK, F), lambda i: (i, 0)),
            pl.BlockSpec((NBLK, 16), lambda i: (i, 0)),
            _full((F, F)), _full((1, F)), _full((1, F)), _full((1, F)),
            _full((16, F)), _full((1, F)),
        ],
        out_specs=[
            pl.BlockSpec((NBLK, F), lambda i: (i, 0)),
            pl.BlockSpec((NBLK, F), lambda i: (i, 0)),
        ],
        out_shape=[
            jax.ShapeDtypeStruct((n, F), jnp.float32),
            jax.ShapeDtypeStruct((n, F), jnp.float32),
        ],
    )(nf, pos, wn, bn_, gn, bbn, wp, bp)


def _enc_edge(ef, we, be, ge, bbe):
    Ep = ef.shape[0]
    grid = Ep // EBLK

    def body(ef_r, we_r, be_r, ge_r, bbe_r, e_r):
        x = jnp.dot(ef_r[...], we_r[...], preferred_element_type=jnp.float32) + be_r[...]
        m = jnp.mean(x, axis=-1, keepdims=True)
        v = jnp.mean((x - m) ** 2, axis=-1, keepdims=True)
        e_r[...] = (x - m) / jnp.sqrt(v + BN_EPS) * ge_r[...] + bbe_r[...]

    return pl.pallas_call(
        body,
        grid=(grid,),
        in_specs=[
            pl.BlockSpec((EBLK, 16), lambda i: (i, 0)),
            _full((16, F)), _full((1, F)), _full((1, F)), _full((1, F)),
        ],
        out_specs=pl.BlockSpec((EBLK, F), lambda i: (i, 0)),
        out_shape=jax.ShapeDtypeStruct((Ep, F), jnp.float32),
    )(ef, we, be, ge, bbe)


def _node_matmuls(h, pe, lp):
    n = h.shape[0]
    grid = n // NBLK
    ws = [lp['B1']['W'], lp['B2']['W'],
          lp['A1']['W'][:F], lp['A1']['W'][F:],
          lp['A2']['W'][:F], lp['A2']['W'][F:],
          lp['C1']['W'], lp['C2']['W']]
    bs = [lp['B1']['b'].reshape(1, F), lp['B2']['b'].reshape(1, F),
          lp['A1']['b'].reshape(1, F), lp['A2']['b'].reshape(1, F),
          lp['C1']['b'].reshape(1, F), lp['C2']['b'].reshape(1, F)]

    def body(h_r, pe_r, b1w, b2w, a1h, a1p, a2h, a2p, c1w, c2w,
             b1b, b2b, a1b, a2b, c1b, c2b,
             bh1_r, bh2_r, a1_r, vv_r, c1_r, c2p_r):
        hb = h_r[...]
        pb = pe_r[...]
        dot = lambda a, b: jnp.dot(a, b, preferred_element_type=jnp.float32)
        bh1_r[...] = dot(hb, b1w[...]) + b1b[...]
        bh2_r[...] = dot(hb, b2w[...]) + b2b[...]
        a1_r[...] = dot(hb, a1h[...]) + dot(pb, a1p[...]) + a1b[...]
        vv_r[...] = dot(hb, a2h[...]) + dot(pb, a2p[...]) + a2b[...]
        c1_r[...] = dot(pb, c1w[...]) + c1b[...]
        c2p_r[...] = dot(pb, c2w[...]) + c2b[...]

    blk = pl.BlockSpec((NBLK, F), lambda i: (i, 0))
    return pl.pallas_call(
        body,
        grid=(grid,),
        in_specs=[blk, blk] + [_full((F, F))] * 8 + [_full((1, F))] * 6,
        out_specs=[blk] * 6,
        out_shape=[jax.ShapeDtypeStruct((n, F), jnp.float32)] * 6,
    )(h, pe, *ws, *bs)


def _edge_b3(e, w3, b3):
    Ep = e.shape[0]
    grid = Ep // EBLK

    def body(e_r, w_r, b_r, o_r):
        o_r[...] = jnp.dot(e_r[...], w_r[...], preferred_element_type=jnp.float32) + b_r[...]

    blk = pl.BlockSpec((EBLK, F), lambda i: (i, 0))
    return pl.pallas_call(
        body,
        grid=(grid,),
        in_specs=[blk, _full((F, F)), _full((1, F))],
        out_specs=blk,
        out_shape=jax.ShapeDtypeStruct((Ep, F), jnp.float32),
    )(e, w3, b3)


def _edge_stats(hat, n_valid):
    Ep = hat.shape[0]
    grid = Ep // EBLK

    def body(hat_r, o_r):
        i = pl.program_id(0)
        rows = i * EBLK + lax.broadcasted_iota(jnp.int32, (EBLK, F), 0)
        x = jnp.where(rows < n_valid, hat_r[...], 0.0)
        s = jnp.sum(x, axis=0, keepdims=True)
        sq = jnp.sum(x * x, axis=0, keepdims=True)
        part = jnp.concatenate([s, sq], axis=0)

        @pl.when(i == 0)
        def _():
            o_r[...] = jnp.zeros_like(o_r)

        o_r[...] += part

    return pl.pallas_call(
        body,
        grid=(grid,),
        in_specs=[pl.BlockSpec((EBLK, F), lambda i: (i, 0))],
        out_specs=pl.BlockSpec((2, F), lambda i: (0, 0)),
        out_shape=jax.ShapeDtypeStruct((2, F), jnp.float32),
    )(hat)


def _edge_update(hat, e_prev, e_i, stats, g, b, w3, b3, n_valid):
    Ep = hat.shape[0]
    grid = Ep // EBLK

    def body(hat_r, ep_r, ei_r, st_r, g_r, b_r, w_r, b3_r, en_r, o_r):
        s = st_r[0, :]
        sq = st_r[1, :]
        m = s / n_valid
        var = sq / n_valid - m * m
        scale = g_r[0, :] / jnp.sqrt(var + BN_EPS)
        shift = b_r[0, :] - m * scale
        en = ep_r[...] + jnp.maximum(hat_r[...] * scale + shift, 0.0) + ei_r[...]
        en_r[...] = en
        o_r[...] = jnp.dot(en, w_r[...], preferred_element_type=jnp.float32) + b3_r[...]

    blk = pl.BlockSpec((EBLK, F), lambda i: (i, 0))
    return pl.pallas_call(
        body,
        grid=(grid,),
        in_specs=[blk, blk, blk, _full((2, F)), _full((1, F)), _full((1, F)),
                  _full((F, F)), _full((1, F))],
        out_specs=[blk, blk],
        out_shape=[jax.ShapeDtypeStruct((Ep, F), jnp.float32),
                   jax.ShapeDtypeStruct((Ep, F), jnp.float32)],
    )(hat, e_prev, e_i, stats, g, b, w3, b3)


def _node_stats(a1, hagg):
    n = a1.shape[0]
    grid = n // NBLK

    def body(a_r, h_r, o_r):
        i = pl.program_id(0)
        x = a_r[...] + h_r[...]
        s = jnp.sum(x, axis=0, keepdims=True)
        sq = jnp.sum(x * x, axis=0, keepdims=True)
        part = jnp.concatenate([s, sq], axis=0)

        @pl.when(i == 0)
        def _():
            o_r[...] = jnp.zeros_like(o_r)

        o_r[...] += part

    blk = pl.BlockSpec((NBLK, F), lambda i: (i, 0))
    return pl.pallas_call(
        body,
        grid=(grid,),
        in_specs=[blk, blk],
        out_specs=pl.BlockSpec((2, F), lambda i: (0, 0)),
        out_shape=jax.ShapeDtypeStruct((2, F), jnp.float32),
    )(a1, hagg)


def _node_update(a1, hagg, stats, g, b, h_prev, h_i,
                 c1, pagg, p_prev, p_i, n_valid):
    n = a1.shape[0]
    grid = n // NBLK

    def body(a_r, ha_r, st_r, g_r, b_r, hp_r, hi_r, c_r, pa_r, pp_r, pi_r,
             hn_r, pn_r):
        s = st_r[0, :]
        sq = st_r[1, :]
        m = s / n_valid
        var = sq / n_valid - m * m
        scale = g_r[0, :] / jnp.sqrt(var + BN_EPS)
        shift = b_r[0, :] - m * scale
        x = a_r[...] + ha_r[...]
        hn_r[...] = hp_r[...] + jnp.maximum(x * scale + shift, 0.0) + hi_r[...]
        pn_r[...] = pp_r[...] + jnp.tanh(c_r[...] + pa_r[...]) + pi_r[...]

    blk = pl.BlockSpec((NBLK, F), lambda i: (i, 0))
    return pl.pallas_call(
        body,
        grid=(grid,),
        in_specs=[blk, blk, _full((2, F)), _full((1, F)), _full((1, F)),
                  blk, blk, blk, blk, blk, blk],
        out_specs=[blk, blk],
        out_shape=[jax.ShapeDtypeStruct((n, F), jnp.float32),
                   jax.ShapeDtypeStruct((n, F), jnp.float32)],
    )(a1, hagg, stats, g, b, h_prev, h_i, c1, pagg, p_prev, p_i)


def _pool_mlp(h, gid_row, params, n_graphs):
    n = h.shape[0]

    def body(h_r, gid_r, w1, b1, g1, bb1, w2, b2, g2, bb2, w3, b3, o_r):
        gids = gid_r[...]
        onehot = (lax.broadcasted_iota(jnp.int32, (n_graphs, n), 0)
                  == gids).astype(jnp.float32)
        hg = jnp.dot(onehot, h_r[...], preferred_element_type=jnp.float32)

        def bn_elu(x, gg, bb):
            m = jnp.mean(x, axis=0, keepdims=True)
            var = jnp.mean((x - m) ** 2, axis=0, keepdims=True)
            y = (x - m) / jnp.sqrt(var + BN_EPS) * gg + bb
            return jnp.where(y > 0, y, jnp.exp(jnp.minimum(y, 0.0)) - 1.0)

        x = bn_elu(jnp.dot(hg, w1[...], preferred_element_type=jnp.float32) + b1[...],
                   g1[...], bb1[...])
        x = bn_elu(jnp.dot(x, w2[...], preferred_element_type=jnp.float32) + b2[...],
                   g2[...], bb2[...])
        o_r[...] = jnp.dot(x, w3[...], preferred_element_type=jnp.float32) + b3[...]

    mp = params
    return pl.pallas_call(
        body,
        in_specs=[
            pl.BlockSpec((n, F), lambda: (0, 0)),
            pl.BlockSpec((1, n), lambda: (0, 0)),
            _full2((F, F)), _full2((1, F)), _full2((1, F)), _full2((1, F)),
            _full2((F, 32)), _full2((1, 32)), _full2((1, 32)), _full2((1, 32)),
            _full2((32, 1)), _full2((1, 1)),
        ],
        out_specs=pl.BlockSpec((n_graphs, 1), lambda: (0, 0)),
        out_shape=jax.ShapeDtypeStruct((n_graphs, 1), jnp.float32),
    )(h, gid_row,
      mp['mlp_l1']['W'], mp['mlp_l1']['b'].reshape(1, F),
      mp['mlp_bn1']['g'].reshape(1, F), mp['mlp_bn1']['b'].reshape(1, F),
      mp['mlp_l2']['W'], mp['mlp_l2']['b'].reshape(1, 32),
      mp['mlp_bn2']['g'].reshape(1, 32), mp['mlp_bn2']['b'].reshape(1, 32),
      mp['mlp_l3']['W'], mp['mlp_l3']['b'].reshape(1, 1))


def _full2(shape):
    return pl.BlockSpec(shape, lambda: (0, 0))


def kernel(node_feats, edge_feats, pos_enc, fp, edge_index, graph_ids, params):
    n = node_feats.shape[0]
    e_cnt = edge_index.shape[1]
    n_graphs = fp.shape[0]

    src, dst = edge_index[0], edge_index[1]
    perm = jnp.argsort(dst)
    dst_s = dst[perm]
    src_s = src[perm]
    ef_s = edge_feats[perm]

    n_ranges = -(-n // NPT)
    Ep = e_cnt + K
    Ep = -(-Ep // EBLK) * EBLK
    esz = (-(-(n_ranges + 17) // 16)) * 16

    src_p = jnp.zeros((Ep,), jnp.int32).at[:e_cnt].set(src_s)
    dst_p = jnp.zeros((Ep,), jnp.int32).at[:e_cnt].set(dst_s)
    bounds = jnp.arange(n_ranges + 1, dtype=jnp.int32) * NPT
    estart = jnp.searchsorted(dst_s, bounds).astype(jnp.int32)
    estart_p = jnp.full((esz,), e_cnt, jnp.int32).at[:n_ranges + 1].set(estart)
    ef_p = jnp.zeros((Ep, 16), jnp.float32).at[:e_cnt].set(ef_s)

    p = params
    h, pe = _enc_node(
        node_feats, pos_enc,
        p['enc_node']['W'], p['enc_node']['b'].reshape(1, F),
        p['ln_node']['g'].reshape(1, F), p['ln_node']['b'].reshape(1, F),
        p['enc_pose']['W'], p['enc_pose']['b'].reshape(1, F))
    e = _enc_edge(
        ef_p,
        p['enc_edge']['W'], p['enc_edge']['b'].reshape(1, F),
        p['ln_edge']['g'].reshape(1, F), p['ln_edge']['b'].reshape(1, F))

    h_i, e_i, p_i = h, e, pe
    n_layers = len(p['layers'])
    b3e = _edge_b3(e, p['layers'][0]['B3']['W'],
                   p['layers'][0]['B3']['b'].reshape(1, F))
    e_prev = e

    for li, lp in enumerate(p['layers']):
        bh1, bh2, a1, vv, c1, c2p = _node_matmuls(h, pe, lp)
        hat, hagg, pagg = _sc_layer(bh1, bh2, vv, c2p, b3e,
                                    src_p, dst_p, estart_p, n_ranges)
        if li + 1 < n_layers:
            stats_e = _edge_stats(hat, e_cnt)
            nlp = p['layers'][li + 1]
            e_prev, b3e = _edge_update(
                hat, e_prev, e_i, stats_e,
                lp['bn_e']['g'].reshape(1, F), lp['bn_e']['b'].reshape(1, F),
                nlp['B3']['W'], nlp['B3']['b'].reshape(1, F), e_cnt)
        stats_h = _node_stats(a1, hagg)
        h, pe = _node_update(
            a1, hagg, stats_h,
            lp['bn_h']['g'].reshape(1, F), lp['bn_h']['b'].reshape(1, F),
            h, h_i, c1, pagg, pe, p_i, n)

    gid_row = graph_ids.reshape(1, n).astype(jnp.int32)
    return _pool_mlp(h, gid_row, p, n_graphs)


# speedup vs baseline: 1.2255x; 1.2255x over previous
import functools

import jax
import jax.numpy as jnp
from jax import lax
from jax.experimental import pallas as pl
from jax.experimental.pallas import tpu as pltpu
from jax.experimental.pallas import tpu_sc as plsc

F = 128
NPT = 128
K = 128
NW = 32
EBLK = 1152
NBLK = 2000
BN_EPS = 1e-5
ETA_EPS = 1e-6


def _sigmoid16(x):
    return 1.0 / (1.0 + jnp.exp(-x))


def _sc_layer(bh1, bh2, vv, c2p, b3e, src_p, dst_p, estart_p, nst_p,
                  n_ranges):
    Ep = b3e.shape[0]
    Npad = n_ranges * NPT
    ESZ = estart_p.shape[0]
    NSZ = nst_p.shape[0]
    mesh = plsc.VectorSubcoreMesh(core_axis_name="c", subcore_axis_name="s",
                                  num_cores=2, num_subcores=16)

    @functools.partial(
        pl.kernel,
        out_type=(
            jax.ShapeDtypeStruct((Ep, F), jnp.float32),
            jax.ShapeDtypeStruct((Npad, F), jnp.float32),
            jax.ShapeDtypeStruct((Npad, F), jnp.float32),
        ),
        mesh=mesh,
        scratch_types=[
            pltpu.VMEM((K, F), jnp.float32),
            pltpu.VMEM((K, F), jnp.float32),
            pltpu.VMEM((K, F), jnp.float32),
            pltpu.VMEM((NPT, F), jnp.float32),
            pltpu.VMEM((NPT, F), jnp.float32),
            pltpu.VMEM((NPT, F), jnp.float32),
            pltpu.VMEM((K,), jnp.int32),
            pltpu.VMEM((K,), jnp.int32),
            pltpu.VMEM((K + 16,), jnp.int32),
            pltpu.VMEM((ESZ,), jnp.int32),
            pltpu.VMEM((NPT + 32,), jnp.int32),
            pltpu.SemaphoreType.DMA,
            pltpu.SemaphoreType.DMA,
            pltpu.SemaphoreType.DMA,
            pltpu.SemaphoreType.DMA,
        ],
    )
    def k(bh1_h, bh2_h, vv_h, c2p_h, b3e_h, src_h, dst_h, est_h, nst_h,
          hat_h, hagg_h, pagg_h,
          A, B, C, ss, hacc, pacc, sidx, didx, dsm, estv, nstv,
          s0, s1, s2, s3):
        w = lax.axis_index("s") * 2 + lax.axis_index("c")
        pltpu.sync_copy(est_h, estv)

        n_mine = (n_ranges - 1 - w) // NW + 1

        def sval(ref, i):
            return ref[pl.ds(i, 16)][0]

        def per_range(ri, _):
            r = w + ri * NW
            base = r * NPT
            e0 = sval(estv, r)
            e1 = sval(estv, r + 1)
            c0 = (e0 // 8) * 8
            nch = jnp.maximum(e1 - c0 + K - 1, 0) // K
            pltpu.sync_copy(nst_h.at[pl.ds(base, NPT + 32)], nstv)

            def zero_rows(n, _):
                zz = jnp.zeros((16,), jnp.float32)
                for j in range(F // 16):
                    sl = pl.ds(j * 16, 16)
                    ss[n, sl] = zz
                    hacc[n, sl] = zz
                    pacc[n, sl] = zz
                return 0

            lax.fori_loop(0, NPT, zero_rows, 0)

            def chunk_nodes(c):
                nlo = jnp.clip(sval(dsm, 0) - base, 0, NPT - 1)
                nhi = jnp.clip(dsm[pl.ds(K - 16, 16)][15] - base, 0, NPT - 1)
                return nlo, nhi + 1

            def phase_a(ci, _):
                c = c0 + ci * K
                ca = pltpu.async_copy(src_h.at[pl.ds(c, K)], sidx, s0)
                cb = pltpu.async_copy(dst_h.at[pl.ds(c, K)], didx, s1)
                cc = pltpu.async_copy(dst_h.at[pl.ds(c, K)],
                                      dsm.at[pl.ds(0, K)], s2)
                cd = pltpu.async_copy(b3e_h.at[pl.ds(c, K)], A, s3)
                ca.wait(); cb.wait()
                cg1 = pltpu.async_copy(bh1_h.at[didx], B, s0)
                cg2 = pltpu.async_copy(bh2_h.at[sidx], C, s1)
                cc.wait(); cd.wait(); cg1.wait(); cg2.wait()

                def bulk(el, _):
                    for j in range(F // 16):
                        sl = pl.ds(j * 16, 16)
                        A[el, sl] = A[el, sl] + B[el, sl] + C[el, sl]
                    return 0

                lax.fori_loop(0, K, bulk, 0)
                pltpu.sync_copy(A, hat_h.at[pl.ds(c, K)])

                nlo, nhi = chunk_nodes(c)

                def per_node(nl, _):
                    es = jnp.maximum(sval(nstv, nl), c)
                    ee = jnp.minimum(sval(nstv, nl + 1), c + K)

                    def edge_body(e, accs):
                        el = e - c
                        return tuple(
                            accs[j] + _sigmoid16(A[el, pl.ds(j * 16, 16)])
                            for j in range(F // 16))

                    accs = lax.fori_loop(
                        es, ee, edge_body,
                        tuple(jnp.zeros((16,), jnp.float32)
                              for _ in range(F // 16)))
                    for j in range(F // 16):
                        sl = pl.ds(j * 16, 16)
                        ss[nl, sl] = ss[nl, sl] + accs[j]
                    return 0

                lax.fori_loop(nlo, nhi, per_node, 0)
                return 0

            lax.fori_loop(0, nch, phase_a, 0)

            def invert(n, _):
                for j in range(F // 16):
                    sl = pl.ds(j * 16, 16)
                    ss[n, sl] = 1.0 / (ss[n, sl] + ETA_EPS)
                return 0

            lax.fori_loop(0, NPT, invert, 0)

            def phase_b(ci, _):
                c = c0 + ci * K
                ca = pltpu.async_copy(src_h.at[pl.ds(c, K)], sidx, s0)
                cc = pltpu.async_copy(dst_h.at[pl.ds(c, K)],
                                      dsm.at[pl.ds(0, K)], s2)
                cd = pltpu.async_copy(hat_h.at[pl.ds(c, K)], A, s3)
                ca.wait()
                cg1 = pltpu.async_copy(vv_h.at[sidx], B, s0)
                cg2 = pltpu.async_copy(c2p_h.at[sidx], C, s1)
                cc.wait(); cd.wait(); cg1.wait(); cg2.wait()

                nlo, nhi = chunk_nodes(c)

                def per_node(nl, _):
                    es = jnp.maximum(sval(nstv, nl), c)
                    ee = jnp.minimum(sval(nstv, nl + 1), c + K)
                    inv = [ss[nl, pl.ds(j * 16, 16)] for j in range(F // 16)]

                    def edge_body(e, accs):
                        el = e - c
                        out = []
                        for j in range(F // 16):
                            sl = pl.ds(j * 16, 16)
                            eta = _sigmoid16(A[el, sl]) * inv[j]
                            out.append(accs[j] + eta * B[el, sl])
                            out.append(accs[j + 8] + eta * C[el, sl])
                        return tuple(out[0::2]) + tuple(out[1::2])

                    accs = lax.fori_loop(
                        es, ee, edge_body,
                        tuple(jnp.zeros((16,), jnp.float32)
                              for _ in range(2 * (F // 16))))
                    for j in range(F // 16):
                        sl = pl.ds(j * 16, 16)
                        hacc[nl, sl] = hacc[nl, sl] + accs[j]
                        pacc[nl, sl] = pacc[nl, sl] + accs[j + 8]
                    return 0

                lax.fori_loop(nlo, nhi, per_node, 0)
                return 0

            lax.fori_loop(0, nch, phase_b, 0)

            pltpu.sync_copy(hacc, hagg_h.at[pl.ds(base, NPT)])
            pltpu.sync_copy(pacc, pagg_h.at[pl.ds(base, NPT)])
            return 0

        lax.fori_loop(0, n_mine, per_range, 0)

    return k(bh1, bh2, vv, c2p, b3e, src_p, dst_p, estart_p, nst_p)


def _full(shape):
    return pl.BlockSpec(shape, lambda i: (0, 0))


def _enc_node(nf, pos, wn, bn_, gn, bbn, wp, bp):
    n = nf.shape[0]
    grid = n // NBLK

    def body(nf_r, pos_r, wn_r, bn_r, gn_r, bbn_r, wp_r, bp_r, h_r, pe_r):
        x = jnp.dot(nf_r[...], wn_r[...], preferred_element_type=jnp.float32) + bn_r[...]
        m = jnp.mean(x, axis=-1, keepdims=True)
        v = jnp.mean((x - m) ** 2, axis=-1, keepdims=True)
        h_r[...] = (x - m) / jnp.sqrt(v + BN_EPS) * gn_r[...] + bbn_r[...]
        pe_r[...] = jnp.dot(pos_r[...], wp_r[...], preferred_element_type=jnp.float32) + bp_r[...]

    return pl.pallas_call(
        body,
        grid=(grid,),
        in_specs=[
            pl.BlockSpec((NBLK, F), lambda i: (i, 0)),
            pl.BlockSpec((NBLK, 16), lambda i: (i, 0)),
            _full((F, F)), _full((1, F)), _full((1, F)), _full((1, F)),
            _full((16, F)), _full((1, F)),
        ],
        out_specs=[
            pl.BlockSpec((NBLK, F), lambda i: (i, 0)),
            pl.BlockSpec((NBLK, F), lambda i: (i, 0)),
        ],
        out_shape=[
            jax.ShapeDtypeStruct((n, F), jnp.float32),
            jax.ShapeDtypeStruct((n, F), jnp.float32),
        ],
    )(nf, pos, wn, bn_, gn, bbn, wp, bp)


def _enc_edge(ef, we, be, ge, bbe):
    Ep = ef.shape[0]
    grid = Ep // EBLK

    def body(ef_r, we_r, be_r, ge_r, bbe_r, e_r):
        x = jnp.dot(ef_r[...], we_r[...], preferred_element_type=jnp.float32) + be_r[...]
        m = jnp.mean(x, axis=-1, keepdims=True)
        v = jnp.mean((x - m) ** 2, axis=-1, keepdims=True)
        e_r[...] = (x - m) / jnp.sqrt(v + BN_EPS) * ge_r[...] + bbe_r[...]

    return pl.pallas_call(
        body,
        grid=(grid,),
        in_specs=[
            pl.BlockSpec((EBLK, 16), lambda i: (i, 0)),
            _full((16, F)), _full((1, F)), _full((1, F)), _full((1, F)),
        ],
        out_specs=pl.BlockSpec((EBLK, F), lambda i: (i, 0)),
        out_shape=jax.ShapeDtypeStruct((Ep, F), jnp.float32),
    )(ef, we, be, ge, bbe)


def _node_matmuls(h, pe, lp):
    n = h.shape[0]
    grid = n // NBLK
    ws = [lp['B1']['W'], lp['B2']['W'],
          lp['A1']['W'][:F], lp['A1']['W'][F:],
          lp['A2']['W'][:F], lp['A2']['W'][F:],
          lp['C1']['W'], lp['C2']['W']]
    bs = [lp['B1']['b'].reshape(1, F), lp['B2']['b'].reshape(1, F),
          lp['A1']['b'].reshape(1, F), lp['A2']['b'].reshape(1, F),
          lp['C1']['b'].reshape(1, F), lp['C2']['b'].reshape(1, F)]

    def body(h_r, pe_r, b1w, b2w, a1h, a1p, a2h, a2p, c1w, c2w,
             b1b, b2b, a1b, a2b, c1b, c2b,
             bh1_r, bh2_r, a1_r, vv_r, c1_r, c2p_r):
        hb = h_r[...]
        pb = pe_r[...]
        dot = lambda a, b: jnp.dot(a, b, preferred_element_type=jnp.float32)
        bh1_r[...] = dot(hb, b1w[...]) + b1b[...]
        bh2_r[...] = dot(hb, b2w[...]) + b2b[...]
        a1_r[...] = dot(hb, a1h[...]) + dot(pb, a1p[...]) + a1b[...]
        vv_r[...] = dot(hb, a2h[...]) + dot(pb, a2p[...]) + a2b[...]
        c1_r[...] = dot(pb, c1w[...]) + c1b[...]
        c2p_r[...] = dot(pb, c2w[...]) + c2b[...]

    blk = pl.BlockSpec((NBLK, F), lambda i: (i, 0))
    return pl.pallas_call(
        body,
        grid=(grid,),
        in_specs=[blk, blk] + [_full((F, F))] * 8 + [_full((1, F))] * 6,
        out_specs=[blk] * 6,
        out_shape=[jax.ShapeDtypeStruct((n, F), jnp.float32)] * 6,
    )(h, pe, *ws, *bs)


def _edge_b3(e, w3, b3):
    Ep = e.shape[0]
    grid = Ep // EBLK

    def body(e_r, w_r, b_r, o_r):
        o_r[...] = jnp.dot(e_r[...], w_r[...], preferred_element_type=jnp.float32) + b_r[...]

    blk = pl.BlockSpec((EBLK, F), lambda i: (i, 0))
    return pl.pallas_call(
        body,
        grid=(grid,),
        in_specs=[blk, _full((F, F)), _full((1, F))],
        out_specs=blk,
        out_shape=jax.ShapeDtypeStruct((Ep, F), jnp.float32),
    )(e, w3, b3)


def _edge_stats(hat, n_valid):
    Ep = hat.shape[0]
    grid = Ep // EBLK

    def body(hat_r, o_r):
        i = pl.program_id(0)
        rows = i * EBLK + lax.broadcasted_iota(jnp.int32, (EBLK, F), 0)
        x = jnp.where(rows < n_valid, hat_r[...], 0.0)
        s = jnp.sum(x, axis=0, keepdims=True)
        sq = jnp.sum(x * x, axis=0, keepdims=True)
        part = jnp.concatenate([s, sq], axis=0)

        @pl.when(i == 0)
        def _():
            o_r[...] = jnp.zeros_like(o_r)

        o_r[...] += part

    return pl.pallas_call(
        body,
        grid=(grid,),
        in_specs=[pl.BlockSpec((EBLK, F), lambda i: (i, 0))],
        out_specs=pl.BlockSpec((2, F), lambda i: (0, 0)),
        out_shape=jax.ShapeDtypeStruct((2, F), jnp.float32),
    )(hat)


def _edge_update(hat, e_prev, e_i, stats, g, b, w3, b3, n_valid):
    Ep = hat.shape[0]
    grid = Ep // EBLK

    def body(hat_r, ep_r, ei_r, st_r, g_r, b_r, w_r, b3_r, en_r, o_r):
        s = st_r[0, :]
        sq = st_r[1, :]
        m = s / n_valid
        var = sq / n_valid - m * m
        scale = g_r[0, :] / jnp.sqrt(var + BN_EPS)
        shift = b_r[0, :] - m * scale
        en = ep_r[...] + jnp.maximum(hat_r[...] * scale + shift, 0.0) + ei_r[...]
        en_r[...] = en
        o_r[...] = jnp.dot(en, w_r[...], preferred_element_type=jnp.float32) + b3_r[...]

    blk = pl.BlockSpec((EBLK, F), lambda i: (i, 0))
    return pl.pallas_call(
        body,
        grid=(grid,),
        in_specs=[blk, blk, blk, _full((2, F)), _full((1, F)), _full((1, F)),
                  _full((F, F)), _full((1, F))],
        out_specs=[blk, blk],
        out_shape=[jax.ShapeDtypeStruct((Ep, F), jnp.float32),
                   jax.ShapeDtypeStruct((Ep, F), jnp.float32)],
    )(hat, e_prev, e_i, stats, g, b, w3, b3)


def _node_stats(a1, hagg):
    n = a1.shape[0]
    grid = n // NBLK

    def body(a_r, h_r, o_r):
        i = pl.program_id(0)
        x = a_r[...] + h_r[...]
        s = jnp.sum(x, axis=0, keepdims=True)
        sq = jnp.sum(x * x, axis=0, keepdims=True)
        part = jnp.concatenate([s, sq], axis=0)

        @pl.when(i == 0)
        def _():
            o_r[...] = jnp.zeros_like(o_r)

        o_r[...] += part

    blk = pl.BlockSpec((NBLK, F), lambda i: (i, 0))
    return pl.pallas_call(
        body,
        grid=(grid,),
        in_specs=[blk, blk],
        out_specs=pl.BlockSpec((2, F), lambda i: (0, 0)),
        out_shape=jax.ShapeDtypeStruct((2, F), jnp.float32),
    )(a1, hagg)


def _node_update(a1, hagg, stats, g, b, h_prev, h_i,
                 c1, pagg, p_prev, p_i, n_valid):
    n = a1.shape[0]
    grid = n // NBLK

    def body(a_r, ha_r, st_r, g_r, b_r, hp_r, hi_r, c_r, pa_r, pp_r, pi_r,
             hn_r, pn_r):
        s = st_r[0, :]
        sq = st_r[1, :]
        m = s / n_valid
        var = sq / n_valid - m * m
        scale = g_r[0, :] / jnp.sqrt(var + BN_EPS)
        shift = b_r[0, :] - m * scale
        x = a_r[...] + ha_r[...]
        hn_r[...] = hp_r[...] + jnp.maximum(x * scale + shift, 0.0) + hi_r[...]
        pn_r[...] = pp_r[...] + jnp.tanh(c_r[...] + pa_r[...]) + pi_r[...]

    blk = pl.BlockSpec((NBLK, F), lambda i: (i, 0))
    return pl.pallas_call(
        body,
        grid=(grid,),
        in_specs=[blk, blk, _full((2, F)), _full((1, F)), _full((1, F)),
                  blk, blk, blk, blk, blk, blk],
        out_specs=[blk, blk],
        out_shape=[jax.ShapeDtypeStruct((n, F), jnp.float32),
                   jax.ShapeDtypeStruct((n, F), jnp.float32)],
    )(a1, hagg, stats, g, b, h_prev, h_i, c1, pagg, p_prev, p_i)


def _pool_mlp(h, gid_row, params, n_graphs):
    n = h.shape[0]

    def body(h_r, gid_r, w1, b1, g1, bb1, w2, b2, g2, bb2, w3, b3, o_r):
        gids = gid_r[...]
        onehot = (lax.broadcasted_iota(jnp.int32, (n_graphs, n), 0)
                  == gids).astype(jnp.float32)
        hg = jnp.dot(onehot, h_r[...], preferred_element_type=jnp.float32)

        def bn_elu(x, gg, bb):
            m = jnp.mean(x, axis=0, keepdims=True)
            var = jnp.mean((x - m) ** 2, axis=0, keepdims=True)
            y = (x - m) / jnp.sqrt(var + BN_EPS) * gg + bb
            return jnp.where(y > 0, y, jnp.exp(jnp.minimum(y, 0.0)) - 1.0)

        x = bn_elu(jnp.dot(hg, w1[...], preferred_element_type=jnp.float32) + b1[...],
                   g1[...], bb1[...])
        x = bn_elu(jnp.dot(x, w2[...], preferred_element_type=jnp.float32) + b2[...],
                   g2[...], bb2[...])
        o_r[...] = jnp.dot(x, w3[...], preferred_element_type=jnp.float32) + b3[...]

    mp = params
    return pl.pallas_call(
        body,
        in_specs=[
            pl.BlockSpec((n, F), lambda: (0, 0)),
            pl.BlockSpec((1, n), lambda: (0, 0)),
            _full2((F, F)), _full2((1, F)), _full2((1, F)), _full2((1, F)),
            _full2((F, 32)), _full2((1, 32)), _full2((1, 32)), _full2((1, 32)),
            _full2((32, 1)), _full2((1, 1)),
        ],
        out_specs=pl.BlockSpec((n_graphs, 1), lambda: (0, 0)),
        out_shape=jax.ShapeDtypeStruct((n_graphs, 1), jnp.float32),
    )(h, gid_row,
      mp['mlp_l1']['W'], mp['mlp_l1']['b'].reshape(1, F),
      mp['mlp_bn1']['g'].reshape(1, F), mp['mlp_bn1']['b'].reshape(1, F),
      mp['mlp_l2']['W'], mp['mlp_l2']['b'].reshape(1, 32),
      mp['mlp_bn2']['g'].reshape(1, 32), mp['mlp_bn2']['b'].reshape(1, 32),
      mp['mlp_l3']['W'], mp['mlp_l3']['b'].reshape(1, 1))


def _full2(shape):
    return pl.BlockSpec(shape, lambda: (0, 0))


def kernel(node_feats, edge_feats, pos_enc, fp, edge_index, graph_ids, params):
    n = node_feats.shape[0]
    e_cnt = edge_index.shape[1]
    n_graphs = fp.shape[0]

    src, dst = edge_index[0], edge_index[1]
    perm = jnp.argsort(dst)
    dst_s = dst[perm]
    src_s = src[perm]
    ef_s = edge_feats[perm]

    n_ranges = -(-n // NPT)
    Ep = e_cnt + K
    Ep = -(-Ep // EBLK) * EBLK
    esz = (-(-(n_ranges + 17) // 16)) * 16

    npad = n_ranges * NPT
    src_p = jnp.zeros((Ep,), jnp.int32).at[:e_cnt].set(src_s)
    dst_p = jnp.full((Ep,), n - 1, jnp.int32).at[:e_cnt].set(dst_s)
    bounds = jnp.arange(n_ranges + 1, dtype=jnp.int32) * NPT
    estart = jnp.searchsorted(dst_s, bounds).astype(jnp.int32)
    estart_p = jnp.full((esz,), e_cnt, jnp.int32).at[:n_ranges + 1].set(estart)
    nst = jnp.searchsorted(dst_s, jnp.arange(npad + 1, dtype=jnp.int32)
                           ).astype(jnp.int32)
    nst_p = jnp.full((npad + 32,), e_cnt, jnp.int32).at[:npad + 1].set(nst)
    ef_p = jnp.zeros((Ep, 16), jnp.float32).at[:e_cnt].set(ef_s)

    p = params
    h, pe = _enc_node(
        node_feats, pos_enc,
        p['enc_node']['W'], p['enc_node']['b'].reshape(1, F),
        p['ln_node']['g'].reshape(1, F), p['ln_node']['b'].reshape(1, F),
        p['enc_pose']['W'], p['enc_pose']['b'].reshape(1, F))
    e = _enc_edge(
        ef_p,
        p['enc_edge']['W'], p['enc_edge']['b'].reshape(1, F),
        p['ln_edge']['g'].reshape(1, F), p['ln_edge']['b'].reshape(1, F))

    h_i, e_i, p_i = h, e, pe
    n_layers = len(p['layers'])
    b3e = _edge_b3(e, p['layers'][0]['B3']['W'],
                   p['layers'][0]['B3']['b'].reshape(1, F))
    e_prev = e

    for li, lp in enumerate(p['layers']):
        bh1, bh2, a1, vv, c1, c2p = _node_matmuls(h, pe, lp)
        hat, hagg, pagg = _sc_layer(bh1, bh2, vv, c2p, b3e,
                                    src_p, dst_p, estart_p, nst_p, n_ranges)
        if li + 1 < n_layers:
            stats_e = _edge_stats(hat, e_cnt)
            nlp = p['layers'][li + 1]
            e_prev, b3e = _edge_update(
                hat, e_prev, e_i, stats_e,
                lp['bn_e']['g'].reshape(1, F), lp['bn_e']['b'].reshape(1, F),
                nlp['B3']['W'], nlp['B3']['b'].reshape(1, F), e_cnt)
        stats_h = _node_stats(a1, hagg)
        h, pe = _node_update(
            a1, hagg, stats_h,
            lp['bn_h']['g'].reshape(1, F), lp['bn_h']['b'].reshape(1, F),
            h, h_i, c1, pagg, pe, p_i, n)

    gid_row = graph_ids.reshape(1, n).astype(jnp.int32)
    return _pool_mlp(h, gid_row, p, n_graphs)


# speedup vs baseline: 3.1781x; 2.5933x over previous
import functools

import jax
import jax.numpy as jnp
from jax import lax
from jax.experimental import pallas as pl
from jax.experimental.pallas import tpu as pltpu
from jax.experimental.pallas import tpu_sc as plsc

F = 128
NPT = 128
K = 128
NW = 32
EBLK = 1152
NBLK = 2000
BN_EPS = 1e-5
ETA_EPS = 1e-6


def _sigmoid16(x):
    return 1.0 / (1.0 + jnp.exp(-x))


def _sc_layer(bh1, bh2, vv, c2p, b3e, src_p, dst_p, estart_p, nst_p,
                  n_ranges):
    Ep = b3e.shape[0]
    Npad = n_ranges * NPT
    ESZ = estart_p.shape[0]
    NSZ = nst_p.shape[0]
    mesh = plsc.VectorSubcoreMesh(core_axis_name="c", subcore_axis_name="s",
                                  num_cores=2, num_subcores=16)

    @functools.partial(
        pl.kernel,
        out_type=(
            jax.ShapeDtypeStruct((Ep, F), jnp.float32),
            jax.ShapeDtypeStruct((Npad, F), jnp.float32),
            jax.ShapeDtypeStruct((Npad, F), jnp.float32),
        ),
        mesh=mesh,
        scratch_types=[
            pltpu.VMEM((K, F), jnp.float32),
            pltpu.VMEM((K, F), jnp.float32),
            pltpu.VMEM((K, F), jnp.float32),
            pltpu.VMEM((NPT, F), jnp.float32),
            pltpu.VMEM((NPT, F), jnp.float32),
            pltpu.VMEM((NPT, F), jnp.float32),
            pltpu.VMEM((K,), jnp.int32),
            pltpu.VMEM((K,), jnp.int32),
            pltpu.VMEM((K + 16,), jnp.int32),
            pltpu.VMEM((ESZ,), jnp.int32),
            pltpu.VMEM((NPT + 32,), jnp.int32),
            pltpu.SemaphoreType.DMA,
            pltpu.SemaphoreType.DMA,
            pltpu.SemaphoreType.DMA,
            pltpu.SemaphoreType.DMA,
        ],
    )
    def k(bh1_h, bh2_h, vv_h, c2p_h, b3e_h, src_h, dst_h, est_h, nst_h,
          hat_h, hagg_h, pagg_h,
          A, B, C, ss, hacc, pacc, sidx, didx, dsm, estv, nstv,
          s0, s1, s2, s3):
        w = lax.axis_index("s") * 2 + lax.axis_index("c")
        pltpu.sync_copy(est_h, estv)

        n_mine = (n_ranges - 1 - w) // NW + 1

        def sval(ref, i):
            return ref[pl.ds(i, 16)][0]

        def per_range(ri, _):
            r = w + ri * NW
            base = r * NPT
            e0 = sval(estv, r)
            e1 = sval(estv, r + 1)
            c0 = (e0 // 8) * 8
            nch = jnp.maximum(e1 - c0 + K - 1, 0) // K
            pltpu.sync_copy(nst_h.at[pl.ds(base, NPT + 32)], nstv)

            def zero_rows(n, _):
                zz = jnp.zeros((16,), jnp.float32)
                for j in range(F // 16):
                    sl = pl.ds(j * 16, 16)
                    ss[n, sl] = zz
                    hacc[n, sl] = zz
                    pacc[n, sl] = zz
                return 0

            lax.fori_loop(0, NPT, zero_rows, 0)

            def chunk_nodes(c):
                nlo = jnp.clip(sval(dsm, 0) - base, 0, NPT - 1)
                nhi = jnp.clip(dsm[pl.ds(K - 16, 16)][15] - base, 0, NPT - 1)
                return nlo, nhi + 1

            def phase_a(ci, _):
                c = c0 + ci * K
                ca = pltpu.async_copy(src_h.at[pl.ds(c, K)], sidx, s0)
                cb = pltpu.async_copy(dst_h.at[pl.ds(c, K)], didx, s1)
                cc = pltpu.async_copy(dst_h.at[pl.ds(c, K)],
                                      dsm.at[pl.ds(0, K)], s2)
                cd = pltpu.async_copy(b3e_h.at[pl.ds(c, K)], A, s3)
                ca.wait(); cb.wait()
                cg1 = pltpu.async_copy(bh1_h.at[didx], B, s0)
                cg2 = pltpu.async_copy(bh2_h.at[sidx], C, s1)
                cc.wait(); cd.wait(); cg1.wait(); cg2.wait()

                def bulk(el, _):
                    for j in range(F // 16):
                        sl = pl.ds(j * 16, 16)
                        A[el, sl] = A[el, sl] + B[el, sl] + C[el, sl]
                    return 0

                lax.fori_loop(0, K, bulk, 0)
                pltpu.sync_copy(A, hat_h.at[pl.ds(c, K)])

                nlo, nhi = chunk_nodes(c)

                def per_node(nl, _):
                    es = jnp.maximum(sval(nstv, nl), c)
                    ee = jnp.minimum(sval(nstv, nl + 1), c + K)

                    def edge_body(e, accs):
                        el = e - c
                        return tuple(
                            accs[j] + _sigmoid16(A[el, pl.ds(j * 16, 16)])
                            for j in range(F // 16))

                    accs = lax.fori_loop(
                        es, ee, edge_body,
                        tuple(jnp.zeros((16,), jnp.float32)
                              for _ in range(F // 16)))
                    for j in range(F // 16):
                        sl = pl.ds(j * 16, 16)
                        ss[nl, sl] = ss[nl, sl] + accs[j]
                    return 0

                lax.fori_loop(nlo, nhi, per_node, 0)
                return 0

            lax.fori_loop(0, nch, phase_a, 0)

            def invert(n, _):
                for j in range(F // 16):
                    sl = pl.ds(j * 16, 16)
                    ss[n, sl] = 1.0 / (ss[n, sl] + ETA_EPS)
                return 0

            lax.fori_loop(0, NPT, invert, 0)

            def phase_b(ci, _):
                c = c0 + ci * K
                ca = pltpu.async_copy(src_h.at[pl.ds(c, K)], sidx, s0)
                cc = pltpu.async_copy(dst_h.at[pl.ds(c, K)],
                                      dsm.at[pl.ds(0, K)], s2)
                cd = pltpu.async_copy(hat_h.at[pl.ds(c, K)], A, s3)
                ca.wait()
                cg1 = pltpu.async_copy(vv_h.at[sidx], B, s0)
                cg2 = pltpu.async_copy(c2p_h.at[sidx], C, s1)
                cc.wait(); cd.wait(); cg1.wait(); cg2.wait()

                nlo, nhi = chunk_nodes(c)

                def per_node(nl, _):
                    es = jnp.maximum(sval(nstv, nl), c)
                    ee = jnp.minimum(sval(nstv, nl + 1), c + K)
                    inv = [ss[nl, pl.ds(j * 16, 16)] for j in range(F // 16)]

                    def edge_body(e, accs):
                        el = e - c
                        out = []
                        for j in range(F // 16):
                            sl = pl.ds(j * 16, 16)
                            eta = _sigmoid16(A[el, sl]) * inv[j]
                            out.append(accs[j] + eta * B[el, sl])
                            out.append(accs[j + 8] + eta * C[el, sl])
                        return tuple(out[0::2]) + tuple(out[1::2])

                    accs = lax.fori_loop(
                        es, ee, edge_body,
                        tuple(jnp.zeros((16,), jnp.float32)
                              for _ in range(2 * (F // 16))))
                    for j in range(F // 16):
                        sl = pl.ds(j * 16, 16)
                        hacc[nl, sl] = hacc[nl, sl] + accs[j]
                        pacc[nl, sl] = pacc[nl, sl] + accs[j + 8]
                    return 0

                lax.fori_loop(nlo, nhi, per_node, 0)
                return 0

            lax.fori_loop(0, nch, phase_b, 0)

            pltpu.sync_copy(hacc, hagg_h.at[pl.ds(base, NPT)])
            pltpu.sync_copy(pacc, pagg_h.at[pl.ds(base, NPT)])
            return 0

        lax.fori_loop(0, n_mine, per_range, 0)

    return k(bh1, bh2, vv, c2p, b3e, src_p, dst_p, estart_p, nst_p)


def _full(shape):
    return pl.BlockSpec(shape, lambda i: (0, 0))


def _enc_node(nf, pos, wn, bn_, gn, bbn, wp, bp):
    n = nf.shape[0]
    grid = n // NBLK

    def body(nf_r, pos_r, wn_r, bn_r, gn_r, bbn_r, wp_r, bp_r, h_r, pe_r):
        x = jnp.dot(nf_r[...], wn_r[...], preferred_element_type=jnp.float32) + bn_r[...]
        m = jnp.mean(x, axis=-1, keepdims=True)
        v = jnp.mean((x - m) ** 2, axis=-1, keepdims=True)
        h_r[...] = (x - m) / jnp.sqrt(v + BN_EPS) * gn_r[...] + bbn_r[...]
        pe_r[...] = jnp.dot(pos_r[...], wp_r[...], preferred_element_type=jnp.float32) + bp_r[...]

    return pl.pallas_call(
        body,
        grid=(grid,),
        in_specs=[
            pl.BlockSpec((NBLK, F), lambda i: (i, 0)),
            pl.BlockSpec((NBLK, 16), lambda i: (i, 0)),
            _full((F, F)), _full((1, F)), _full((1, F)), _full((1, F)),
            _full((16, F)), _full((1, F)),
        ],
        out_specs=[
            pl.BlockSpec((NBLK, F), lambda i: (i, 0)),
            pl.BlockSpec((NBLK, F), lambda i: (i, 0)),
        ],
        out_shape=[
            jax.ShapeDtypeStruct((n, F), jnp.float32),
            jax.ShapeDtypeStruct((n, F), jnp.float32),
        ],
    )(nf, pos, wn, bn_, gn, bbn, wp, bp)


def _enc_edge(ef, we, be, ge, bbe):
    Ep = ef.shape[0]
    grid = Ep // EBLK

    def body(ef_r, we_r, be_r, ge_r, bbe_r, e_r):
        x = jnp.dot(ef_r[...], we_r[...], preferred_element_type=jnp.float32) + be_r[...]
        m = jnp.mean(x, axis=-1, keepdims=True)
        v = jnp.mean((x - m) ** 2, axis=-1, keepdims=True)
        e_r[...] = (x - m) / jnp.sqrt(v + BN_EPS) * ge_r[...] + bbe_r[...]

    return pl.pallas_call(
        body,
        grid=(grid,),
        in_specs=[
            pl.BlockSpec((EBLK, 16), lambda i: (i, 0)),
            _full((16, F)), _full((1, F)), _full((1, F)), _full((1, F)),
        ],
        out_specs=pl.BlockSpec((EBLK, F), lambda i: (i, 0)),
        out_shape=jax.ShapeDtypeStruct((Ep, F), jnp.float32),
    )(ef, we, be, ge, bbe)


def _node_matmuls(h, pe, lp):
    n = h.shape[0]
    grid = n // NBLK
    ws = [lp['B1']['W'], lp['B2']['W'],
          lp['A1']['W'][:F], lp['A1']['W'][F:],
          lp['A2']['W'][:F], lp['A2']['W'][F:],
          lp['C1']['W'], lp['C2']['W']]
    bs = [lp['B1']['b'].reshape(1, F), lp['B2']['b'].reshape(1, F),
          lp['A1']['b'].reshape(1, F), lp['A2']['b'].reshape(1, F),
          lp['C1']['b'].reshape(1, F), lp['C2']['b'].reshape(1, F)]

    def body(h_r, pe_r, b1w, b2w, a1h, a1p, a2h, a2p, c1w, c2w,
             b1b, b2b, a1b, a2b, c1b, c2b,
             bh1_r, bh2_r, a1_r, vv_r, c1_r, c2p_r):
        hb = h_r[...]
        pb = pe_r[...]
        dot = lambda a, b: jnp.dot(a, b, preferred_element_type=jnp.float32)
        bh1_r[...] = dot(hb, b1w[...]) + b1b[...]
        bh2_r[...] = dot(hb, b2w[...]) + b2b[...]
        a1_r[...] = dot(hb, a1h[...]) + dot(pb, a1p[...]) + a1b[...]
        vv_r[...] = dot(hb, a2h[...]) + dot(pb, a2p[...]) + a2b[...]
        c1_r[...] = dot(pb, c1w[...]) + c1b[...]
        c2p_r[...] = dot(pb, c2w[...]) + c2b[...]

    blk = pl.BlockSpec((NBLK, F), lambda i: (i, 0))
    return pl.pallas_call(
        body,
        grid=(grid,),
        in_specs=[blk, blk] + [_full((F, F))] * 8 + [_full((1, F))] * 6,
        out_specs=[blk] * 6,
        out_shape=[jax.ShapeDtypeStruct((n, F), jnp.float32)] * 6,
    )(h, pe, *ws, *bs)


def _edge_b3(e, w3, b3):
    Ep = e.shape[0]
    grid = Ep // EBLK

    def body(e_r, w_r, b_r, o_r):
        o_r[...] = jnp.dot(e_r[...], w_r[...], preferred_element_type=jnp.float32) + b_r[...]

    blk = pl.BlockSpec((EBLK, F), lambda i: (i, 0))
    return pl.pallas_call(
        body,
        grid=(grid,),
        in_specs=[blk, _full((F, F)), _full((1, F))],
        out_specs=blk,
        out_shape=jax.ShapeDtypeStruct((Ep, F), jnp.float32),
    )(e, w3, b3)


def _edge_stats(hat, n_valid):
    Ep = hat.shape[0]
    grid = Ep // EBLK

    def body(hat_r, o_r):
        i = pl.program_id(0)
        rows = i * EBLK + lax.broadcasted_iota(jnp.int32, (EBLK, F), 0)
        x = jnp.where(rows < n_valid, hat_r[...], 0.0)
        s = jnp.sum(x, axis=0, keepdims=True)
        sq = jnp.sum(x * x, axis=0, keepdims=True)
        part = jnp.concatenate([s, sq], axis=0)

        @pl.when(i == 0)
        def _():
            o_r[...] = jnp.zeros_like(o_r)

        o_r[...] += part

    return pl.pallas_call(
        body,
        grid=(grid,),
        in_specs=[pl.BlockSpec((EBLK, F), lambda i: (i, 0))],
        out_specs=pl.BlockSpec((2, F), lambda i: (0, 0)),
        out_shape=jax.ShapeDtypeStruct((2, F), jnp.float32),
    )(hat)


def _edge_update(hat, e_prev, e_i, stats, g, b, w3, b3, n_valid):
    Ep = hat.shape[0]
    grid = Ep // EBLK

    def body(hat_r, ep_r, ei_r, st_r, g_r, b_r, w_r, b3_r, en_r, o_r):
        s = st_r[0, :]
        sq = st_r[1, :]
        m = s / n_valid
        var = sq / n_valid - m * m
        scale = g_r[0, :] / jnp.sqrt(var + BN_EPS)
        shift = b_r[0, :] - m * scale
        en = ep_r[...] + jnp.maximum(hat_r[...] * scale + shift, 0.0) + ei_r[...]
        en_r[...] = en
        o_r[...] = jnp.dot(en, w_r[...], preferred_element_type=jnp.float32) + b3_r[...]

    blk = pl.BlockSpec((EBLK, F), lambda i: (i, 0))
    return pl.pallas_call(
        body,
        grid=(grid,),
        in_specs=[blk, blk, blk, _full((2, F)), _full((1, F)), _full((1, F)),
                  _full((F, F)), _full((1, F))],
        out_specs=[blk, blk],
        out_shape=[jax.ShapeDtypeStruct((Ep, F), jnp.float32),
                   jax.ShapeDtypeStruct((Ep, F), jnp.float32)],
    )(hat, e_prev, e_i, stats, g, b, w3, b3)


def _node_stats(a1, hagg):
    n = a1.shape[0]
    grid = n // NBLK

    def body(a_r, h_r, o_r):
        i = pl.program_id(0)
        x = a_r[...] + h_r[...]
        s = jnp.sum(x, axis=0, keepdims=True)
        sq = jnp.sum(x * x, axis=0, keepdims=True)
        part = jnp.concatenate([s, sq], axis=0)

        @pl.when(i == 0)
        def _():
            o_r[...] = jnp.zeros_like(o_r)

        o_r[...] += part

    blk = pl.BlockSpec((NBLK, F), lambda i: (i, 0))
    return pl.pallas_call(
        body,
        grid=(grid,),
        in_specs=[blk, blk],
        out_specs=pl.BlockSpec((2, F), lambda i: (0, 0)),
        out_shape=jax.ShapeDtypeStruct((2, F), jnp.float32),
    )(a1, hagg)


def _node_update(a1, hagg, stats, g, b, h_prev, h_i,
                 c1, pagg, p_prev, p_i, n_valid):
    n = a1.shape[0]
    grid = n // NBLK

    def body(a_r, ha_r, st_r, g_r, b_r, hp_r, hi_r, c_r, pa_r, pp_r, pi_r,
             hn_r, pn_r):
        s = st_r[0, :]
        sq = st_r[1, :]
        m = s / n_valid
        var = sq / n_valid - m * m
        scale = g_r[0, :] / jnp.sqrt(var + BN_EPS)
        shift = b_r[0, :] - m * scale
        x = a_r[...] + ha_r[...]
        hn_r[...] = hp_r[...] + jnp.maximum(x * scale + shift, 0.0) + hi_r[...]
        pn_r[...] = pp_r[...] + jnp.tanh(c_r[...] + pa_r[...]) + pi_r[...]

    blk = pl.BlockSpec((NBLK, F), lambda i: (i, 0))
    return pl.pallas_call(
        body,
        grid=(grid,),
        in_specs=[blk, blk, _full((2, F)), _full((1, F)), _full((1, F)),
                  blk, blk, blk, blk, blk, blk],
        out_specs=[blk, blk],
        out_shape=[jax.ShapeDtypeStruct((n, F), jnp.float32),
                   jax.ShapeDtypeStruct((n, F), jnp.float32)],
    )(a1, hagg, stats, g, b, h_prev, h_i, c1, pagg, p_prev, p_i)


def _pool_mlp(h, gid_row, params, n_graphs):
    n = h.shape[0]

    def body(h_r, gid_r, w1, b1, g1, bb1, w2, b2, g2, bb2, w3, b3, o_r):
        gids = gid_r[...]
        onehot = (lax.broadcasted_iota(jnp.int32, (n_graphs, n), 0)
                  == gids).astype(jnp.float32)
        hg = jnp.dot(onehot, h_r[...], preferred_element_type=jnp.float32)

        def bn_elu(x, gg, bb):
            m = jnp.mean(x, axis=0, keepdims=True)
            var = jnp.mean((x - m) ** 2, axis=0, keepdims=True)
            y = (x - m) / jnp.sqrt(var + BN_EPS) * gg + bb
            return jnp.where(y > 0, y, jnp.exp(jnp.minimum(y, 0.0)) - 1.0)

        x = bn_elu(jnp.dot(hg, w1[...], preferred_element_type=jnp.float32) + b1[...],
                   g1[...], bb1[...])
        x = bn_elu(jnp.dot(x, w2[...], preferred_element_type=jnp.float32) + b2[...],
                   g2[...], bb2[...])
        o_r[...] = jnp.dot(x, w3[...], preferred_element_type=jnp.float32) + b3[...]

    mp = params
    return pl.pallas_call(
        body,
        in_specs=[
            pl.BlockSpec((n, F), lambda: (0, 0)),
            pl.BlockSpec((1, n), lambda: (0, 0)),
            _full2((F, F)), _full2((1, F)), _full2((1, F)), _full2((1, F)),
            _full2((F, 32)), _full2((1, 32)), _full2((1, 32)), _full2((1, 32)),
            _full2((32, 1)), _full2((1, 1)),
        ],
        out_specs=pl.BlockSpec((n_graphs, 1), lambda: (0, 0)),
        out_shape=jax.ShapeDtypeStruct((n_graphs, 1), jnp.float32),
    )(h, gid_row,
      mp['mlp_l1']['W'], mp['mlp_l1']['b'].reshape(1, F),
      mp['mlp_bn1']['g'].reshape(1, F), mp['mlp_bn1']['b'].reshape(1, F),
      mp['mlp_l2']['W'], mp['mlp_l2']['b'].reshape(1, 32),
      mp['mlp_bn2']['g'].reshape(1, 32), mp['mlp_bn2']['b'].reshape(1, 32),
      mp['mlp_l3']['W'], mp['mlp_l3']['b'].reshape(1, 1))


def _full2(shape):
    return pl.BlockSpec(shape, lambda: (0, 0))


def kernel(node_feats, edge_feats, pos_enc, fp, edge_index, graph_ids, params):
    n = node_feats.shape[0]
    e_cnt = edge_index.shape[1]
    n_graphs = fp.shape[0]

    src, dst = edge_index[0], edge_index[1]
    perm = jnp.argsort(dst)
    dst_s = dst[perm]
    src_s = src[perm]
    ef_s = edge_feats[perm]

    n_ranges = -(-n // NPT)
    Ep = e_cnt + K
    Ep = -(-Ep // EBLK) * EBLK
    esz = (-(-(n_ranges + 17) // 16)) * 16

    npad = n_ranges * NPT
    src_p = jnp.zeros((Ep,), jnp.int32).at[:e_cnt].set(src_s)
    dst_p = jnp.full((Ep,), n - 1, jnp.int32).at[:e_cnt].set(dst_s)
    counts = jnp.zeros((npad,), jnp.int32).at[dst].add(1, mode='drop')
    nst = jnp.concatenate([jnp.zeros((1,), jnp.int32),
                           jnp.cumsum(counts, dtype=jnp.int32)])
    estart = nst[jnp.arange(n_ranges + 1, dtype=jnp.int32) * NPT]
    estart_p = jnp.full((esz,), e_cnt, jnp.int32).at[:n_ranges + 1].set(estart)
    nst_p = jnp.full((npad + 32,), e_cnt, jnp.int32).at[:npad + 1].set(nst)
    ef_p = jnp.zeros((Ep, 16), jnp.float32).at[:e_cnt].set(ef_s)

    p = params
    h, pe = _enc_node(
        node_feats, pos_enc,
        p['enc_node']['W'], p['enc_node']['b'].reshape(1, F),
        p['ln_node']['g'].reshape(1, F), p['ln_node']['b'].reshape(1, F),
        p['enc_pose']['W'], p['enc_pose']['b'].reshape(1, F))
    e = _enc_edge(
        ef_p,
        p['enc_edge']['W'], p['enc_edge']['b'].reshape(1, F),
        p['ln_edge']['g'].reshape(1, F), p['ln_edge']['b'].reshape(1, F))

    h_i, e_i, p_i = h, e, pe
    n_layers = len(p['layers'])
    b3e = _edge_b3(e, p['layers'][0]['B3']['W'],
                   p['layers'][0]['B3']['b'].reshape(1, F))
    e_prev = e

    for li, lp in enumerate(p['layers']):
        bh1, bh2, a1, vv, c1, c2p = _node_matmuls(h, pe, lp)
        hat, hagg, pagg = _sc_layer(bh1, bh2, vv, c2p, b3e,
                                    src_p, dst_p, estart_p, nst_p, n_ranges)
        if li + 1 < n_layers:
            stats_e = _edge_stats(hat, e_cnt)
            nlp = p['layers'][li + 1]
            e_prev, b3e = _edge_update(
                hat, e_prev, e_i, stats_e,
                lp['bn_e']['g'].reshape(1, F), lp['bn_e']['b'].reshape(1, F),
                nlp['B3']['W'], nlp['B3']['b'].reshape(1, F), e_cnt)
        stats_h = _node_stats(a1, hagg)
        h, pe = _node_update(
            a1, hagg, stats_h,
            lp['bn_h']['g'].reshape(1, F), lp['bn_h']['b'].reshape(1, F),
            h, h_i, c1, pagg, pe, p_i, n)

    gid_row = graph_ids.reshape(1, n).astype(jnp.int32)
    return _pool_mlp(h, gid_row, p, n_graphs)


# speedup vs baseline: 4.1885x; 1.3179x over previous
import functools

import jax
import jax.numpy as jnp
from jax import lax
from jax.experimental import pallas as pl
from jax.experimental.pallas import tpu as pltpu
from jax.experimental.pallas import tpu_sc as plsc

F = 128
NPT = 64
K = 128
NW = 32
EBLK = 1152
NBLK = 2000
BN_EPS = 1e-5
ETA_EPS = 1e-6


def _sigmoid16(x):
    return 1.0 / (1.0 + jnp.exp(-x))


def _sc_layer(bh1, bh2, vv, c2p, b3e, src_p, dst_p, estart_p, nst_p,
                 n_ranges):
    Ep = b3e.shape[0]
    Npad = n_ranges * NPT
    ESZ = estart_p.shape[0]
    mesh = plsc.VectorSubcoreMesh(core_axis_name="c", subcore_axis_name="s",
                                  num_cores=2, num_subcores=16)

    @functools.partial(
        pl.kernel,
        out_type=(
            jax.ShapeDtypeStruct((Ep, F), jnp.float32),
            jax.ShapeDtypeStruct((Npad, F), jnp.float32),
            jax.ShapeDtypeStruct((Npad, F), jnp.float32),
        ),
        mesh=mesh,
        scratch_types=[
            pltpu.VMEM((K, F), jnp.float32),
            pltpu.VMEM((K, F), jnp.float32),
            pltpu.VMEM((K, F), jnp.float32),
            pltpu.VMEM((K, F), jnp.float32),
            pltpu.VMEM((K, F), jnp.float32),
            pltpu.VMEM((K, F), jnp.float32),
            pltpu.VMEM((NPT, F), jnp.float32),
            pltpu.VMEM((NPT, F), jnp.float32),
            pltpu.VMEM((NPT, F), jnp.float32),
            pltpu.VMEM((K,), jnp.int32),
            pltpu.VMEM((K,), jnp.int32),
            pltpu.VMEM((K,), jnp.int32),
            pltpu.VMEM((K,), jnp.int32),
            pltpu.VMEM((K + 16,), jnp.int32),
            pltpu.VMEM((K + 16,), jnp.int32),
            pltpu.VMEM((ESZ,), jnp.int32),
            pltpu.VMEM((NPT + 32,), jnp.int32),
            pltpu.SemaphoreType.DMA,
            pltpu.SemaphoreType.DMA,
            pltpu.SemaphoreType.DMA,
            pltpu.SemaphoreType.DMA,
            pltpu.SemaphoreType.DMA,
        ],
    )
    def k(bh1_h, bh2_h, vv_h, c2p_h, b3e_h, src_h, dst_h, est_h, nst_h,
          hat_h, hagg_h, pagg_h,
          A0, A1, B0, B1, C0, C1, ss, hacc, pacc,
          sidx0, sidx1, didx0, didx1, dsm0, dsm1, estv, nstv,
          isem0, isem1, dsem0, dsem1, wsem):
        w = lax.axis_index("s") * 2 + lax.axis_index("c")
        pltpu.sync_copy(est_h, estv)

        A_ = (A0, A1)
        B_ = (B0, B1)
        C_ = (C0, C1)
        SI = (sidx0, sidx1)
        DI = (didx0, didx1)
        DS = (dsm0, dsm1)
        IS = (isem0, isem1)
        DSEM = (dsem0, dsem1)

        n_mine = (n_ranges - 1 - w) // NW + 1

        def sval(ref, i):
            return ref[pl.ds(i, 16)][0]

        def per_range(ri, _):
            r = w + ri * NW
            base = r * NPT
            e0 = sval(estv, r)
            e1 = sval(estv, r + 1)
            c0 = (e0 // 8) * 8
            nch = jnp.maximum(e1 - c0 + K - 1, 0) // K
            pltpu.sync_copy(nst_h.at[pl.ds(base, NPT + 32)], nstv)

            def zero_rows(nl, _):
                zz = jnp.zeros((16,), jnp.float32)
                for j in range(F // 16):
                    sl = pl.ds(j * 16, 16)
                    ss[nl, sl] = zz
                    hacc[nl, sl] = zz
                    pacc[nl, sl] = zz
                return 0

            lax.fori_loop(0, NPT, zero_rows, 0)

            def chunk_nodes(b):
                nlo = jnp.clip(sval(DS[b], 0) - base, 0, NPT - 1)
                nhi = jnp.clip(DS[b][pl.ds(K - 16, 16)][15] - base,
                               0, NPT - 1)
                return nlo, nhi + 1

            def issue_idx(ci, b):
                c = c0 + ci * K
                pltpu.async_copy(src_h.at[pl.ds(c, K)], SI[b], IS[b])
                pltpu.async_copy(dst_h.at[pl.ds(c, K)], DI[b], IS[b])
                pltpu.async_copy(dst_h.at[pl.ds(c, K)],
                                 DS[b].at[pl.ds(0, K)], IS[b])

            def wait_idx(b):
                pltpu.make_async_copy(src_h.at[pl.ds(0, K)], SI[b],
                                      IS[b]).wait()
                pltpu.make_async_copy(dst_h.at[pl.ds(0, K)], DI[b],
                                      IS[b]).wait()
                pltpu.make_async_copy(dst_h.at[pl.ds(0, K)],
                                      DS[b].at[pl.ds(0, K)], IS[b]).wait()

            def issue_data(ci, b, is_a):
                c = c0 + ci * K
                if is_a:
                    pltpu.async_copy(b3e_h.at[pl.ds(c, K)], A_[b], DSEM[b])
                    pltpu.async_copy(bh1_h.at[DI[b]], B_[b], DSEM[b])
                    pltpu.async_copy(bh2_h.at[SI[b]], C_[b], DSEM[b])
                else:
                    pltpu.async_copy(hat_h.at[pl.ds(c, K)], A_[b], DSEM[b])
                    pltpu.async_copy(vv_h.at[SI[b]], B_[b], DSEM[b])
                    pltpu.async_copy(c2p_h.at[SI[b]], C_[b], DSEM[b])

            def wait_data(b):
                for buf in (A_[b], B_[b], C_[b]):
                    pltpu.make_async_copy(b3e_h.at[pl.ds(0, K)], buf,
                                          DSEM[b]).wait()

            def prefetch(cur, nb, is_a):
                nxt = cur + 1

                @pl.when(nxt < nch)
                def _():
                    issue_idx(nxt, nb)
                    wait_idx(nb)
                    issue_data(nxt, nb, is_a)

            def compute_a(cur, b):
                c = c0 + cur * K

                def bulk(el, _):
                    for j in range(F // 16):
                        sl = pl.ds(j * 16, 16)
                        A_[b][el, sl] = (A_[b][el, sl] + B_[b][el, sl]
                                         + C_[b][el, sl])
                    return 0

                lax.fori_loop(0, K, bulk, 0)
                pltpu.async_copy(A_[b], hat_h.at[pl.ds(c, K)], wsem)

                nlo, nhi = chunk_nodes(b)

                def per_node(nl, _):
                    es = jnp.maximum(sval(nstv, nl), c)
                    ee = jnp.minimum(sval(nstv, nl + 1), c + K)

                    def edge_body(e, accs):
                        el = e - c
                        return tuple(
                            accs[j] + _sigmoid16(A_[b][el, pl.ds(j * 16, 16)])
                            for j in range(F // 16))

                    accs = lax.fori_loop(
                        es, ee, edge_body,
                        tuple(jnp.zeros((16,), jnp.float32)
                              for _ in range(F // 16)))
                    for j in range(F // 16):
                        sl = pl.ds(j * 16, 16)
                        ss[nl, sl] = ss[nl, sl] + accs[j]
                    return 0

                lax.fori_loop(nlo, nhi, per_node, 0)
                pltpu.make_async_copy(A_[b], hat_h.at[pl.ds(c, K)],
                                      wsem).wait()

            def compute_b(cur, b):
                c = c0 + cur * K
                nlo, nhi = chunk_nodes(b)

                def per_node(nl, _):
                    es = jnp.maximum(sval(nstv, nl), c)
                    ee = jnp.minimum(sval(nstv, nl + 1), c + K)
                    inv = [ss[nl, pl.ds(j * 16, 16)] for j in range(F // 16)]

                    def edge_body(e, accs):
                        el = e - c
                        out = []
                        for j in range(F // 16):
                            sl = pl.ds(j * 16, 16)
                            eta = _sigmoid16(A_[b][el, sl]) * inv[j]
                            out.append(accs[j] + eta * B_[b][el, sl])
                            out.append(accs[j + 8] + eta * C_[b][el, sl])
                        return tuple(out[0::2]) + tuple(out[1::2])

                    accs = lax.fori_loop(
                        es, ee, edge_body,
                        tuple(jnp.zeros((16,), jnp.float32)
                              for _ in range(2 * (F // 16))))
                    for j in range(F // 16):
                        sl = pl.ds(j * 16, 16)
                        hacc[nl, sl] = hacc[nl, sl] + accs[j]
                        pacc[nl, sl] = pacc[nl, sl] + accs[j + 8]
                    return 0

                lax.fori_loop(nlo, nhi, per_node, 0)

            def run_phase(is_a):
                @pl.when(nch > 0)
                def _():
                    issue_idx(0, 0)
                    wait_idx(0)
                    issue_data(0, 0, is_a)

                    def step(ci, _):
                        for b in range(2):
                            cur_b = b

                            def mk(cur_b):
                                def inner():
                                    cur = 2 * ci + cur_b
                                    prefetch(cur, (cur_b + 1) % 2, is_a)
                                    wait_data(cur_b)
                                    if is_a:
                                        compute_a(cur, cur_b)
                                    else:
                                        compute_b(cur, cur_b)
                                return inner

                            pl.when(2 * ci + b < nch)(mk(b))
                        return 0

                    lax.fori_loop(0, (nch + 1) // 2, step, 0)

            run_phase(True)

            def invert(nl, _):
                for j in range(F // 16):
                    sl = pl.ds(j * 16, 16)
                    ss[nl, sl] = 1.0 / (ss[nl, sl] + ETA_EPS)
                return 0

            lax.fori_loop(0, NPT, invert, 0)

            run_phase(False)

            pltpu.sync_copy(hacc, hagg_h.at[pl.ds(base, NPT)])
            pltpu.sync_copy(pacc, pagg_h.at[pl.ds(base, NPT)])
            return 0

        lax.fori_loop(0, n_mine, per_range, 0)

    return k(bh1, bh2, vv, c2p, b3e, src_p, dst_p, estart_p, nst_p)


def _full(shape):
    return pl.BlockSpec(shape, lambda i: (0, 0))


def _enc_node(nf, pos, wn, bn_, gn, bbn, wp, bp):
    n = nf.shape[0]
    grid = n // NBLK

    def body(nf_r, pos_r, wn_r, bn_r, gn_r, bbn_r, wp_r, bp_r, h_r, pe_r):
        x = jnp.dot(nf_r[...], wn_r[...], preferred_element_type=jnp.float32) + bn_r[...]
        m = jnp.mean(x, axis=-1, keepdims=True)
        v = jnp.mean((x - m) ** 2, axis=-1, keepdims=True)
        h_r[...] = (x - m) / jnp.sqrt(v + BN_EPS) * gn_r[...] + bbn_r[...]
        pe_r[...] = jnp.dot(pos_r[...], wp_r[...], preferred_element_type=jnp.float32) + bp_r[...]

    return pl.pallas_call(
        body,
        grid=(grid,),
        in_specs=[
            pl.BlockSpec((NBLK, F), lambda i: (i, 0)),
            pl.BlockSpec((NBLK, 16), lambda i: (i, 0)),
            _full((F, F)), _full((1, F)), _full((1, F)), _full((1, F)),
            _full((16, F)), _full((1, F)),
        ],
        out_specs=[
            pl.BlockSpec((NBLK, F), lambda i: (i, 0)),
            pl.BlockSpec((NBLK, F), lambda i: (i, 0)),
        ],
        out_shape=[
            jax.ShapeDtypeStruct((n, F), jnp.float32),
            jax.ShapeDtypeStruct((n, F), jnp.float32),
        ],
    )(nf, pos, wn, bn_, gn, bbn, wp, bp)


def _enc_edge(ef, we, be, ge, bbe):
    Ep = ef.shape[0]
    grid = Ep // EBLK

    def body(ef_r, we_r, be_r, ge_r, bbe_r, e_r):
        x = jnp.dot(ef_r[...], we_r[...], preferred_element_type=jnp.float32) + be_r[...]
        m = jnp.mean(x, axis=-1, keepdims=True)
        v = jnp.mean((x - m) ** 2, axis=-1, keepdims=True)
        e_r[...] = (x - m) / jnp.sqrt(v + BN_EPS) * ge_r[...] + bbe_r[...]

    return pl.pallas_call(
        body,
        grid=(grid,),
        in_specs=[
            pl.BlockSpec((EBLK, 16), lambda i: (i, 0)),
            _full((16, F)), _full((1, F)), _full((1, F)), _full((1, F)),
        ],
        out_specs=pl.BlockSpec((EBLK, F), lambda i: (i, 0)),
        out_shape=jax.ShapeDtypeStruct((Ep, F), jnp.float32),
    )(ef, we, be, ge, bbe)


def _node_matmuls(h, pe, lp):
    n = h.shape[0]
    grid = n // NBLK
    ws = [lp['B1']['W'], lp['B2']['W'],
          lp['A1']['W'][:F], lp['A1']['W'][F:],
          lp['A2']['W'][:F], lp['A2']['W'][F:],
          lp['C1']['W'], lp['C2']['W']]
    bs = [lp['B1']['b'].reshape(1, F), lp['B2']['b'].reshape(1, F),
          lp['A1']['b'].reshape(1, F), lp['A2']['b'].reshape(1, F),
          lp['C1']['b'].reshape(1, F), lp['C2']['b'].reshape(1, F)]

    def body(h_r, pe_r, b1w, b2w, a1h, a1p, a2h, a2p, c1w, c2w,
             b1b, b2b, a1b, a2b, c1b, c2b,
             bh1_r, bh2_r, a1_r, vv_r, c1_r, c2p_r):
        hb = h_r[...]
        pb = pe_r[...]
        dot = lambda a, b: jnp.dot(a, b, preferred_element_type=jnp.float32)
        bh1_r[...] = dot(hb, b1w[...]) + b1b[...]
        bh2_r[...] = dot(hb, b2w[...]) + b2b[...]
        a1_r[...] = dot(hb, a1h[...]) + dot(pb, a1p[...]) + a1b[...]
        vv_r[...] = dot(hb, a2h[...]) + dot(pb, a2p[...]) + a2b[...]
        c1_r[...] = dot(pb, c1w[...]) + c1b[...]
        c2p_r[...] = dot(pb, c2w[...]) + c2b[...]

    blk = pl.BlockSpec((NBLK, F), lambda i: (i, 0))
    return pl.pallas_call(
        body,
        grid=(grid,),
        in_specs=[blk, blk] + [_full((F, F))] * 8 + [_full((1, F))] * 6,
        out_specs=[blk] * 6,
        out_shape=[jax.ShapeDtypeStruct((n, F), jnp.float32)] * 6,
    )(h, pe, *ws, *bs)


def _edge_b3(e, w3, b3):
    Ep = e.shape[0]
    grid = Ep // EBLK

    def body(e_r, w_r, b_r, o_r):
        o_r[...] = jnp.dot(e_r[...], w_r[...], preferred_element_type=jnp.float32) + b_r[...]

    blk = pl.BlockSpec((EBLK, F), lambda i: (i, 0))
    return pl.pallas_call(
        body,
        grid=(grid,),
        in_specs=[blk, _full((F, F)), _full((1, F))],
        out_specs=blk,
        out_shape=jax.ShapeDtypeStruct((Ep, F), jnp.float32),
    )(e, w3, b3)


def _edge_stats(hat, n_valid):
    Ep = hat.shape[0]
    grid = Ep // EBLK

    def body(hat_r, o_r):
        i = pl.program_id(0)
        rows = i * EBLK + lax.broadcasted_iota(jnp.int32, (EBLK, F), 0)
        x = jnp.where(rows < n_valid, hat_r[...], 0.0)
        s = jnp.sum(x, axis=0, keepdims=True)
        sq = jnp.sum(x * x, axis=0, keepdims=True)
        part = jnp.concatenate([s, sq], axis=0)

        @pl.when(i == 0)
        def _():
            o_r[...] = jnp.zeros_like(o_r)

        o_r[...] += part

    return pl.pallas_call(
        body,
        grid=(grid,),
        in_specs=[pl.BlockSpec((EBLK, F), lambda i: (i, 0))],
        out_specs=pl.BlockSpec((2, F), lambda i: (0, 0)),
        out_shape=jax.ShapeDtypeStruct((2, F), jnp.float32),
    )(hat)


def _edge_update(hat, e_prev, e_i, stats, g, b, w3, b3, n_valid):
    Ep = hat.shape[0]
    grid = Ep // EBLK

    def body(hat_r, ep_r, ei_r, st_r, g_r, b_r, w_r, b3_r, en_r, o_r):
        s = st_r[0, :]
        sq = st_r[1, :]
        m = s / n_valid
        var = sq / n_valid - m * m
        scale = g_r[0, :] / jnp.sqrt(var + BN_EPS)
        shift = b_r[0, :] - m * scale
        en = ep_r[...] + jnp.maximum(hat_r[...] * scale + shift, 0.0) + ei_r[...]
        en_r[...] = en
        o_r[...] = jnp.dot(en, w_r[...], preferred_element_type=jnp.float32) + b3_r[...]

    blk = pl.BlockSpec((EBLK, F), lambda i: (i, 0))
    return pl.pallas_call(
        body,
        grid=(grid,),
        in_specs=[blk, blk, blk, _full((2, F)), _full((1, F)), _full((1, F)),
                  _full((F, F)), _full((1, F))],
        out_specs=[blk, blk],
        out_shape=[jax.ShapeDtypeStruct((Ep, F), jnp.float32),
                   jax.ShapeDtypeStruct((Ep, F), jnp.float32)],
    )(hat, e_prev, e_i, stats, g, b, w3, b3)


def _node_stats(a1, hagg):
    n = a1.shape[0]
    grid = n // NBLK

    def body(a_r, h_r, o_r):
        i = pl.program_id(0)
        x = a_r[...] + h_r[...]
        s = jnp.sum(x, axis=0, keepdims=True)
        sq = jnp.sum(x * x, axis=0, keepdims=True)
        part = jnp.concatenate([s, sq], axis=0)

        @pl.when(i == 0)
        def _():
            o_r[...] = jnp.zeros_like(o_r)

        o_r[...] += part

    blk = pl.BlockSpec((NBLK, F), lambda i: (i, 0))
    return pl.pallas_call(
        body,
        grid=(grid,),
        in_specs=[blk, blk],
        out_specs=pl.BlockSpec((2, F), lambda i: (0, 0)),
        out_shape=jax.ShapeDtypeStruct((2, F), jnp.float32),
    )(a1, hagg)


def _node_update(a1, hagg, stats, g, b, h_prev, h_i,
                 c1, pagg, p_prev, p_i, n_valid):
    n = a1.shape[0]
    grid = n // NBLK

    def body(a_r, ha_r, st_r, g_r, b_r, hp_r, hi_r, c_r, pa_r, pp_r, pi_r,
             hn_r, pn_r):
        s = st_r[0, :]
        sq = st_r[1, :]
        m = s / n_valid
        var = sq / n_valid - m * m
        scale = g_r[0, :] / jnp.sqrt(var + BN_EPS)
        shift = b_r[0, :] - m * scale
        x = a_r[...] + ha_r[...]
        hn_r[...] = hp_r[...] + jnp.maximum(x * scale + shift, 0.0) + hi_r[...]
        pn_r[...] = pp_r[...] + jnp.tanh(c_r[...] + pa_r[...]) + pi_r[...]

    blk = pl.BlockSpec((NBLK, F), lambda i: (i, 0))
    return pl.pallas_call(
        body,
        grid=(grid,),
        in_specs=[blk, blk, _full((2, F)), _full((1, F)), _full((1, F)),
                  blk, blk, blk, blk, blk, blk],
        out_specs=[blk, blk],
        out_shape=[jax.ShapeDtypeStruct((n, F), jnp.float32),
                   jax.ShapeDtypeStruct((n, F), jnp.float32)],
    )(a1, hagg, stats, g, b, h_prev, h_i, c1, pagg, p_prev, p_i)


def _pool_mlp(h, gid_row, params, n_graphs):
    n = h.shape[0]

    def body(h_r, gid_r, w1, b1, g1, bb1, w2, b2, g2, bb2, w3, b3, o_r):
        gids = gid_r[...]
        onehot = (lax.broadcasted_iota(jnp.int32, (n_graphs, n), 0)
                  == gids).astype(jnp.float32)
        hg = jnp.dot(onehot, h_r[...], preferred_element_type=jnp.float32)

        def bn_elu(x, gg, bb):
            m = jnp.mean(x, axis=0, keepdims=True)
            var = jnp.mean((x - m) ** 2, axis=0, keepdims=True)
            y = (x - m) / jnp.sqrt(var + BN_EPS) * gg + bb
            return jnp.where(y > 0, y, jnp.exp(jnp.minimum(y, 0.0)) - 1.0)

        x = bn_elu(jnp.dot(hg, w1[...], preferred_element_type=jnp.float32) + b1[...],
                   g1[...], bb1[...])
        x = bn_elu(jnp.dot(x, w2[...], preferred_element_type=jnp.float32) + b2[...],
                   g2[...], bb2[...])
        o_r[...] = jnp.dot(x, w3[...], preferred_element_type=jnp.float32) + b3[...]

    mp = params
    return pl.pallas_call(
        body,
        in_specs=[
            pl.BlockSpec((n, F), lambda: (0, 0)),
            pl.BlockSpec((1, n), lambda: (0, 0)),
            _full2((F, F)), _full2((1, F)), _full2((1, F)), _full2((1, F)),
            _full2((F, 32)), _full2((1, 32)), _full2((1, 32)), _full2((1, 32)),
            _full2((32, 1)), _full2((1, 1)),
        ],
        out_specs=pl.BlockSpec((n_graphs, 1), lambda: (0, 0)),
        out_shape=jax.ShapeDtypeStruct((n_graphs, 1), jnp.float32),
    )(h, gid_row,
      mp['mlp_l1']['W'], mp['mlp_l1']['b'].reshape(1, F),
      mp['mlp_bn1']['g'].reshape(1, F), mp['mlp_bn1']['b'].reshape(1, F),
      mp['mlp_l2']['W'], mp['mlp_l2']['b'].reshape(1, 32),
      mp['mlp_bn2']['g'].reshape(1, 32), mp['mlp_bn2']['b'].reshape(1, 32),
      mp['mlp_l3']['W'], mp['mlp_l3']['b'].reshape(1, 1))


def _full2(shape):
    return pl.BlockSpec(shape, lambda: (0, 0))


def kernel(node_feats, edge_feats, pos_enc, fp, edge_index, graph_ids, params):
    n = node_feats.shape[0]
    e_cnt = edge_index.shape[1]
    n_graphs = fp.shape[0]

    src, dst = edge_index[0], edge_index[1]
    perm = jnp.argsort(dst)
    dst_s = dst[perm]
    src_s = src[perm]
    ef_s = edge_feats[perm]

    n_ranges = -(-n // NPT)
    Ep = e_cnt + K
    Ep = -(-Ep // EBLK) * EBLK
    esz = (-(-(n_ranges + 17) // 16)) * 16

    npad = n_ranges * NPT
    src_p = jnp.zeros((Ep,), jnp.int32).at[:e_cnt].set(src_s)
    dst_p = jnp.full((Ep,), n - 1, jnp.int32).at[:e_cnt].set(dst_s)
    counts = jnp.zeros((npad,), jnp.int32).at[dst].add(1, mode='drop')
    nst = jnp.concatenate([jnp.zeros((1,), jnp.int32),
                           jnp.cumsum(counts, dtype=jnp.int32)])
    estart = nst[jnp.arange(n_ranges + 1, dtype=jnp.int32) * NPT]
    estart_p = jnp.full((esz,), e_cnt, jnp.int32).at[:n_ranges + 1].set(estart)
    nst_p = jnp.full((npad + 32,), e_cnt, jnp.int32).at[:npad + 1].set(nst)
    ef_p = jnp.zeros((Ep, 16), jnp.float32).at[:e_cnt].set(ef_s)

    p = params
    h, pe = _enc_node(
        node_feats, pos_enc,
        p['enc_node']['W'], p['enc_node']['b'].reshape(1, F),
        p['ln_node']['g'].reshape(1, F), p['ln_node']['b'].reshape(1, F),
        p['enc_pose']['W'], p['enc_pose']['b'].reshape(1, F))
    e = _enc_edge(
        ef_p,
        p['enc_edge']['W'], p['enc_edge']['b'].reshape(1, F),
        p['ln_edge']['g'].reshape(1, F), p['ln_edge']['b'].reshape(1, F))

    h_i, e_i, p_i = h, e, pe
    n_layers = len(p['layers'])
    b3e = _edge_b3(e, p['layers'][0]['B3']['W'],
                   p['layers'][0]['B3']['b'].reshape(1, F))
    e_prev = e

    for li, lp in enumerate(p['layers']):
        bh1, bh2, a1, vv, c1, c2p = _node_matmuls(h, pe, lp)
        hat, hagg, pagg = _sc_layer(bh1, bh2, vv, c2p, b3e,
                                    src_p, dst_p, estart_p, nst_p, n_ranges)
        if li + 1 < n_layers:
            stats_e = _edge_stats(hat, e_cnt)
            nlp = p['layers'][li + 1]
            e_prev, b3e = _edge_update(
                hat, e_prev, e_i, stats_e,
                lp['bn_e']['g'].reshape(1, F), lp['bn_e']['b'].reshape(1, F),
                nlp['B3']['W'], nlp['B3']['b'].reshape(1, F), e_cnt)
        stats_h = _node_stats(a1, hagg)
        h, pe = _node_update(
            a1, hagg, stats_h,
            lp['bn_h']['g'].reshape(1, F), lp['bn_h']['b'].reshape(1, F),
            h, h_i, c1, pagg, pe, p_i, n)

    gid_row = graph_ids.reshape(1, n).astype(jnp.int32)
    return _pool_mlp(h, gid_row, p, n_graphs)


# speedup vs baseline: 4.6621x; 1.1131x over previous
import functools

import jax
import jax.numpy as jnp
from jax import lax
from jax.experimental import pallas as pl
from jax.experimental.pallas import tpu as pltpu
from jax.experimental.pallas import tpu_sc as plsc

F = 128
NPT = 64
K = 64
NW = 32
EBLK = 1152
NBLK = 2000
BN_EPS = 1e-5
ETA_EPS = 1e-6


def _sigmoid16(x):
    return 1.0 / (1.0 + jnp.exp(-x))


def _sc_layer(bh1, bh2, vv, c2p, b3e, src_p, dst_p, estart_p, nst_p,
                 n_ranges):
    Ep = b3e.shape[0]
    Npad = n_ranges * NPT
    ESZ = estart_p.shape[0]
    mesh = plsc.VectorSubcoreMesh(core_axis_name="c", subcore_axis_name="s",
                                  num_cores=2, num_subcores=16)

    @functools.partial(
        pl.kernel,
        out_type=(
            jax.ShapeDtypeStruct((Ep, F), jnp.float32),
            jax.ShapeDtypeStruct((Npad, F), jnp.float32),
            jax.ShapeDtypeStruct((Npad, F), jnp.float32),
        ),
        mesh=mesh,
        scratch_types=[
            pltpu.VMEM((K, F), jnp.float32),
            pltpu.VMEM((K, F), jnp.float32),
            pltpu.VMEM((K, F), jnp.float32),
            pltpu.VMEM((K, F), jnp.float32),
            pltpu.VMEM((K, F), jnp.float32),
            pltpu.VMEM((K, F), jnp.float32),
            pltpu.VMEM((K, F), jnp.float32),
            pltpu.VMEM((K, F), jnp.float32),
            pltpu.VMEM((K, F), jnp.float32),
            pltpu.VMEM((K, F), jnp.float32),
            pltpu.VMEM((NPT, F), jnp.float32),
            pltpu.VMEM((NPT, F), jnp.float32),
            pltpu.VMEM((NPT, F), jnp.float32),
            pltpu.VMEM((K,), jnp.int32),
            pltpu.VMEM((K,), jnp.int32),
            pltpu.VMEM((K,), jnp.int32),
            pltpu.VMEM((K,), jnp.int32),
            pltpu.VMEM((K + 16,), jnp.int32),
            pltpu.VMEM((K + 16,), jnp.int32),
            pltpu.VMEM((ESZ,), jnp.int32),
            pltpu.VMEM((NPT + 32,), jnp.int32),
            pltpu.SemaphoreType.DMA,
            pltpu.SemaphoreType.DMA,
            pltpu.SemaphoreType.DMA,
            pltpu.SemaphoreType.DMA,
            pltpu.SemaphoreType.DMA,
        ],
    )
    def k(bh1_h, bh2_h, vv_h, c2p_h, b3e_h, src_h, dst_h, est_h, nst_h,
          hat_h, hagg_h, pagg_h,
          A0, A1, B0, B1, C0, C1, D0, D1, E0, E1, ss, hacc, pacc,
          sidx0, sidx1, didx0, didx1, dsm0, dsm1, estv, nstv,
          isem0, isem1, dsem0, dsem1, wsem):
        w = lax.axis_index("s") * 2 + lax.axis_index("c")
        pltpu.sync_copy(est_h, estv)

        A_ = (A0, A1)
        B_ = (B0, B1)
        C_ = (C0, C1)
        D_ = (D0, D1)
        E_ = (E0, E1)
        SI = (sidx0, sidx1)
        DI = (didx0, didx1)
        DS = (dsm0, dsm1)
        IS = (isem0, isem1)
        DSEM = (dsem0, dsem1)

        n_mine = (n_ranges - 1 - w) // NW + 1

        def sval(ref, i):
            return ref[pl.ds(i, 16)][0]

        def per_range(ri, _):
            r = w + ri * NW
            base = r * NPT
            e0 = sval(estv, r)
            e1 = sval(estv, r + 1)
            c0 = (e0 // 8) * 8
            nch = jnp.maximum(e1 - c0 + K - 1, 0) // K
            pltpu.sync_copy(nst_h.at[pl.ds(base, NPT + 32)], nstv)

            def zero_rows(nl, _):
                zz = jnp.zeros((16,), jnp.float32)
                for j in range(F // 16):
                    sl = pl.ds(j * 16, 16)
                    ss[nl, sl] = zz
                    hacc[nl, sl] = zz
                    pacc[nl, sl] = zz
                return 0

            lax.fori_loop(0, NPT, zero_rows, 0)

            def issue_idx(ci, b):
                c = c0 + ci * K
                pltpu.async_copy(src_h.at[pl.ds(c, K)], SI[b], IS[b])
                pltpu.async_copy(dst_h.at[pl.ds(c, K)], DI[b], IS[b])
                pltpu.async_copy(dst_h.at[pl.ds(c, K)],
                                 DS[b].at[pl.ds(0, K)], IS[b])

            def wait_idx(b):
                pltpu.make_async_copy(src_h.at[pl.ds(0, K)], SI[b],
                                      IS[b]).wait()
                pltpu.make_async_copy(dst_h.at[pl.ds(0, K)], DI[b],
                                      IS[b]).wait()
                pltpu.make_async_copy(dst_h.at[pl.ds(0, K)],
                                      DS[b].at[pl.ds(0, K)], IS[b]).wait()

            def issue_data(ci, b):
                c = c0 + ci * K
                pltpu.async_copy(b3e_h.at[pl.ds(c, K)], A_[b], DSEM[b])
                pltpu.async_copy(bh1_h.at[DI[b]], B_[b], DSEM[b])
                pltpu.async_copy(bh2_h.at[SI[b]], C_[b], DSEM[b])
                pltpu.async_copy(vv_h.at[SI[b]], D_[b], DSEM[b])
                pltpu.async_copy(c2p_h.at[SI[b]], E_[b], DSEM[b])

            def wait_data(b):
                for buf in (A_[b], B_[b], C_[b], D_[b], E_[b]):
                    pltpu.make_async_copy(b3e_h.at[pl.ds(0, K)], buf,
                                          DSEM[b]).wait()

            def prefetch(cur, nb):
                nxt = cur + 1

                @pl.when(nxt < nch)
                def _():
                    issue_idx(nxt, nb)
                    wait_idx(nb)
                    issue_data(nxt, nb)

            def compute(cur, b):
                c = c0 + cur * K

                def bulk(el, _):
                    for j in range(F // 16):
                        sl = pl.ds(j * 16, 16)
                        A_[b][el, sl] = (A_[b][el, sl] + B_[b][el, sl]
                                         + C_[b][el, sl])
                    return 0

                lax.fori_loop(0, K, bulk, 0)
                pltpu.async_copy(A_[b], hat_h.at[pl.ds(c, K)], wsem)

                nlo = jnp.clip(sval(DS[b], 0) - base, 0, NPT - 1)
                nhi = jnp.clip(DS[b][pl.ds(K - 16, 16)][15] - base,
                               0, NPT - 1) + 1

                def per_node(nl, _):
                    es = jnp.maximum(sval(nstv, nl), c)
                    ee = jnp.minimum(sval(nstv, nl + 1), c + K)

                    def edge_body(e, accs):
                        el = e - c
                        out = []
                        for j in range(F // 16):
                            sl = pl.ds(j * 16, 16)
                            sig = _sigmoid16(A_[b][el, sl])
                            out.append(accs[j] + sig)
                            out.append(accs[j + 8] + sig * D_[b][el, sl])
                            out.append(accs[j + 16] + sig * E_[b][el, sl])
                        return (tuple(out[0::3]) + tuple(out[1::3])
                                + tuple(out[2::3]))

                    accs = lax.fori_loop(
                        es, ee, edge_body,
                        tuple(jnp.zeros((16,), jnp.float32)
                              for _ in range(3 * (F // 16))))
                    for j in range(F // 16):
                        sl = pl.ds(j * 16, 16)
                        ss[nl, sl] = ss[nl, sl] + accs[j]
                        hacc[nl, sl] = hacc[nl, sl] + accs[j + 8]
                        pacc[nl, sl] = pacc[nl, sl] + accs[j + 16]
                    return 0

                lax.fori_loop(nlo, nhi, per_node, 0)
                pltpu.make_async_copy(A_[b], hat_h.at[pl.ds(c, K)],
                                      wsem).wait()

            @pl.when(nch > 0)
            def _():
                issue_idx(0, 0)
                wait_idx(0)
                issue_data(0, 0)

                def step(ci, _):
                    for b in range(2):
                        def mk(cur_b):
                            def inner():
                                cur = 2 * ci + cur_b
                                prefetch(cur, (cur_b + 1) % 2)
                                wait_data(cur_b)
                                compute(cur, cur_b)
                            return inner

                        pl.when(2 * ci + b < nch)(mk(b))
                    return 0

                lax.fori_loop(0, (nch + 1) // 2, step, 0)

            def fin(nl, _):
                for j in range(F // 16):
                    sl = pl.ds(j * 16, 16)
                    inv = 1.0 / (ss[nl, sl] + ETA_EPS)
                    hacc[nl, sl] = hacc[nl, sl] * inv
                    pacc[nl, sl] = pacc[nl, sl] * inv
                return 0

            lax.fori_loop(0, NPT, fin, 0)

            pltpu.sync_copy(hacc, hagg_h.at[pl.ds(base, NPT)])
            pltpu.sync_copy(pacc, pagg_h.at[pl.ds(base, NPT)])
            return 0

        lax.fori_loop(0, n_mine, per_range, 0)

    return k(bh1, bh2, vv, c2p, b3e, src_p, dst_p, estart_p, nst_p)


def _full(shape):
    return pl.BlockSpec(shape, lambda i: (0, 0))


def _enc_node(nf, pos, wn, bn_, gn, bbn, wp, bp):
    n = nf.shape[0]
    grid = n // NBLK

    def body(nf_r, pos_r, wn_r, bn_r, gn_r, bbn_r, wp_r, bp_r, h_r, pe_r):
        x = jnp.dot(nf_r[...], wn_r[...], preferred_element_type=jnp.float32) + bn_r[...]
        m = jnp.mean(x, axis=-1, keepdims=True)
        v = jnp.mean((x - m) ** 2, axis=-1, keepdims=True)
        h_r[...] = (x - m) / jnp.sqrt(v + BN_EPS) * gn_r[...] + bbn_r[...]
        pe_r[...] = jnp.dot(pos_r[...], wp_r[...], preferred_element_type=jnp.float32) + bp_r[...]

    return pl.pallas_call(
        body,
        grid=(grid,),
        in_specs=[
            pl.BlockSpec((NBLK, F), lambda i: (i, 0)),
            pl.BlockSpec((NBLK, 16), lambda i: (i, 0)),
            _full((F, F)), _full((1, F)), _full((1, F)), _full((1, F)),
            _full((16, F)), _full((1, F)),
        ],
        out_specs=[
            pl.BlockSpec((NBLK, F), lambda i: (i, 0)),
            pl.BlockSpec((NBLK, F), lambda i: (i, 0)),
        ],
        out_shape=[
            jax.ShapeDtypeStruct((n, F), jnp.float32),
            jax.ShapeDtypeStruct((n, F), jnp.float32),
        ],
    )(nf, pos, wn, bn_, gn, bbn, wp, bp)


def _enc_edge(ef, we, be, ge, bbe):
    Ep = ef.shape[0]
    grid = Ep // EBLK

    def body(ef_r, we_r, be_r, ge_r, bbe_r, e_r):
        x = jnp.dot(ef_r[...], we_r[...], preferred_element_type=jnp.float32) + be_r[...]
        m = jnp.mean(x, axis=-1, keepdims=True)
        v = jnp.mean((x - m) ** 2, axis=-1, keepdims=True)
        e_r[...] = (x - m) / jnp.sqrt(v + BN_EPS) * ge_r[...] + bbe_r[...]

    return pl.pallas_call(
        body,
        grid=(grid,),
        in_specs=[
            pl.BlockSpec((EBLK, 16), lambda i: (i, 0)),
            _full((16, F)), _full((1, F)), _full((1, F)), _full((1, F)),
        ],
        out_specs=pl.BlockSpec((EBLK, F), lambda i: (i, 0)),
        out_shape=jax.ShapeDtypeStruct((Ep, F), jnp.float32),
    )(ef, we, be, ge, bbe)


def _node_matmuls(h, pe, lp):
    n = h.shape[0]
    grid = n // NBLK
    ws = [lp['B1']['W'], lp['B2']['W'],
          lp['A1']['W'][:F], lp['A1']['W'][F:],
          lp['A2']['W'][:F], lp['A2']['W'][F:],
          lp['C1']['W'], lp['C2']['W']]
    bs = [lp['B1']['b'].reshape(1, F), lp['B2']['b'].reshape(1, F),
          lp['A1']['b'].reshape(1, F), lp['A2']['b'].reshape(1, F),
          lp['C1']['b'].reshape(1, F), lp['C2']['b'].reshape(1, F)]

    def body(h_r, pe_r, b1w, b2w, a1h, a1p, a2h, a2p, c1w, c2w,
             b1b, b2b, a1b, a2b, c1b, c2b,
             bh1_r, bh2_r, a1_r, vv_r, c1_r, c2p_r):
        hb = h_r[...]
        pb = pe_r[...]
        dot = lambda a, b: jnp.dot(a, b, preferred_element_type=jnp.float32)
        bh1_r[...] = dot(hb, b1w[...]) + b1b[...]
        bh2_r[...] = dot(hb, b2w[...]) + b2b[...]
        a1_r[...] = dot(hb, a1h[...]) + dot(pb, a1p[...]) + a1b[...]
        vv_r[...] = dot(hb, a2h[...]) + dot(pb, a2p[...]) + a2b[...]
        c1_r[...] = dot(pb, c1w[...]) + c1b[...]
        c2p_r[...] = dot(pb, c2w[...]) + c2b[...]

    blk = pl.BlockSpec((NBLK, F), lambda i: (i, 0))
    return pl.pallas_call(
        body,
        grid=(grid,),
        in_specs=[blk, blk] + [_full((F, F))] * 8 + [_full((1, F))] * 6,
        out_specs=[blk] * 6,
        out_shape=[jax.ShapeDtypeStruct((n, F), jnp.float32)] * 6,
    )(h, pe, *ws, *bs)


def _edge_b3(e, w3, b3):
    Ep = e.shape[0]
    grid = Ep // EBLK

    def body(e_r, w_r, b_r, o_r):
        o_r[...] = jnp.dot(e_r[...], w_r[...], preferred_element_type=jnp.float32) + b_r[...]

    blk = pl.BlockSpec((EBLK, F), lambda i: (i, 0))
    return pl.pallas_call(
        body,
        grid=(grid,),
        in_specs=[blk, _full((F, F)), _full((1, F))],
        out_specs=blk,
        out_shape=jax.ShapeDtypeStruct((Ep, F), jnp.float32),
    )(e, w3, b3)


def _edge_stats(hat, n_valid):
    Ep = hat.shape[0]
    grid = Ep // EBLK

    def body(hat_r, o_r):
        i = pl.program_id(0)
        rows = i * EBLK + lax.broadcasted_iota(jnp.int32, (EBLK, F), 0)
        x = jnp.where(rows < n_valid, hat_r[...], 0.0)
        s = jnp.sum(x, axis=0, keepdims=True)
        sq = jnp.sum(x * x, axis=0, keepdims=True)
        part = jnp.concatenate([s, sq], axis=0)

        @pl.when(i == 0)
        def _():
            o_r[...] = jnp.zeros_like(o_r)

        o_r[...] += part

    return pl.pallas_call(
        body,
        grid=(grid,),
        in_specs=[pl.BlockSpec((EBLK, F), lambda i: (i, 0))],
        out_specs=pl.BlockSpec((2, F), lambda i: (0, 0)),
        out_shape=jax.ShapeDtypeStruct((2, F), jnp.float32),
    )(hat)


def _edge_update(hat, e_prev, e_i, stats, g, b, w3, b3, n_valid):
    Ep = hat.shape[0]
    grid = Ep // EBLK

    def body(hat_r, ep_r, ei_r, st_r, g_r, b_r, w_r, b3_r, en_r, o_r):
        s = st_r[0, :]
        sq = st_r[1, :]
        m = s / n_valid
        var = sq / n_valid - m * m
        scale = g_r[0, :] / jnp.sqrt(var + BN_EPS)
        shift = b_r[0, :] - m * scale
        en = ep_r[...] + jnp.maximum(hat_r[...] * scale + shift, 0.0) + ei_r[...]
        en_r[...] = en
        o_r[...] = jnp.dot(en, w_r[...], preferred_element_type=jnp.float32) + b3_r[...]

    blk = pl.BlockSpec((EBLK, F), lambda i: (i, 0))
    return pl.pallas_call(
        body,
        grid=(grid,),
        in_specs=[blk, blk, blk, _full((2, F)), _full((1, F)), _full((1, F)),
                  _full((F, F)), _full((1, F))],
        out_specs=[blk, blk],
        out_shape=[jax.ShapeDtypeStruct((Ep, F), jnp.float32),
                   jax.ShapeDtypeStruct((Ep, F), jnp.float32)],
    )(hat, e_prev, e_i, stats, g, b, w3, b3)


def _node_stats(a1, hagg):
    n = a1.shape[0]
    grid = n // NBLK

    def body(a_r, h_r, o_r):
        i = pl.program_id(0)
        x = a_r[...] + h_r[...]
        s = jnp.sum(x, axis=0, keepdims=True)
        sq = jnp.sum(x * x, axis=0, keepdims=True)
        part = jnp.concatenate([s, sq], axis=0)

        @pl.when(i == 0)
        def _():
            o_r[...] = jnp.zeros_like(o_r)

        o_r[...] += part

    blk = pl.BlockSpec((NBLK, F), lambda i: (i, 0))
    return pl.pallas_call(
        body,
        grid=(grid,),
        in_specs=[blk, blk],
        out_specs=pl.BlockSpec((2, F), lambda i: (0, 0)),
        out_shape=jax.ShapeDtypeStruct((2, F), jnp.float32),
    )(a1, hagg)


def _node_update(a1, hagg, stats, g, b, h_prev, h_i,
                 c1, pagg, p_prev, p_i, n_valid):
    n = a1.shape[0]
    grid = n // NBLK

    def body(a_r, ha_r, st_r, g_r, b_r, hp_r, hi_r, c_r, pa_r, pp_r, pi_r,
             hn_r, pn_r):
        s = st_r[0, :]
        sq = st_r[1, :]
        m = s / n_valid
        var = sq / n_valid - m * m
        scale = g_r[0, :] / jnp.sqrt(var + BN_EPS)
        shift = b_r[0, :] - m * scale
        x = a_r[...] + ha_r[...]
        hn_r[...] = hp_r[...] + jnp.maximum(x * scale + shift, 0.0) + hi_r[...]
        pn_r[...] = pp_r[...] + jnp.tanh(c_r[...] + pa_r[...]) + pi_r[...]

    blk = pl.BlockSpec((NBLK, F), lambda i: (i, 0))
    return pl.pallas_call(
        body,
        grid=(grid,),
        in_specs=[blk, blk, _full((2, F)), _full((1, F)), _full((1, F)),
                  blk, blk, blk, blk, blk, blk],
        out_specs=[blk, blk],
        out_shape=[jax.ShapeDtypeStruct((n, F), jnp.float32),
                   jax.ShapeDtypeStruct((n, F), jnp.float32)],
    )(a1, hagg, stats, g, b, h_prev, h_i, c1, pagg, p_prev, p_i)


def _pool_mlp(h, gid_row, params, n_graphs):
    n = h.shape[0]

    def body(h_r, gid_r, w1, b1, g1, bb1, w2, b2, g2, bb2, w3, b3, o_r):
        gids = gid_r[...]
        onehot = (lax.broadcasted_iota(jnp.int32, (n_graphs, n), 0)
                  == gids).astype(jnp.float32)
        hg = jnp.dot(onehot, h_r[...], preferred_element_type=jnp.float32)

        def bn_elu(x, gg, bb):
            m = jnp.mean(x, axis=0, keepdims=True)
            var = jnp.mean((x - m) ** 2, axis=0, keepdims=True)
            y = (x - m) / jnp.sqrt(var + BN_EPS) * gg + bb
            return jnp.where(y > 0, y, jnp.exp(jnp.minimum(y, 0.0)) - 1.0)

        x = bn_elu(jnp.dot(hg, w1[...], preferred_element_type=jnp.float32) + b1[...],
                   g1[...], bb1[...])
        x = bn_elu(jnp.dot(x, w2[...], preferred_element_type=jnp.float32) + b2[...],
                   g2[...], bb2[...])
        o_r[...] = jnp.dot(x, w3[...], preferred_element_type=jnp.float32) + b3[...]

    mp = params
    return pl.pallas_call(
        body,
        in_specs=[
            pl.BlockSpec((n, F), lambda: (0, 0)),
            pl.BlockSpec((1, n), lambda: (0, 0)),
            _full2((F, F)), _full2((1, F)), _full2((1, F)), _full2((1, F)),
            _full2((F, 32)), _full2((1, 32)), _full2((1, 32)), _full2((1, 32)),
            _full2((32, 1)), _full2((1, 1)),
        ],
        out_specs=pl.BlockSpec((n_graphs, 1), lambda: (0, 0)),
        out_shape=jax.ShapeDtypeStruct((n_graphs, 1), jnp.float32),
    )(h, gid_row,
      mp['mlp_l1']['W'], mp['mlp_l1']['b'].reshape(1, F),
      mp['mlp_bn1']['g'].reshape(1, F), mp['mlp_bn1']['b'].reshape(1, F),
      mp['mlp_l2']['W'], mp['mlp_l2']['b'].reshape(1, 32),
      mp['mlp_bn2']['g'].reshape(1, 32), mp['mlp_bn2']['b'].reshape(1, 32),
      mp['mlp_l3']['W'], mp['mlp_l3']['b'].reshape(1, 1))


def _full2(shape):
    return pl.BlockSpec(shape, lambda: (0, 0))


def kernel(node_feats, edge_feats, pos_enc, fp, edge_index, graph_ids, params):
    n = node_feats.shape[0]
    e_cnt = edge_index.shape[1]
    n_graphs = fp.shape[0]

    src, dst = edge_index[0], edge_index[1]
    perm = jnp.argsort(dst)
    dst_s = dst[perm]
    src_s = src[perm]
    ef_s = edge_feats[perm]

    n_ranges = -(-n // NPT)
    Ep = e_cnt + K
    Ep = -(-Ep // EBLK) * EBLK
    esz = (-(-(n_ranges + 17) // 16)) * 16

    npad = n_ranges * NPT
    src_p = jnp.zeros((Ep,), jnp.int32).at[:e_cnt].set(src_s)
    dst_p = jnp.full((Ep,), n - 1, jnp.int32).at[:e_cnt].set(dst_s)
    counts = jnp.zeros((npad,), jnp.int32).at[dst].add(1, mode='drop')
    nst = jnp.concatenate([jnp.zeros((1,), jnp.int32),
                           jnp.cumsum(counts, dtype=jnp.int32)])
    estart = nst[jnp.arange(n_ranges + 1, dtype=jnp.int32) * NPT]
    estart_p = jnp.full((esz,), e_cnt, jnp.int32).at[:n_ranges + 1].set(estart)
    nst_p = jnp.full((npad + 32,), e_cnt, jnp.int32).at[:npad + 1].set(nst)
    ef_p = jnp.zeros((Ep, 16), jnp.float32).at[:e_cnt].set(ef_s)

    p = params
    h, pe = _enc_node(
        node_feats, pos_enc,
        p['enc_node']['W'], p['enc_node']['b'].reshape(1, F),
        p['ln_node']['g'].reshape(1, F), p['ln_node']['b'].reshape(1, F),
        p['enc_pose']['W'], p['enc_pose']['b'].reshape(1, F))
    e = _enc_edge(
        ef_p,
        p['enc_edge']['W'], p['enc_edge']['b'].reshape(1, F),
        p['ln_edge']['g'].reshape(1, F), p['ln_edge']['b'].reshape(1, F))

    h_i, e_i, p_i = h, e, pe
    n_layers = len(p['layers'])
    b3e = _edge_b3(e, p['layers'][0]['B3']['W'],
                   p['layers'][0]['B3']['b'].reshape(1, F))
    e_prev = e

    for li, lp in enumerate(p['layers']):
        bh1, bh2, a1, vv, c1, c2p = _node_matmuls(h, pe, lp)
        hat, hagg, pagg = _sc_layer(bh1, bh2, vv, c2p, b3e,
                                    src_p, dst_p, estart_p, nst_p, n_ranges)
        if li + 1 < n_layers:
            stats_e = _edge_stats(hat, e_cnt)
            nlp = p['layers'][li + 1]
            e_prev, b3e = _edge_update(
                hat, e_prev, e_i, stats_e,
                lp['bn_e']['g'].reshape(1, F), lp['bn_e']['b'].reshape(1, F),
                nlp['B3']['W'], nlp['B3']['b'].reshape(1, F), e_cnt)
        stats_h = _node_stats(a1, hagg)
        h, pe = _node_update(
            a1, hagg, stats_h,
            lp['bn_h']['g'].reshape(1, F), lp['bn_h']['b'].reshape(1, F),
            h, h_i, c1, pagg, pe, p_i, n)

    gid_row = graph_ids.reshape(1, n).astype(jnp.int32)
    return _pool_mlp(h, gid_row, p, n_graphs)


# speedup vs baseline: 4.8472x; 1.0397x over previous
import functools

import jax
import jax.numpy as jnp
from jax import lax
from jax.experimental import pallas as pl
from jax.experimental.pallas import tpu as pltpu
from jax.experimental.pallas import tpu_sc as plsc

F = 128
NPT = 64
K = 64
NW = 32
EBLK = 1152
NBLK = 2000
BN_EPS = 1e-5
ETA_EPS = 1e-6


def _sigmoid16(x):
    return 1.0 / (1.0 + jnp.exp(-x))


def _sc_layer(bh1, bh2, vv, c2p, b3e, src_p, dst_p, estart_p, nst_p,
                 n_ranges):
    Ep = b3e.shape[0]
    Npad = n_ranges * NPT
    ESZ = estart_p.shape[0]
    mesh = plsc.VectorSubcoreMesh(core_axis_name="c", subcore_axis_name="s",
                                  num_cores=2, num_subcores=16)

    @functools.partial(
        pl.kernel,
        out_type=(
            jax.ShapeDtypeStruct((Ep, F), jnp.float32),
            jax.ShapeDtypeStruct((Npad, F), jnp.float32),
            jax.ShapeDtypeStruct((Npad, F), jnp.float32),
        ),
        mesh=mesh,
        scratch_types=[
            pltpu.VMEM((K, F), jnp.float32),
            pltpu.VMEM((K, F), jnp.float32),
            pltpu.VMEM((K, F), jnp.float32),
            pltpu.VMEM((K, F), jnp.float32),
            pltpu.VMEM((K, F), jnp.float32),
            pltpu.VMEM((K, F), jnp.float32),
            pltpu.VMEM((K, F), jnp.float32),
            pltpu.VMEM((K, F), jnp.float32),
            pltpu.VMEM((K, F), jnp.float32),
            pltpu.VMEM((K, F), jnp.float32),
            pltpu.VMEM((NPT, F), jnp.float32),
            pltpu.VMEM((NPT, F), jnp.float32),
            pltpu.VMEM((NPT, F), jnp.float32),
            pltpu.VMEM((K,), jnp.int32),
            pltpu.VMEM((K,), jnp.int32),
            pltpu.VMEM((K,), jnp.int32),
            pltpu.VMEM((K,), jnp.int32),
            pltpu.VMEM((K + 16,), jnp.int32),
            pltpu.VMEM((K + 16,), jnp.int32),
            pltpu.VMEM((ESZ,), jnp.int32),
            pltpu.VMEM((NPT + 32,), jnp.int32),
            pltpu.SemaphoreType.DMA,
            pltpu.SemaphoreType.DMA,
            pltpu.SemaphoreType.DMA,
            pltpu.SemaphoreType.DMA,
            pltpu.SemaphoreType.DMA,
        ],
    )
    def k(bh1_h, bh2_h, vv_h, c2p_h, b3e_h, src_h, dst_h, est_h, nst_h,
          hat_h, hagg_h, pagg_h,
          A0, A1, B0, B1, C0, C1, D0, D1, E0, E1, ss, hacc, pacc,
          sidx0, sidx1, didx0, didx1, dsm0, dsm1, estv, nstv,
          isem0, isem1, dsem0, dsem1, wsem):
        w = lax.axis_index("s") * 2 + lax.axis_index("c")
        pltpu.sync_copy(est_h, estv)

        A_ = (A0, A1)
        B_ = (B0, B1)
        C_ = (C0, C1)
        D_ = (D0, D1)
        E_ = (E0, E1)
        SI = (sidx0, sidx1)
        DI = (didx0, didx1)
        DS = (dsm0, dsm1)
        IS = (isem0, isem1)
        DSEM = (dsem0, dsem1)

        n_mine = (n_ranges - 1 - w) // NW + 1

        def sval(ref, i):
            return ref[pl.ds(i, 16)][0]

        def per_range(ri, _):
            r = w + ri * NW
            base = r * NPT
            e0 = sval(estv, r)
            e1 = sval(estv, r + 1)
            c0 = (e0 // 8) * 8
            nch = jnp.maximum(e1 - c0 + K - 1, 0) // K
            pltpu.sync_copy(nst_h.at[pl.ds(base, NPT + 32)], nstv)

            def zero_rows(nl, _):
                zz = jnp.zeros((16,), jnp.float32)
                for j in range(F // 16):
                    sl = pl.ds(j * 16, 16)
                    ss[nl, sl] = zz
                    hacc[nl, sl] = zz
                    pacc[nl, sl] = zz
                return 0

            lax.fori_loop(0, NPT, zero_rows, 0)

            def issue_idx(ci, b):
                c = c0 + ci * K
                pltpu.async_copy(src_h.at[pl.ds(c, K)], SI[b], IS[b])
                pltpu.async_copy(dst_h.at[pl.ds(c, K)], DI[b], IS[b])
                pltpu.async_copy(dst_h.at[pl.ds(c, K)],
                                 DS[b].at[pl.ds(0, K)], IS[b])

            def wait_idx(b):
                pltpu.make_async_copy(src_h.at[pl.ds(0, K)], SI[b],
                                      IS[b]).wait()
                pltpu.make_async_copy(dst_h.at[pl.ds(0, K)], DI[b],
                                      IS[b]).wait()
                pltpu.make_async_copy(dst_h.at[pl.ds(0, K)],
                                      DS[b].at[pl.ds(0, K)], IS[b]).wait()

            def issue_data(ci, b):
                c = c0 + ci * K
                pltpu.async_copy(b3e_h.at[pl.ds(c, K)], A_[b], DSEM[b])
                pltpu.async_copy(bh1_h.at[DI[b]], B_[b], DSEM[b])
                pltpu.async_copy(bh2_h.at[SI[b]], C_[b], DSEM[b])
                pltpu.async_copy(vv_h.at[SI[b]], D_[b], DSEM[b])
                pltpu.async_copy(c2p_h.at[SI[b]], E_[b], DSEM[b])

            def wait_data(b):
                for buf in (A_[b], B_[b], C_[b], D_[b], E_[b]):
                    pltpu.make_async_copy(b3e_h.at[pl.ds(0, K)], buf,
                                          DSEM[b]).wait()

            def prefetch(cur, nb):
                nxt = cur + 1

                @pl.when(nxt < nch)
                def _():
                    issue_idx(nxt, nb)
                    wait_idx(nb)
                    issue_data(nxt, nb)

            def compute(cur, b):
                c = c0 + cur * K

                def bulk(el, _):
                    for j in range(F // 16):
                        sl = pl.ds(j * 16, 16)
                        A_[b][el, sl] = (A_[b][el, sl] + B_[b][el, sl]
                                         + C_[b][el, sl])
                    return 0

                lax.fori_loop(0, K, bulk, 0)
                pltpu.async_copy(A_[b], hat_h.at[pl.ds(c, K)], wsem)

                nlo = jnp.clip(sval(DS[b], 0) - base, 0, NPT - 1)
                nhi = jnp.clip(DS[b][pl.ds(K - 16, 16)][15] - base,
                               0, NPT - 1) + 1

                def per_node(nl, _):
                    es = jnp.maximum(sval(nstv, nl), c)
                    ee = jnp.minimum(sval(nstv, nl + 1), c + K)

                    def edge_body(e, accs):
                        el = e - c
                        out = []
                        for j in range(F // 16):
                            sl = pl.ds(j * 16, 16)
                            sig = _sigmoid16(A_[b][el, sl])
                            out.append(accs[j] + sig)
                            out.append(accs[j + 8] + sig * D_[b][el, sl])
                            out.append(accs[j + 16] + sig * E_[b][el, sl])
                        return (tuple(out[0::3]) + tuple(out[1::3])
                                + tuple(out[2::3]))

                    accs = lax.fori_loop(
                        es, ee, edge_body,
                        tuple(jnp.zeros((16,), jnp.float32)
                              for _ in range(3 * (F // 16))))
                    for j in range(F // 16):
                        sl = pl.ds(j * 16, 16)
                        ss[nl, sl] = ss[nl, sl] + accs[j]
                        hacc[nl, sl] = hacc[nl, sl] + accs[j + 8]
                        pacc[nl, sl] = pacc[nl, sl] + accs[j + 16]
                    return 0

                lax.fori_loop(nlo, nhi, per_node, 0)
                pltpu.make_async_copy(A_[b], hat_h.at[pl.ds(c, K)],
                                      wsem).wait()

            @pl.when(nch > 0)
            def _():
                issue_idx(0, 0)
                wait_idx(0)
                issue_data(0, 0)

                def step(ci, _):
                    for b in range(2):
                        def mk(cur_b):
                            def inner():
                                cur = 2 * ci + cur_b
                                prefetch(cur, (cur_b + 1) % 2)
                                wait_data(cur_b)
                                compute(cur, cur_b)
                            return inner

                        pl.when(2 * ci + b < nch)(mk(b))
                    return 0

                lax.fori_loop(0, (nch + 1) // 2, step, 0)

            def fin(nl, _):
                for j in range(F // 16):
                    sl = pl.ds(j * 16, 16)
                    inv = 1.0 / (ss[nl, sl] + ETA_EPS)
                    hacc[nl, sl] = hacc[nl, sl] * inv
                    pacc[nl, sl] = pacc[nl, sl] * inv
                return 0

            lax.fori_loop(0, NPT, fin, 0)

            pltpu.sync_copy(hacc, hagg_h.at[pl.ds(base, NPT)])
            pltpu.sync_copy(pacc, pagg_h.at[pl.ds(base, NPT)])
            return 0

        lax.fori_loop(0, n_mine, per_range, 0)

    return k(bh1, bh2, vv, c2p, b3e, src_p, dst_p, estart_p, nst_p)


def _full(shape):
    return pl.BlockSpec(shape, lambda i: (0, 0))


def _enc_node(nf, pos, wn, bn_, gn, bbn, wp, bp):
    n = nf.shape[0]
    grid = n // NBLK

    def body(nf_r, pos_r, wn_r, bn_r, gn_r, bbn_r, wp_r, bp_r, h_r, pe_r):
        x = jnp.dot(nf_r[...], wn_r[...], preferred_element_type=jnp.float32) + bn_r[...]
        m = jnp.mean(x, axis=-1, keepdims=True)
        v = jnp.mean((x - m) ** 2, axis=-1, keepdims=True)
        h_r[...] = (x - m) / jnp.sqrt(v + BN_EPS) * gn_r[...] + bbn_r[...]
        pe_r[...] = jnp.dot(pos_r[...], wp_r[...], preferred_element_type=jnp.float32) + bp_r[...]

    return pl.pallas_call(
        body,
        grid=(grid,),
        in_specs=[
            pl.BlockSpec((NBLK, F), lambda i: (i, 0)),
            pl.BlockSpec((NBLK, 16), lambda i: (i, 0)),
            _full((F, F)), _full((1, F)), _full((1, F)), _full((1, F)),
            _full((16, F)), _full((1, F)),
        ],
        out_specs=[
            pl.BlockSpec((NBLK, F), lambda i: (i, 0)),
            pl.BlockSpec((NBLK, F), lambda i: (i, 0)),
        ],
        out_shape=[
            jax.ShapeDtypeStruct((n, F), jnp.float32),
            jax.ShapeDtypeStruct((n, F), jnp.float32),
        ],
    )(nf, pos, wn, bn_, gn, bbn, wp, bp)


def _enc_edge(ef, we, be, ge, bbe):
    Ep = ef.shape[0]
    grid = Ep // EBLK

    def body(ef_r, we_r, be_r, ge_r, bbe_r, e_r):
        x = jnp.dot(ef_r[...], we_r[...], preferred_element_type=jnp.float32) + be_r[...]
        m = jnp.mean(x, axis=-1, keepdims=True)
        v = jnp.mean((x - m) ** 2, axis=-1, keepdims=True)
        e_r[...] = (x - m) / jnp.sqrt(v + BN_EPS) * ge_r[...] + bbe_r[...]

    return pl.pallas_call(
        body,
        grid=(grid,),
        in_specs=[
            pl.BlockSpec((EBLK, 16), lambda i: (i, 0)),
            _full((16, F)), _full((1, F)), _full((1, F)), _full((1, F)),
        ],
        out_specs=pl.BlockSpec((EBLK, F), lambda i: (i, 0)),
        out_shape=jax.ShapeDtypeStruct((Ep, F), jnp.float32),
    )(ef, we, be, ge, bbe)


def _node_matmuls(h, pe, lp):
    n = h.shape[0]
    grid = n // NBLK
    ws = [lp['B1']['W'], lp['B2']['W'],
          lp['A1']['W'][:F], lp['A1']['W'][F:],
          lp['A2']['W'][:F], lp['A2']['W'][F:],
          lp['C1']['W'], lp['C2']['W']]
    bs = [lp['B1']['b'].reshape(1, F), lp['B2']['b'].reshape(1, F),
          lp['A1']['b'].reshape(1, F), lp['A2']['b'].reshape(1, F),
          lp['C1']['b'].reshape(1, F), lp['C2']['b'].reshape(1, F)]

    def body(h_r, pe_r, b1w, b2w, a1h, a1p, a2h, a2p, c1w, c2w,
             b1b, b2b, a1b, a2b, c1b, c2b,
             bh1_r, bh2_r, a1_r, vv_r, c1_r, c2p_r):
        hb = h_r[...]
        pb = pe_r[...]
        dot = lambda a, b: jnp.dot(a, b, preferred_element_type=jnp.float32)
        bh1_r[...] = dot(hb, b1w[...]) + b1b[...]
        bh2_r[...] = dot(hb, b2w[...]) + b2b[...]
        a1_r[...] = dot(hb, a1h[...]) + dot(pb, a1p[...]) + a1b[...]
        vv_r[...] = dot(hb, a2h[...]) + dot(pb, a2p[...]) + a2b[...]
        c1_r[...] = dot(pb, c1w[...]) + c1b[...]
        c2p_r[...] = dot(pb, c2w[...]) + c2b[...]

    blk = pl.BlockSpec((NBLK, F), lambda i: (i, 0))
    return pl.pallas_call(
        body,
        grid=(grid,),
        in_specs=[blk, blk] + [_full((F, F))] * 8 + [_full((1, F))] * 6,
        out_specs=[blk] * 6,
        out_shape=[jax.ShapeDtypeStruct((n, F), jnp.float32)] * 6,
    )(h, pe, *ws, *bs)


def _node_update_mm(a1, hagg, stats, g, b, h_prev, h_i,
                    c1, pagg, p_prev, p_i, n_valid, nlp):
    n = a1.shape[0]
    grid = n // NBLK
    ws = [nlp['B1']['W'], nlp['B2']['W'],
          nlp['A1']['W'][:F], nlp['A1']['W'][F:],
          nlp['A2']['W'][:F], nlp['A2']['W'][F:],
          nlp['C1']['W'], nlp['C2']['W']]
    bs = [nlp['B1']['b'].reshape(1, F), nlp['B2']['b'].reshape(1, F),
          nlp['A1']['b'].reshape(1, F), nlp['A2']['b'].reshape(1, F),
          nlp['C1']['b'].reshape(1, F), nlp['C2']['b'].reshape(1, F)]

    def body(a_r, ha_r, st_r, g_r, b_r, hp_r, hi_r, c_r, pa_r, pp_r, pi_r,
             b1w, b2w, a1h, a1p, a2h, a2p, c1w, c2w,
             b1b, b2b, a1b, a2b, c1b, c2b,
             hn_r, pn_r, bh1_r, bh2_r, na1_r, vv_r, nc1_r, c2p_r):
        s = st_r[0, :]
        sq = st_r[1, :]
        m = s / n_valid
        var = sq / n_valid - m * m
        scale = g_r[0, :] / jnp.sqrt(var + BN_EPS)
        shift = b_r[0, :] - m * scale
        x = a_r[...] + ha_r[...]
        hb = hp_r[...] + jnp.maximum(x * scale + shift, 0.0) + hi_r[...]
        pb = pp_r[...] + jnp.tanh(c_r[...] + pa_r[...]) + pi_r[...]
        hn_r[...] = hb
        pn_r[...] = pb
        dot = lambda u, v_: jnp.dot(u, v_, preferred_element_type=jnp.float32)
        bh1_r[...] = dot(hb, b1w[...]) + b1b[...]
        bh2_r[...] = dot(hb, b2w[...]) + b2b[...]
        na1_r[...] = dot(hb, a1h[...]) + dot(pb, a1p[...]) + a1b[...]
        vv_r[...] = dot(hb, a2h[...]) + dot(pb, a2p[...]) + a2b[...]
        nc1_r[...] = dot(pb, c1w[...]) + c1b[...]
        c2p_r[...] = dot(pb, c2w[...]) + c2b[...]

    blk = pl.BlockSpec((NBLK, F), lambda i: (i, 0))
    return pl.pallas_call(
        body,
        grid=(grid,),
        in_specs=([blk, blk, _full((2, F)), _full((1, F)), _full((1, F)),
                   blk, blk, blk, blk, blk, blk]
                  + [_full((F, F))] * 8 + [_full((1, F))] * 6),
        out_specs=[blk] * 8,
        out_shape=[jax.ShapeDtypeStruct((n, F), jnp.float32)] * 8,
    )(a1, hagg, stats, g, b, h_prev, h_i, c1, pagg, p_prev, p_i, *ws, *bs)


def _edge_b3(e, w3, b3):
    Ep = e.shape[0]
    grid = Ep // EBLK

    def body(e_r, w_r, b_r, o_r):
        o_r[...] = jnp.dot(e_r[...], w_r[...], preferred_element_type=jnp.float32) + b_r[...]

    blk = pl.BlockSpec((EBLK, F), lambda i: (i, 0))
    return pl.pallas_call(
        body,
        grid=(grid,),
        in_specs=[blk, _full((F, F)), _full((1, F))],
        out_specs=blk,
        out_shape=jax.ShapeDtypeStruct((Ep, F), jnp.float32),
    )(e, w3, b3)


def _edge_stats(hat, n_valid):
    Ep = hat.shape[0]
    grid = Ep // EBLK

    def body(hat_r, o_r):
        i = pl.program_id(0)
        rows = i * EBLK + lax.broadcasted_iota(jnp.int32, (EBLK, F), 0)
        x = jnp.where(rows < n_valid, hat_r[...], 0.0)
        s = jnp.sum(x, axis=0, keepdims=True)
        sq = jnp.sum(x * x, axis=0, keepdims=True)
        part = jnp.concatenate([s, sq], axis=0)

        @pl.when(i == 0)
        def _():
            o_r[...] = jnp.zeros_like(o_r)

        o_r[...] += part

    return pl.pallas_call(
        body,
        grid=(grid,),
        in_specs=[pl.BlockSpec((EBLK, F), lambda i: (i, 0))],
        out_specs=pl.BlockSpec((2, F), lambda i: (0, 0)),
        out_shape=jax.ShapeDtypeStruct((2, F), jnp.float32),
    )(hat)


def _edge_update(hat, e_prev, e_i, stats, g, b, w3, b3, n_valid):
    Ep = hat.shape[0]
    grid = Ep // EBLK

    def body(hat_r, ep_r, ei_r, st_r, g_r, b_r, w_r, b3_r, en_r, o_r):
        s = st_r[0, :]
        sq = st_r[1, :]
        m = s / n_valid
        var = sq / n_valid - m * m
        scale = g_r[0, :] / jnp.sqrt(var + BN_EPS)
        shift = b_r[0, :] - m * scale
        en = ep_r[...] + jnp.maximum(hat_r[...] * scale + shift, 0.0) + ei_r[...]
        en_r[...] = en
        o_r[...] = jnp.dot(en, w_r[...], preferred_element_type=jnp.float32) + b3_r[...]

    blk = pl.BlockSpec((EBLK, F), lambda i: (i, 0))
    return pl.pallas_call(
        body,
        grid=(grid,),
        in_specs=[blk, blk, blk, _full((2, F)), _full((1, F)), _full((1, F)),
                  _full((F, F)), _full((1, F))],
        out_specs=[blk, blk],
        out_shape=[jax.ShapeDtypeStruct((Ep, F), jnp.float32),
                   jax.ShapeDtypeStruct((Ep, F), jnp.float32)],
    )(hat, e_prev, e_i, stats, g, b, w3, b3)


def _node_stats(a1, hagg):
    n = a1.shape[0]
    grid = n // NBLK

    def body(a_r, h_r, o_r):
        i = pl.program_id(0)
        x = a_r[...] + h_r[...]
        s = jnp.sum(x, axis=0, keepdims=True)
        sq = jnp.sum(x * x, axis=0, keepdims=True)
        part = jnp.concatenate([s, sq], axis=0)

        @pl.when(i == 0)
        def _():
            o_r[...] = jnp.zeros_like(o_r)

        o_r[...] += part

    blk = pl.BlockSpec((NBLK, F), lambda i: (i, 0))
    return pl.pallas_call(
        body,
        grid=(grid,),
        in_specs=[blk, blk],
        out_specs=pl.BlockSpec((2, F), lambda i: (0, 0)),
        out_shape=jax.ShapeDtypeStruct((2, F), jnp.float32),
    )(a1, hagg)


def _node_update(a1, hagg, stats, g, b, h_prev, h_i,
                 c1, pagg, p_prev, p_i, n_valid):
    n = a1.shape[0]
    grid = n // NBLK

    def body(a_r, ha_r, st_r, g_r, b_r, hp_r, hi_r, c_r, pa_r, pp_r, pi_r,
             hn_r, pn_r):
        s = st_r[0, :]
        sq = st_r[1, :]
        m = s / n_valid
        var = sq / n_valid - m * m
        scale = g_r[0, :] / jnp.sqrt(var + BN_EPS)
        shift = b_r[0, :] - m * scale
        x = a_r[...] + ha_r[...]
        hn_r[...] = hp_r[...] + jnp.maximum(x * scale + shift, 0.0) + hi_r[...]
        pn_r[...] = pp_r[...] + jnp.tanh(c_r[...] + pa_r[...]) + pi_r[...]

    blk = pl.BlockSpec((NBLK, F), lambda i: (i, 0))
    return pl.pallas_call(
        body,
        grid=(grid,),
        in_specs=[blk, blk, _full((2, F)), _full((1, F)), _full((1, F)),
                  blk, blk, blk, blk, blk, blk],
        out_specs=[blk, blk],
        out_shape=[jax.ShapeDtypeStruct((n, F), jnp.float32),
                   jax.ShapeDtypeStruct((n, F), jnp.float32)],
    )(a1, hagg, stats, g, b, h_prev, h_i, c1, pagg, p_prev, p_i)


def _pool_mlp(h, gid_row, params, n_graphs):
    n = h.shape[0]

    def body(h_r, gid_r, w1, b1, g1, bb1, w2, b2, g2, bb2, w3, b3, o_r):
        gids = gid_r[...]
        onehot = (lax.broadcasted_iota(jnp.int32, (n_graphs, n), 0)
                  == gids).astype(jnp.float32)
        hg = jnp.dot(onehot, h_r[...], preferred_element_type=jnp.float32)

        def bn_elu(x, gg, bb):
            m = jnp.mean(x, axis=0, keepdims=True)
            var = jnp.mean((x - m) ** 2, axis=0, keepdims=True)
            y = (x - m) / jnp.sqrt(var + BN_EPS) * gg + bb
            return jnp.where(y > 0, y, jnp.exp(jnp.minimum(y, 0.0)) - 1.0)

        x = bn_elu(jnp.dot(hg, w1[...], preferred_element_type=jnp.float32) + b1[...],
                   g1[...], bb1[...])
        x = bn_elu(jnp.dot(x, w2[...], preferred_element_type=jnp.float32) + b2[...],
                   g2[...], bb2[...])
        o_r[...] = jnp.dot(x, w3[...], preferred_element_type=jnp.float32) + b3[...]

    mp = params
    return pl.pallas_call(
        body,
        in_specs=[
            pl.BlockSpec((n, F), lambda: (0, 0)),
            pl.BlockSpec((1, n), lambda: (0, 0)),
            _full2((F, F)), _full2((1, F)), _full2((1, F)), _full2((1, F)),
            _full2((F, 32)), _full2((1, 32)), _full2((1, 32)), _full2((1, 32)),
            _full2((32, 1)), _full2((1, 1)),
        ],
        out_specs=pl.BlockSpec((n_graphs, 1), lambda: (0, 0)),
        out_shape=jax.ShapeDtypeStruct((n_graphs, 1), jnp.float32),
    )(h, gid_row,
      mp['mlp_l1']['W'], mp['mlp_l1']['b'].reshape(1, F),
      mp['mlp_bn1']['g'].reshape(1, F), mp['mlp_bn1']['b'].reshape(1, F),
      mp['mlp_l2']['W'], mp['mlp_l2']['b'].reshape(1, 32),
      mp['mlp_bn2']['g'].reshape(1, 32), mp['mlp_bn2']['b'].reshape(1, 32),
      mp['mlp_l3']['W'], mp['mlp_l3']['b'].reshape(1, 1))


def _full2(shape):
    return pl.BlockSpec(shape, lambda: (0, 0))


def kernel(node_feats, edge_feats, pos_enc, fp, edge_index, graph_ids, params):
    n = node_feats.shape[0]
    e_cnt = edge_index.shape[1]
    n_graphs = fp.shape[0]

    src, dst = edge_index[0], edge_index[1]
    perm = jnp.argsort(dst)
    dst_s = dst[perm]
    src_s = src[perm]
    ef_s = edge_feats[perm]

    n_ranges = -(-n // NPT)
    Ep = e_cnt + K
    Ep = -(-Ep // EBLK) * EBLK
    esz = (-(-(n_ranges + 17) // 16)) * 16

    npad = n_ranges * NPT
    src_p = jnp.zeros((Ep,), jnp.int32).at[:e_cnt].set(src_s)
    dst_p = jnp.full((Ep,), n - 1, jnp.int32).at[:e_cnt].set(dst_s)
    counts = jnp.zeros((npad,), jnp.int32).at[dst].add(1, mode='drop')
    nst = jnp.concatenate([jnp.zeros((1,), jnp.int32),
                           jnp.cumsum(counts, dtype=jnp.int32)])
    estart = nst[jnp.arange(n_ranges + 1, dtype=jnp.int32) * NPT]
    estart_p = jnp.full((esz,), e_cnt, jnp.int32).at[:n_ranges + 1].set(estart)
    nst_p = jnp.full((npad + 32,), e_cnt, jnp.int32).at[:npad + 1].set(nst)
    ef_p = jnp.zeros((Ep, 16), jnp.float32).at[:e_cnt].set(ef_s)

    p = params
    h, pe = _enc_node(
        node_feats, pos_enc,
        p['enc_node']['W'], p['enc_node']['b'].reshape(1, F),
        p['ln_node']['g'].reshape(1, F), p['ln_node']['b'].reshape(1, F),
        p['enc_pose']['W'], p['enc_pose']['b'].reshape(1, F))
    e = _enc_edge(
        ef_p,
        p['enc_edge']['W'], p['enc_edge']['b'].reshape(1, F),
        p['ln_edge']['g'].reshape(1, F), p['ln_edge']['b'].reshape(1, F))

    h_i, e_i, p_i = h, e, pe
    n_layers = len(p['layers'])
    b3e = _edge_b3(e, p['layers'][0]['B3']['W'],
                   p['layers'][0]['B3']['b'].reshape(1, F))
    e_prev = e

    mm = _node_matmuls(h, pe, p['layers'][0])
    for li, lp in enumerate(p['layers']):
        bh1, bh2, a1, vv, c1, c2p = mm
        hat, hagg, pagg = _sc_layer(bh1, bh2, vv, c2p, b3e,
                                    src_p, dst_p, estart_p, nst_p, n_ranges)
        if li + 1 < n_layers:
            stats_e = _edge_stats(hat, e_cnt)
            nlp = p['layers'][li + 1]
            e_prev, b3e = _edge_update(
                hat, e_prev, e_i, stats_e,
                lp['bn_e']['g'].reshape(1, F), lp['bn_e']['b'].reshape(1, F),
                nlp['B3']['W'], nlp['B3']['b'].reshape(1, F), e_cnt)
        stats_h = _node_stats(a1, hagg)
        if li + 1 < n_layers:
            h, pe, *mm = _node_update_mm(
                a1, hagg, stats_h,
                lp['bn_h']['g'].reshape(1, F), lp['bn_h']['b'].reshape(1, F),
                h, h_i, c1, pagg, pe, p_i, n, p['layers'][li + 1])
        else:
            h, pe = _node_update(
                a1, hagg, stats_h,
                lp['bn_h']['g'].reshape(1, F), lp['bn_h']['b'].reshape(1, F),
                h, h_i, c1, pagg, pe, p_i, n)

    gid_row = graph_ids.reshape(1, n).astype(jnp.int32)
    return _pool_mlp(h, gid_row, p, n_graphs)


# speedup vs baseline: 4.8499x; 1.0006x over previous
import functools

import jax
import jax.numpy as jnp
from jax import lax
from jax.experimental import pallas as pl
from jax.experimental.pallas import tpu as pltpu
from jax.experimental.pallas import tpu_sc as plsc

F = 128
NPT = 64
K = 64
NW = 32
EBLK = 1152
NBLK = 2000
BN_EPS = 1e-5
ETA_EPS = 1e-6


def _sigmoid16(x):
    return 1.0 / (1.0 + jnp.exp(-x))


def _sc_layer(bh1, bh2, vv, c2p, b3e, src_p, dst_p, estart_p, nst_p,
                 n_ranges):
    Ep = b3e.shape[0]
    Npad = n_ranges * NPT
    ESZ = estart_p.shape[0]
    mesh = plsc.VectorSubcoreMesh(core_axis_name="c", subcore_axis_name="s",
                                  num_cores=2, num_subcores=16)

    @functools.partial(
        pl.kernel,
        out_type=(
            jax.ShapeDtypeStruct((Ep, F), jnp.float32),
            jax.ShapeDtypeStruct((Npad, F), jnp.float32),
            jax.ShapeDtypeStruct((Npad, F), jnp.float32),
        ),
        mesh=mesh,
        scratch_types=[
            pltpu.VMEM((K, F), jnp.float32),
            pltpu.VMEM((K, F), jnp.float32),
            pltpu.VMEM((K, F), jnp.float32),
            pltpu.VMEM((K, F), jnp.float32),
            pltpu.VMEM((K, F), jnp.float32),
            pltpu.VMEM((K, F), jnp.float32),
            pltpu.VMEM((K, F), jnp.float32),
            pltpu.VMEM((K, F), jnp.float32),
            pltpu.VMEM((K, F), jnp.float32),
            pltpu.VMEM((K, F), jnp.float32),
            pltpu.VMEM((NPT, F), jnp.float32),
            pltpu.VMEM((NPT, F), jnp.float32),
            pltpu.VMEM((NPT, F), jnp.float32),
            pltpu.VMEM((K,), jnp.int32),
            pltpu.VMEM((K,), jnp.int32),
            pltpu.VMEM((K,), jnp.int32),
            pltpu.VMEM((K,), jnp.int32),
            pltpu.VMEM((K + 16,), jnp.int32),
            pltpu.VMEM((K + 16,), jnp.int32),
            pltpu.VMEM((ESZ,), jnp.int32),
            pltpu.VMEM((NPT + 32,), jnp.int32),
            pltpu.SemaphoreType.DMA,
            pltpu.SemaphoreType.DMA,
            pltpu.SemaphoreType.DMA,
            pltpu.SemaphoreType.DMA,
            pltpu.SemaphoreType.DMA,
        ],
    )
    def k(bh1_h, bh2_h, vv_h, c2p_h, b3e_h, src_h, dst_h, est_h, nst_h,
          hat_h, hagg_h, pagg_h,
          A0, A1, B0, B1, C0, C1, D0, D1, E0, E1, ss, hacc, pacc,
          sidx0, sidx1, didx0, didx1, dsm0, dsm1, estv, nstv,
          isem0, isem1, dsem0, dsem1, wsem):
        w = lax.axis_index("s") * 2 + lax.axis_index("c")
        pltpu.sync_copy(est_h, estv)

        A_ = (A0, A1)
        B_ = (B0, B1)
        C_ = (C0, C1)
        D_ = (D0, D1)
        E_ = (E0, E1)
        SI = (sidx0, sidx1)
        DI = (didx0, didx1)
        DS = (dsm0, dsm1)
        IS = (isem0, isem1)
        DSEM = (dsem0, dsem1)

        n_mine = (n_ranges - 1 - w) // NW + 1

        def sval(ref, i):
            return ref[pl.ds(i, 16)][0]

        def per_range(ri, _):
            r = w + ri * NW
            base = r * NPT
            e0 = sval(estv, r)
            e1 = sval(estv, r + 1)
            c0 = (e0 // 8) * 8
            nch = jnp.maximum(e1 - c0 + K - 1, 0) // K
            pltpu.sync_copy(nst_h.at[pl.ds(base, NPT + 32)], nstv)

            def zero_rows(nl, _):
                zz = jnp.zeros((16,), jnp.float32)
                for j in range(F // 16):
                    sl = pl.ds(j * 16, 16)
                    ss[nl, sl] = zz
                    hacc[nl, sl] = zz
                    pacc[nl, sl] = zz
                return 0

            lax.fori_loop(0, NPT, zero_rows, 0)

            def issue_idx(ci, b):
                c = c0 + ci * K
                pltpu.async_copy(src_h.at[pl.ds(c, K)], SI[b], IS[b])
                pltpu.async_copy(dst_h.at[pl.ds(c, K)], DI[b], IS[b])
                pltpu.async_copy(dst_h.at[pl.ds(c, K)],
                                 DS[b].at[pl.ds(0, K)], IS[b])

            def wait_idx(b):
                pltpu.make_async_copy(src_h.at[pl.ds(0, K)], SI[b],
                                      IS[b]).wait()
                pltpu.make_async_copy(dst_h.at[pl.ds(0, K)], DI[b],
                                      IS[b]).wait()
                pltpu.make_async_copy(dst_h.at[pl.ds(0, K)],
                                      DS[b].at[pl.ds(0, K)], IS[b]).wait()

            def issue_data(ci, b):
                c = c0 + ci * K
                pltpu.async_copy(b3e_h.at[pl.ds(c, K)], A_[b], DSEM[b])
                pltpu.async_copy(bh1_h.at[DI[b]], B_[b], DSEM[b])
                pltpu.async_copy(bh2_h.at[SI[b]], C_[b], DSEM[b])
                pltpu.async_copy(vv_h.at[SI[b]], D_[b], DSEM[b])
                pltpu.async_copy(c2p_h.at[SI[b]], E_[b], DSEM[b])

            def wait_data(b):
                for buf in (A_[b], B_[b], C_[b], D_[b], E_[b]):
                    pltpu.make_async_copy(b3e_h.at[pl.ds(0, K)], buf,
                                          DSEM[b]).wait()

            def prefetch(cur, nb):
                nxt = cur + 1

                @pl.when(nxt < nch)
                def _():
                    issue_idx(nxt, nb)
                    wait_idx(nb)
                    issue_data(nxt, nb)

            def compute(cur, b):
                c = c0 + cur * K

                def bulk(el, _):
                    for j in range(F // 16):
                        sl = pl.ds(j * 16, 16)
                        A_[b][el, sl] = (A_[b][el, sl] + B_[b][el, sl]
                                         + C_[b][el, sl])
                    return 0

                lax.fori_loop(0, K, bulk, 0)
                pltpu.async_copy(A_[b], hat_h.at[pl.ds(c, K)], wsem)

                nlo = jnp.clip(sval(DS[b], 0) - base, 0, NPT - 1)
                nhi = jnp.clip(DS[b][pl.ds(K - 16, 16)][15] - base,
                               0, NPT - 1) + 1

                def per_node(nl, _):
                    es = jnp.maximum(sval(nstv, nl), c)
                    ee = jnp.minimum(sval(nstv, nl + 1), c + K)

                    def edge_body(e, accs):
                        el = e - c
                        out = []
                        for j in range(F // 16):
                            sl = pl.ds(j * 16, 16)
                            sig = _sigmoid16(A_[b][el, sl])
                            out.append(accs[j] + sig)
                            out.append(accs[j + 8] + sig * D_[b][el, sl])
                            out.append(accs[j + 16] + sig * E_[b][el, sl])
                        return (tuple(out[0::3]) + tuple(out[1::3])
                                + tuple(out[2::3]))

                    accs = lax.fori_loop(
                        es, ee, edge_body,
                        tuple(jnp.zeros((16,), jnp.float32)
                              for _ in range(3 * (F // 16))))
                    for j in range(F // 16):
                        sl = pl.ds(j * 16, 16)
                        ss[nl, sl] = ss[nl, sl] + accs[j]
                        hacc[nl, sl] = hacc[nl, sl] + accs[j + 8]
                        pacc[nl, sl] = pacc[nl, sl] + accs[j + 16]
                    return 0

                lax.fori_loop(nlo, nhi, per_node, 0)
                pltpu.make_async_copy(A_[b], hat_h.at[pl.ds(c, K)],
                                      wsem).wait()

            @pl.when(nch > 0)
            def _():
                issue_idx(0, 0)
                wait_idx(0)
                issue_data(0, 0)

                def step(ci, _):
                    for b in range(2):
                        def mk(cur_b):
                            def inner():
                                cur = 2 * ci + cur_b
                                prefetch(cur, (cur_b + 1) % 2)
                                wait_data(cur_b)
                                compute(cur, cur_b)
                            return inner

                        pl.when(2 * ci + b < nch)(mk(b))
                    return 0

                lax.fori_loop(0, (nch + 1) // 2, step, 0)

            def fin(nl, _):
                for j in range(F // 16):
                    sl = pl.ds(j * 16, 16)
                    inv = 1.0 / (ss[nl, sl] + ETA_EPS)
                    hacc[nl, sl] = hacc[nl, sl] * inv
                    pacc[nl, sl] = pacc[nl, sl] * inv
                return 0

            lax.fori_loop(0, NPT, fin, 0)

            pltpu.sync_copy(hacc, hagg_h.at[pl.ds(base, NPT)])
            pltpu.sync_copy(pacc, pagg_h.at[pl.ds(base, NPT)])
            return 0

        lax.fori_loop(0, n_mine, per_range, 0)

    return k(bh1, bh2, vv, c2p, b3e, src_p, dst_p, estart_p, nst_p)


def _full(shape):
    return pl.BlockSpec(shape, lambda i: (0, 0))


def _enc_node(nf, pos, wn, bn_, gn, bbn, wp, bp):
    n = nf.shape[0]
    grid = n // NBLK

    def body(nf_r, pos_r, wn_r, bn_r, gn_r, bbn_r, wp_r, bp_r, h_r, pe_r):
        x = jnp.dot(nf_r[...], wn_r[...], preferred_element_type=jnp.float32) + bn_r[...]
        m = jnp.mean(x, axis=-1, keepdims=True)
        v = jnp.mean((x - m) ** 2, axis=-1, keepdims=True)
        h_r[...] = (x - m) / jnp.sqrt(v + BN_EPS) * gn_r[...] + bbn_r[...]
        pe_r[...] = jnp.dot(pos_r[...], wp_r[...], preferred_element_type=jnp.float32) + bp_r[...]

    return pl.pallas_call(
        body,
        grid=(grid,),
        in_specs=[
            pl.BlockSpec((NBLK, F), lambda i: (i, 0)),
            pl.BlockSpec((NBLK, 16), lambda i: (i, 0)),
            _full((F, F)), _full((1, F)), _full((1, F)), _full((1, F)),
            _full((16, F)), _full((1, F)),
        ],
        out_specs=[
            pl.BlockSpec((NBLK, F), lambda i: (i, 0)),
            pl.BlockSpec((NBLK, F), lambda i: (i, 0)),
        ],
        out_shape=[
            jax.ShapeDtypeStruct((n, F), jnp.float32),
            jax.ShapeDtypeStruct((n, F), jnp.float32),
        ],
    )(nf, pos, wn, bn_, gn, bbn, wp, bp)


def _enc_edge(ef, we, be, ge, bbe, w3, b3):
    Ep = ef.shape[0]
    grid = Ep // EBLK

    def body(ef_r, we_r, be_r, ge_r, bbe_r, w3_r, b3_r, e_r, b3e_r):
        x = jnp.dot(ef_r[...], we_r[...], preferred_element_type=jnp.float32) + be_r[...]
        m = jnp.mean(x, axis=-1, keepdims=True)
        v = jnp.mean((x - m) ** 2, axis=-1, keepdims=True)
        e0 = (x - m) / jnp.sqrt(v + BN_EPS) * ge_r[...] + bbe_r[...]
        e_r[...] = e0
        b3e_r[...] = jnp.dot(e0, w3_r[...],
                             preferred_element_type=jnp.float32) + b3_r[...]

    blk = pl.BlockSpec((EBLK, F), lambda i: (i, 0))
    return pl.pallas_call(
        body,
        grid=(grid,),
        in_specs=[
            pl.BlockSpec((EBLK, 16), lambda i: (i, 0)),
            _full((16, F)), _full((1, F)), _full((1, F)), _full((1, F)),
            _full((F, F)), _full((1, F)),
        ],
        out_specs=[blk, blk],
        out_shape=[jax.ShapeDtypeStruct((Ep, F), jnp.float32),
                   jax.ShapeDtypeStruct((Ep, F), jnp.float32)],
    )(ef, we, be, ge, bbe, w3, b3)


def _node_matmuls(h, pe, lp):
    n = h.shape[0]
    grid = n // NBLK
    ws = [lp['B1']['W'], lp['B2']['W'],
          lp['A1']['W'][:F], lp['A1']['W'][F:],
          lp['A2']['W'][:F], lp['A2']['W'][F:],
          lp['C1']['W'], lp['C2']['W']]
    bs = [lp['B1']['b'].reshape(1, F), lp['B2']['b'].reshape(1, F),
          lp['A1']['b'].reshape(1, F), lp['A2']['b'].reshape(1, F),
          lp['C1']['b'].reshape(1, F), lp['C2']['b'].reshape(1, F)]

    def body(h_r, pe_r, b1w, b2w, a1h, a1p, a2h, a2p, c1w, c2w,
             b1b, b2b, a1b, a2b, c1b, c2b,
             bh1_r, bh2_r, a1_r, vv_r, c1_r, c2p_r):
        hb = h_r[...]
        pb = pe_r[...]
        dot = lambda a, b: jnp.dot(a, b, preferred_element_type=jnp.float32)
        bh1_r[...] = dot(hb, b1w[...]) + b1b[...]
        bh2_r[...] = dot(hb, b2w[...]) + b2b[...]
        a1_r[...] = dot(hb, a1h[...]) + dot(pb, a1p[...]) + a1b[...]
        vv_r[...] = dot(hb, a2h[...]) + dot(pb, a2p[...]) + a2b[...]
        c1_r[...] = dot(pb, c1w[...]) + c1b[...]
        c2p_r[...] = dot(pb, c2w[...]) + c2b[...]

    blk = pl.BlockSpec((NBLK, F), lambda i: (i, 0))
    return pl.pallas_call(
        body,
        grid=(grid,),
        in_specs=[blk, blk] + [_full((F, F))] * 8 + [_full((1, F))] * 6,
        out_specs=[blk] * 6,
        out_shape=[jax.ShapeDtypeStruct((n, F), jnp.float32)] * 6,
    )(h, pe, *ws, *bs)


def _node_update_mm(a1, hagg, stats, g, b, h_prev, h_i,
                    c1, pagg, p_prev, p_i, n_valid, nlp):
    n = a1.shape[0]
    grid = n // NBLK
    ws = [nlp['B1']['W'], nlp['B2']['W'],
          nlp['A1']['W'][:F], nlp['A1']['W'][F:],
          nlp['A2']['W'][:F], nlp['A2']['W'][F:],
          nlp['C1']['W'], nlp['C2']['W']]
    bs = [nlp['B1']['b'].reshape(1, F), nlp['B2']['b'].reshape(1, F),
          nlp['A1']['b'].reshape(1, F), nlp['A2']['b'].reshape(1, F),
          nlp['C1']['b'].reshape(1, F), nlp['C2']['b'].reshape(1, F)]

    def body(a_r, ha_r, st_r, g_r, b_r, hp_r, hi_r, c_r, pa_r, pp_r, pi_r,
             b1w, b2w, a1h, a1p, a2h, a2p, c1w, c2w,
             b1b, b2b, a1b, a2b, c1b, c2b,
             hn_r, pn_r, bh1_r, bh2_r, na1_r, vv_r, nc1_r, c2p_r):
        s = st_r[0, :]
        sq = st_r[1, :]
        m = s / n_valid
        var = sq / n_valid - m * m
        scale = g_r[0, :] / jnp.sqrt(var + BN_EPS)
        shift = b_r[0, :] - m * scale
        x = a_r[...] + ha_r[...]
        hb = hp_r[...] + jnp.maximum(x * scale + shift, 0.0) + hi_r[...]
        pb = pp_r[...] + jnp.tanh(c_r[...] + pa_r[...]) + pi_r[...]
        hn_r[...] = hb
        pn_r[...] = pb
        dot = lambda u, v_: jnp.dot(u, v_, preferred_element_type=jnp.float32)
        bh1_r[...] = dot(hb, b1w[...]) + b1b[...]
        bh2_r[...] = dot(hb, b2w[...]) + b2b[...]
        na1_r[...] = dot(hb, a1h[...]) + dot(pb, a1p[...]) + a1b[...]
        vv_r[...] = dot(hb, a2h[...]) + dot(pb, a2p[...]) + a2b[...]
        nc1_r[...] = dot(pb, c1w[...]) + c1b[...]
        c2p_r[...] = dot(pb, c2w[...]) + c2b[...]

    blk = pl.BlockSpec((NBLK, F), lambda i: (i, 0))
    return pl.pallas_call(
        body,
        grid=(grid,),
        in_specs=([blk, blk, _full((2, F)), _full((1, F)), _full((1, F)),
                   blk, blk, blk, blk, blk, blk]
                  + [_full((F, F))] * 8 + [_full((1, F))] * 6),
        out_specs=[blk] * 8,
        out_shape=[jax.ShapeDtypeStruct((n, F), jnp.float32)] * 8,
    )(a1, hagg, stats, g, b, h_prev, h_i, c1, pagg, p_prev, p_i, *ws, *bs)


def _edge_b3(e, w3, b3):
    Ep = e.shape[0]
    grid = Ep // EBLK

    def body(e_r, w_r, b_r, o_r):
        o_r[...] = jnp.dot(e_r[...], w_r[...], preferred_element_type=jnp.float32) + b_r[...]

    blk = pl.BlockSpec((EBLK, F), lambda i: (i, 0))
    return pl.pallas_call(
        body,
        grid=(grid,),
        in_specs=[blk, _full((F, F)), _full((1, F))],
        out_specs=blk,
        out_shape=jax.ShapeDtypeStruct((Ep, F), jnp.float32),
    )(e, w3, b3)


def _edge_stats(hat, n_valid):
    Ep = hat.shape[0]
    grid = Ep // EBLK

    def body(hat_r, o_r):
        i = pl.program_id(0)
        rows = i * EBLK + lax.broadcasted_iota(jnp.int32, (EBLK, F), 0)
        x = jnp.where(rows < n_valid, hat_r[...], 0.0)
        s = jnp.sum(x, axis=0, keepdims=True)
        sq = jnp.sum(x * x, axis=0, keepdims=True)
        part = jnp.concatenate([s, sq], axis=0)

        @pl.when(i == 0)
        def _():
            o_r[...] = jnp.zeros_like(o_r)

        o_r[...] += part

    return pl.pallas_call(
        body,
        grid=(grid,),
        in_specs=[pl.BlockSpec((EBLK, F), lambda i: (i, 0))],
        out_specs=pl.BlockSpec((2, F), lambda i: (0, 0)),
        out_shape=jax.ShapeDtypeStruct((2, F), jnp.float32),
    )(hat)


def _edge_update(hat, e_prev, e_i, stats, g, b, w3, b3, n_valid):
    Ep = hat.shape[0]
    grid = Ep // EBLK

    def body(hat_r, ep_r, ei_r, st_r, g_r, b_r, w_r, b3_r, en_r, o_r):
        s = st_r[0, :]
        sq = st_r[1, :]
        m = s / n_valid
        var = sq / n_valid - m * m
        scale = g_r[0, :] / jnp.sqrt(var + BN_EPS)
        shift = b_r[0, :] - m * scale
        en = ep_r[...] + jnp.maximum(hat_r[...] * scale + shift, 0.0) + ei_r[...]
        en_r[...] = en
        o_r[...] = jnp.dot(en, w_r[...], preferred_element_type=jnp.float32) + b3_r[...]

    blk = pl.BlockSpec((EBLK, F), lambda i: (i, 0))
    return pl.pallas_call(
        body,
        grid=(grid,),
        in_specs=[blk, blk, blk, _full((2, F)), _full((1, F)), _full((1, F)),
                  _full((F, F)), _full((1, F))],
        out_specs=[blk, blk],
        out_shape=[jax.ShapeDtypeStruct((Ep, F), jnp.float32),
                   jax.ShapeDtypeStruct((Ep, F), jnp.float32)],
    )(hat, e_prev, e_i, stats, g, b, w3, b3)


def _node_stats(a1, hagg):
    n = a1.shape[0]
    grid = n // NBLK

    def body(a_r, h_r, o_r):
        i = pl.program_id(0)
        x = a_r[...] + h_r[...]
        s = jnp.sum(x, axis=0, keepdims=True)
        sq = jnp.sum(x * x, axis=0, keepdims=True)
        part = jnp.concatenate([s, sq], axis=0)

        @pl.when(i == 0)
        def _():
            o_r[...] = jnp.zeros_like(o_r)

        o_r[...] += part

    blk = pl.BlockSpec((NBLK, F), lambda i: (i, 0))
    return pl.pallas_call(
        body,
        grid=(grid,),
        in_specs=[blk, blk],
        out_specs=pl.BlockSpec((2, F), lambda i: (0, 0)),
        out_shape=jax.ShapeDtypeStruct((2, F), jnp.float32),
    )(a1, hagg)


def _node_update(a1, hagg, stats, g, b, h_prev, h_i,
                 c1, pagg, p_prev, p_i, n_valid):
    n = a1.shape[0]
    grid = n // NBLK

    def body(a_r, ha_r, st_r, g_r, b_r, hp_r, hi_r, c_r, pa_r, pp_r, pi_r,
             hn_r, pn_r):
        s = st_r[0, :]
        sq = st_r[1, :]
        m = s / n_valid
        var = sq / n_valid - m * m
        scale = g_r[0, :] / jnp.sqrt(var + BN_EPS)
        shift = b_r[0, :] - m * scale
        x = a_r[...] + ha_r[...]
        hn_r[...] = hp_r[...] + jnp.maximum(x * scale + shift, 0.0) + hi_r[...]
        pn_r[...] = pp_r[...] + jnp.tanh(c_r[...] + pa_r[...]) + pi_r[...]

    blk = pl.BlockSpec((NBLK, F), lambda i: (i, 0))
    return pl.pallas_call(
        body,
        grid=(grid,),
        in_specs=[blk, blk, _full((2, F)), _full((1, F)), _full((1, F)),
                  blk, blk, blk, blk, blk, blk],
        out_specs=[blk, blk],
        out_shape=[jax.ShapeDtypeStruct((n, F), jnp.float32),
                   jax.ShapeDtypeStruct((n, F), jnp.float32)],
    )(a1, hagg, stats, g, b, h_prev, h_i, c1, pagg, p_prev, p_i)


def _pool_mlp(h, gid_row, params, n_graphs):
    n = h.shape[0]

    def body(h_r, gid_r, w1, b1, g1, bb1, w2, b2, g2, bb2, w3, b3, o_r):
        gids = gid_r[...]
        onehot = (lax.broadcasted_iota(jnp.int32, (n_graphs, n), 0)
                  == gids).astype(jnp.float32)
        hg = jnp.dot(onehot, h_r[...], preferred_element_type=jnp.float32)

        def bn_elu(x, gg, bb):
            m = jnp.mean(x, axis=0, keepdims=True)
            var = jnp.mean((x - m) ** 2, axis=0, keepdims=True)
            y = (x - m) / jnp.sqrt(var + BN_EPS) * gg + bb
            return jnp.where(y > 0, y, jnp.exp(jnp.minimum(y, 0.0)) - 1.0)

        x = bn_elu(jnp.dot(hg, w1[...], preferred_element_type=jnp.float32) + b1[...],
                   g1[...], bb1[...])
        x = bn_elu(jnp.dot(x, w2[...], preferred_element_type=jnp.float32) + b2[...],
                   g2[...], bb2[...])
        o_r[...] = jnp.dot(x, w3[...], preferred_element_type=jnp.float32) + b3[...]

    mp = params
    return pl.pallas_call(
        body,
        in_specs=[
            pl.BlockSpec((n, F), lambda: (0, 0)),
            pl.BlockSpec((1, n), lambda: (0, 0)),
            _full2((F, F)), _full2((1, F)), _full2((1, F)), _full2((1, F)),
            _full2((F, 32)), _full2((1, 32)), _full2((1, 32)), _full2((1, 32)),
            _full2((32, 1)), _full2((1, 1)),
        ],
        out_specs=pl.BlockSpec((n_graphs, 1), lambda: (0, 0)),
        out_shape=jax.ShapeDtypeStruct((n_graphs, 1), jnp.float32),
    )(h, gid_row,
      mp['mlp_l1']['W'], mp['mlp_l1']['b'].reshape(1, F),
      mp['mlp_bn1']['g'].reshape(1, F), mp['mlp_bn1']['b'].reshape(1, F),
      mp['mlp_l2']['W'], mp['mlp_l2']['b'].reshape(1, 32),
      mp['mlp_bn2']['g'].reshape(1, 32), mp['mlp_bn2']['b'].reshape(1, 32),
      mp['mlp_l3']['W'], mp['mlp_l3']['b'].reshape(1, 1))


def _full2(shape):
    return pl.BlockSpec(shape, lambda: (0, 0))


def kernel(node_feats, edge_feats, pos_enc, fp, edge_index, graph_ids, params):
    n = node_feats.shape[0]
    e_cnt = edge_index.shape[1]
    n_graphs = fp.shape[0]

    src, dst = edge_index[0], edge_index[1]
    perm = jnp.argsort(dst)
    dst_s = dst[perm]
    src_s = src[perm]
    ef_s = edge_feats[perm]

    n_ranges = -(-n // NPT)
    Ep = e_cnt + K
    Ep = -(-Ep // EBLK) * EBLK
    esz = (-(-(n_ranges + 17) // 16)) * 16

    npad = n_ranges * NPT
    src_p = jnp.zeros((Ep,), jnp.int32).at[:e_cnt].set(src_s)
    dst_p = jnp.full((Ep,), n - 1, jnp.int32).at[:e_cnt].set(dst_s)
    counts = jnp.zeros((npad,), jnp.int32).at[dst].add(1, mode='drop')
    nst = jnp.concatenate([jnp.zeros((1,), jnp.int32),
                           jnp.cumsum(counts, dtype=jnp.int32)])
    estart = nst[jnp.arange(n_ranges + 1, dtype=jnp.int32) * NPT]
    estart_p = jnp.full((esz,), e_cnt, jnp.int32).at[:n_ranges + 1].set(estart)
    nst_p = jnp.full((npad + 32,), e_cnt, jnp.int32).at[:npad + 1].set(nst)
    ef_p = jnp.zeros((Ep, 16), jnp.float32).at[:e_cnt].set(ef_s)

    p = params
    h, pe = _enc_node(
        node_feats, pos_enc,
        p['enc_node']['W'], p['enc_node']['b'].reshape(1, F),
        p['ln_node']['g'].reshape(1, F), p['ln_node']['b'].reshape(1, F),
        p['enc_pose']['W'], p['enc_pose']['b'].reshape(1, F))
    e, b3e = _enc_edge(
        ef_p,
        p['enc_edge']['W'], p['enc_edge']['b'].reshape(1, F),
        p['ln_edge']['g'].reshape(1, F), p['ln_edge']['b'].reshape(1, F),
        p['layers'][0]['B3']['W'], p['layers'][0]['B3']['b'].reshape(1, F))

    h_i, e_i, p_i = h, e, pe
    n_layers = len(p['layers'])
    e_prev = e

    mm = _node_matmuls(h, pe, p['layers'][0])
    for li, lp in enumerate(p['layers']):
        bh1, bh2, a1, vv, c1, c2p = mm
        hat, hagg, pagg = _sc_layer(bh1, bh2, vv, c2p, b3e,
                                    src_p, dst_p, estart_p, nst_p, n_ranges)
        if li + 1 < n_layers:
            stats_e = _edge_stats(hat, e_cnt)
            nlp = p['layers'][li + 1]
            e_prev, b3e = _edge_update(
                hat, e_prev, e_i, stats_e,
                lp['bn_e']['g'].reshape(1, F), lp['bn_e']['b'].reshape(1, F),
                nlp['B3']['W'], nlp['B3']['b'].reshape(1, F), e_cnt)
        stats_h = _node_stats(a1, hagg)
        if li + 1 < n_layers:
            h, pe, *mm = _node_update_mm(
                a1, hagg, stats_h,
                lp['bn_h']['g'].reshape(1, F), lp['bn_h']['b'].reshape(1, F),
                h, h_i, c1, pagg, pe, p_i, n, p['layers'][li + 1])
        else:
            h, pe = _node_update(
                a1, hagg, stats_h,
                lp['bn_h']['g'].reshape(1, F), lp['bn_h']['b'].reshape(1, F),
                h, h_i, c1, pagg, pe, p_i, n)

    gid_row = graph_ids.reshape(1, n).astype(jnp.int32)
    return _pool_mlp(h, gid_row, p, n_graphs)


# speedup vs baseline: 4.8507x; 1.0002x over previous
import functools

import jax
import jax.numpy as jnp
from jax import lax
from jax.experimental import pallas as pl
from jax.experimental.pallas import tpu as pltpu
from jax.experimental.pallas import tpu_sc as plsc

F = 128
NPT = 64
K = 64
NW = 32
EBLK = 1152
NBLK = 2000
BN_EPS = 1e-5
ETA_EPS = 1e-6


def _sigmoid16(x):
    return 1.0 / (1.0 + jnp.exp(-x))


def _sc_layer(bh1, bh2, vv, c2p, b3e, src_p, dst_p, estart_p, nst_p,
                 n_ranges):
    Ep = b3e.shape[0]
    Npad = n_ranges * NPT
    ESZ = estart_p.shape[0]
    mesh = plsc.VectorSubcoreMesh(core_axis_name="c", subcore_axis_name="s",
                                  num_cores=2, num_subcores=16)

    @functools.partial(
        pl.kernel,
        out_type=(
            jax.ShapeDtypeStruct((Ep, F), jnp.float32),
            jax.ShapeDtypeStruct((Npad, F), jnp.float32),
            jax.ShapeDtypeStruct((Npad, F), jnp.float32),
        ),
        mesh=mesh,
        scratch_types=[
            pltpu.VMEM((K, F), jnp.float32),
            pltpu.VMEM((K, F), jnp.float32),
            pltpu.VMEM((K, F), jnp.float32),
            pltpu.VMEM((K, F), jnp.float32),
            pltpu.VMEM((K, F), jnp.float32),
            pltpu.VMEM((K, F), jnp.float32),
            pltpu.VMEM((K, F), jnp.float32),
            pltpu.VMEM((K, F), jnp.float32),
            pltpu.VMEM((K, F), jnp.float32),
            pltpu.VMEM((K, F), jnp.float32),
            pltpu.VMEM((NPT, F), jnp.float32),
            pltpu.VMEM((NPT, F), jnp.float32),
            pltpu.VMEM((NPT, F), jnp.float32),
            pltpu.VMEM((K,), jnp.int32),
            pltpu.VMEM((K,), jnp.int32),
            pltpu.VMEM((K,), jnp.int32),
            pltpu.VMEM((K,), jnp.int32),
            pltpu.VMEM((K + 16,), jnp.int32),
            pltpu.VMEM((K + 16,), jnp.int32),
            pltpu.VMEM((ESZ,), jnp.int32),
            pltpu.VMEM((NPT + 32,), jnp.int32),
            pltpu.SemaphoreType.DMA,
            pltpu.SemaphoreType.DMA,
            pltpu.SemaphoreType.DMA,
            pltpu.SemaphoreType.DMA,
            pltpu.SemaphoreType.DMA,
        ],
    )
    def k(bh1_h, bh2_h, vv_h, c2p_h, b3e_h, src_h, dst_h, est_h, nst_h,
          hat_h, hagg_h, pagg_h,
          A0, A1, B0, B1, C0, C1, D0, D1, E0, E1, ss, hacc, pacc,
          sidx0, sidx1, didx0, didx1, dsm0, dsm1, estv, nstv,
          isem0, isem1, dsem0, dsem1, wsem):
        w = lax.axis_index("s") * 2 + lax.axis_index("c")
        pltpu.sync_copy(est_h, estv)

        A_ = (A0, A1)
        B_ = (B0, B1)
        C_ = (C0, C1)
        D_ = (D0, D1)
        E_ = (E0, E1)
        SI = (sidx0, sidx1)
        DI = (didx0, didx1)
        DS = (dsm0, dsm1)
        IS = (isem0, isem1)
        DSEM = (dsem0, dsem1)

        n_mine = (n_ranges - 1 - w) // NW + 1

        def sval(ref, i):
            return ref[pl.ds(i, 16)][0]

        def per_range(ri, _):
            r = w + ri * NW
            base = r * NPT
            e0 = sval(estv, r)
            e1 = sval(estv, r + 1)
            c0 = (e0 // 8) * 8
            nch = jnp.maximum(e1 - c0 + K - 1, 0) // K
            pltpu.sync_copy(nst_h.at[pl.ds(base, NPT + 32)], nstv)

            def zero_rows(nl, _):
                zz = jnp.zeros((16,), jnp.float32)
                for j in range(F // 16):
                    sl = pl.ds(j * 16, 16)
                    ss[nl, sl] = zz
                    hacc[nl, sl] = zz
                    pacc[nl, sl] = zz
                return 0

            lax.fori_loop(0, NPT, zero_rows, 0)

            def issue_idx(ci, b):
                c = c0 + ci * K
                pltpu.async_copy(src_h.at[pl.ds(c, K)], SI[b], IS[b])
                pltpu.async_copy(dst_h.at[pl.ds(c, K)], DI[b], IS[b])
                pltpu.async_copy(dst_h.at[pl.ds(c, K)],
                                 DS[b].at[pl.ds(0, K)], IS[b])

            def wait_idx(b):
                pltpu.make_async_copy(src_h.at[pl.ds(0, K)], SI[b],
                                      IS[b]).wait()
                pltpu.make_async_copy(dst_h.at[pl.ds(0, K)], DI[b],
                                      IS[b]).wait()
                pltpu.make_async_copy(dst_h.at[pl.ds(0, K)],
                                      DS[b].at[pl.ds(0, K)], IS[b]).wait()

            def issue_data(ci, b):
                c = c0 + ci * K
                pltpu.async_copy(b3e_h.at[pl.ds(c, K)], A_[b], DSEM[b])
                pltpu.async_copy(bh1_h.at[DI[b]], B_[b], DSEM[b])
                pltpu.async_copy(bh2_h.at[SI[b]], C_[b], DSEM[b])
                pltpu.async_copy(vv_h.at[SI[b]], D_[b], DSEM[b])
                pltpu.async_copy(c2p_h.at[SI[b]], E_[b], DSEM[b])

            def wait_data(b):
                for buf in (A_[b], B_[b], C_[b], D_[b], E_[b]):
                    pltpu.make_async_copy(b3e_h.at[pl.ds(0, K)], buf,
                                          DSEM[b]).wait()

            def prefetch(cur, nb):
                nxt = cur + 1

                @pl.when(nxt < nch)
                def _():
                    issue_idx(nxt, nb)
                    wait_idx(nb)
                    issue_data(nxt, nb)

            def compute(cur, b):
                c = c0 + cur * K

                def bulk(el, _):
                    for j in range(F // 16):
                        sl = pl.ds(j * 16, 16)
                        A_[b][el, sl] = (A_[b][el, sl] + B_[b][el, sl]
                                         + C_[b][el, sl])
                    return 0

                lax.fori_loop(0, K, bulk, 0)
                pltpu.async_copy(A_[b], hat_h.at[pl.ds(c, K)], wsem)

                nlo = jnp.clip(sval(DS[b], 0) - base, 0, NPT - 1)
                nhi = jnp.clip(DS[b][pl.ds(K - 16, 16)][15] - base,
                               0, NPT - 1) + 1

                def per_node(nl, _):
                    es = jnp.maximum(sval(nstv, nl), c)
                    ee = jnp.minimum(sval(nstv, nl + 1), c + K)
                    init = tuple(jnp.zeros((16,), jnp.float32)
                                 for _ in range(3 * (F // 16)))

                    @plsc.parallel_loop(es, ee, unroll=2, carry=init)
                    def accs(e, accs_):
                        el = e - c
                        out = []
                        for j in range(F // 16):
                            sl = pl.ds(j * 16, 16)
                            sig = _sigmoid16(A_[b][el, sl])
                            out.append(accs_[j] + sig)
                            out.append(accs_[j + 8] + sig * D_[b][el, sl])
                            out.append(accs_[j + 16] + sig * E_[b][el, sl])
                        return (tuple(out[0::3]) + tuple(out[1::3])
                                + tuple(out[2::3]))
                    for j in range(F // 16):
                        sl = pl.ds(j * 16, 16)
                        ss[nl, sl] = ss[nl, sl] + accs[j]
                        hacc[nl, sl] = hacc[nl, sl] + accs[j + 8]
                        pacc[nl, sl] = pacc[nl, sl] + accs[j + 16]
                    return 0

                lax.fori_loop(nlo, nhi, per_node, 0)
                pltpu.make_async_copy(A_[b], hat_h.at[pl.ds(c, K)],
                                      wsem).wait()

            @pl.when(nch > 0)
            def _():
                issue_idx(0, 0)
                wait_idx(0)
                issue_data(0, 0)

                def step(ci, _):
                    for b in range(2):
                        def mk(cur_b):
                            def inner():
                                cur = 2 * ci + cur_b
                                prefetch(cur, (cur_b + 1) % 2)
                                wait_data(cur_b)
                                compute(cur, cur_b)
                            return inner

                        pl.when(2 * ci + b < nch)(mk(b))
                    return 0

                lax.fori_loop(0, (nch + 1) // 2, step, 0)

            def fin(nl, _):
                for j in range(F // 16):
                    sl = pl.ds(j * 16, 16)
                    inv = 1.0 / (ss[nl, sl] + ETA_EPS)
                    hacc[nl, sl] = hacc[nl, sl] * inv
                    pacc[nl, sl] = pacc[nl, sl] * inv
                return 0

            lax.fori_loop(0, NPT, fin, 0)

            pltpu.sync_copy(hacc, hagg_h.at[pl.ds(base, NPT)])
            pltpu.sync_copy(pacc, pagg_h.at[pl.ds(base, NPT)])
            return 0

        lax.fori_loop(0, n_mine, per_range, 0)

    return k(bh1, bh2, vv, c2p, b3e, src_p, dst_p, estart_p, nst_p)


def _full(shape):
    return pl.BlockSpec(shape, lambda i: (0, 0))


def _enc_node(nf, pos, wn, bn_, gn, bbn, wp, bp):
    n = nf.shape[0]
    grid = n // NBLK

    def body(nf_r, pos_r, wn_r, bn_r, gn_r, bbn_r, wp_r, bp_r, h_r, pe_r):
        x = jnp.dot(nf_r[...], wn_r[...], preferred_element_type=jnp.float32) + bn_r[...]
        m = jnp.mean(x, axis=-1, keepdims=True)
        v = jnp.mean((x - m) ** 2, axis=-1, keepdims=True)
        h_r[...] = (x - m) / jnp.sqrt(v + BN_EPS) * gn_r[...] + bbn_r[...]
        pe_r[...] = jnp.dot(pos_r[...], wp_r[...], preferred_element_type=jnp.float32) + bp_r[...]

    return pl.pallas_call(
        body,
        grid=(grid,),
        in_specs=[
            pl.BlockSpec((NBLK, F), lambda i: (i, 0)),
            pl.BlockSpec((NBLK, 16), lambda i: (i, 0)),
            _full((F, F)), _full((1, F)), _full((1, F)), _full((1, F)),
            _full((16, F)), _full((1, F)),
        ],
        out_specs=[
            pl.BlockSpec((NBLK, F), lambda i: (i, 0)),
            pl.BlockSpec((NBLK, F), lambda i: (i, 0)),
        ],
        out_shape=[
            jax.ShapeDtypeStruct((n, F), jnp.float32),
            jax.ShapeDtypeStruct((n, F), jnp.float32),
        ],
    )(nf, pos, wn, bn_, gn, bbn, wp, bp)


def _enc_edge(ef, we, be, ge, bbe, w3, b3):
    Ep = ef.shape[0]
    grid = Ep // EBLK

    def body(ef_r, we_r, be_r, ge_r, bbe_r, w3_r, b3_r, e_r, b3e_r):
        x = jnp.dot(ef_r[...], we_r[...], preferred_element_type=jnp.float32) + be_r[...]
        m = jnp.mean(x, axis=-1, keepdims=True)
        v = jnp.mean((x - m) ** 2, axis=-1, keepdims=True)
        e0 = (x - m) / jnp.sqrt(v + BN_EPS) * ge_r[...] + bbe_r[...]
        e_r[...] = e0
        b3e_r[...] = jnp.dot(e0, w3_r[...],
                             preferred_element_type=jnp.float32) + b3_r[...]

    blk = pl.BlockSpec((EBLK, F), lambda i: (i, 0))
    return pl.pallas_call(
        body,
        grid=(grid,),
        in_specs=[
            pl.BlockSpec((EBLK, 16), lambda i: (i, 0)),
            _full((16, F)), _full((1, F)), _full((1, F)), _full((1, F)),
            _full((F, F)), _full((1, F)),
        ],
        out_specs=[blk, blk],
        out_shape=[jax.ShapeDtypeStruct((Ep, F), jnp.float32),
                   jax.ShapeDtypeStruct((Ep, F), jnp.float32)],
    )(ef, we, be, ge, bbe, w3, b3)


def _node_matmuls(h, pe, lp):
    n = h.shape[0]
    grid = n // NBLK
    ws = [lp['B1']['W'], lp['B2']['W'],
          lp['A1']['W'][:F], lp['A1']['W'][F:],
          lp['A2']['W'][:F], lp['A2']['W'][F:],
          lp['C1']['W'], lp['C2']['W']]
    bs = [lp['B1']['b'].reshape(1, F), lp['B2']['b'].reshape(1, F),
          lp['A1']['b'].reshape(1, F), lp['A2']['b'].reshape(1, F),
          lp['C1']['b'].reshape(1, F), lp['C2']['b'].reshape(1, F)]

    def body(h_r, pe_r, b1w, b2w, a1h, a1p, a2h, a2p, c1w, c2w,
             b1b, b2b, a1b, a2b, c1b, c2b,
             bh1_r, bh2_r, a1_r, vv_r, c1_r, c2p_r):
        hb = h_r[...]
        pb = pe_r[...]
        dot = lambda a, b: jnp.dot(a, b, preferred_element_type=jnp.float32)
        bh1_r[...] = dot(hb, b1w[...]) + b1b[...]
        bh2_r[...] = dot(hb, b2w[...]) + b2b[...]
        a1_r[...] = dot(hb, a1h[...]) + dot(pb, a1p[...]) + a1b[...]
        vv_r[...] = dot(hb, a2h[...]) + dot(pb, a2p[...]) + a2b[...]
        c1_r[...] = dot(pb, c1w[...]) + c1b[...]
        c2p_r[...] = dot(pb, c2w[...]) + c2b[...]

    blk = pl.BlockSpec((NBLK, F), lambda i: (i, 0))
    return pl.pallas_call(
        body,
        grid=(grid,),
        in_specs=[blk, blk] + [_full((F, F))] * 8 + [_full((1, F))] * 6,
        out_specs=[blk] * 6,
        out_shape=[jax.ShapeDtypeStruct((n, F), jnp.float32)] * 6,
    )(h, pe, *ws, *bs)


def _node_update_mm(a1, hagg, stats, g, b, h_prev, h_i,
                    c1, pagg, p_prev, p_i, n_valid, nlp):
    n = a1.shape[0]
    grid = n // NBLK
    ws = [nlp['B1']['W'], nlp['B2']['W'],
          nlp['A1']['W'][:F], nlp['A1']['W'][F:],
          nlp['A2']['W'][:F], nlp['A2']['W'][F:],
          nlp['C1']['W'], nlp['C2']['W']]
    bs = [nlp['B1']['b'].reshape(1, F), nlp['B2']['b'].reshape(1, F),
          nlp['A1']['b'].reshape(1, F), nlp['A2']['b'].reshape(1, F),
          nlp['C1']['b'].reshape(1, F), nlp['C2']['b'].reshape(1, F)]

    def body(a_r, ha_r, st_r, g_r, b_r, hp_r, hi_r, c_r, pa_r, pp_r, pi_r,
             b1w, b2w, a1h, a1p, a2h, a2p, c1w, c2w,
             b1b, b2b, a1b, a2b, c1b, c2b,
             hn_r, pn_r, bh1_r, bh2_r, na1_r, vv_r, nc1_r, c2p_r):
        s = st_r[0, :]
        sq = st_r[1, :]
        m = s / n_valid
        var = sq / n_valid - m * m
        scale = g_r[0, :] / jnp.sqrt(var + BN_EPS)
        shift = b_r[0, :] - m * scale
        x = a_r[...] + ha_r[...]
        hb = hp_r[...] + jnp.maximum(x * scale + shift, 0.0) + hi_r[...]
        pb = pp_r[...] + jnp.tanh(c_r[...] + pa_r[...]) + pi_r[...]
        hn_r[...] = hb
        pn_r[...] = pb
        dot = lambda u, v_: jnp.dot(u, v_, preferred_element_type=jnp.float32)
        bh1_r[...] = dot(hb, b1w[...]) + b1b[...]
        bh2_r[...] = dot(hb, b2w[...]) + b2b[...]
        na1_r[...] = dot(hb, a1h[...]) + dot(pb, a1p[...]) + a1b[...]
        vv_r[...] = dot(hb, a2h[...]) + dot(pb, a2p[...]) + a2b[...]
        nc1_r[...] = dot(pb, c1w[...]) + c1b[...]
        c2p_r[...] = dot(pb, c2w[...]) + c2b[...]

    blk = pl.BlockSpec((NBLK, F), lambda i: (i, 0))
    return pl.pallas_call(
        body,
        grid=(grid,),
        in_specs=([blk, blk, _full((2, F)), _full((1, F)), _full((1, F)),
                   blk, blk, blk, blk, blk, blk]
                  + [_full((F, F))] * 8 + [_full((1, F))] * 6),
        out_specs=[blk] * 8,
        out_shape=[jax.ShapeDtypeStruct((n, F), jnp.float32)] * 8,
    )(a1, hagg, stats, g, b, h_prev, h_i, c1, pagg, p_prev, p_i, *ws, *bs)


def _edge_b3(e, w3, b3):
    Ep = e.shape[0]
    grid = Ep // EBLK

    def body(e_r, w_r, b_r, o_r):
        o_r[...] = jnp.dot(e_r[...], w_r[...], preferred_element_type=jnp.float32) + b_r[...]

    blk = pl.BlockSpec((EBLK, F), lambda i: (i, 0))
    return pl.pallas_call(
        body,
        grid=(grid,),
        in_specs=[blk, _full((F, F)), _full((1, F))],
        out_specs=blk,
        out_shape=jax.ShapeDtypeStruct((Ep, F), jnp.float32),
    )(e, w3, b3)


def _edge_stats(hat, n_valid):
    Ep = hat.shape[0]
    grid = Ep // EBLK

    def body(hat_r, o_r):
        i = pl.program_id(0)
        rows = i * EBLK + lax.broadcasted_iota(jnp.int32, (EBLK, F), 0)
        x = jnp.where(rows < n_valid, hat_r[...], 0.0)
        s = jnp.sum(x, axis=0, keepdims=True)
        sq = jnp.sum(x * x, axis=0, keepdims=True)
        part = jnp.concatenate([s, sq], axis=0)

        @pl.when(i == 0)
        def _():
            o_r[...] = jnp.zeros_like(o_r)

        o_r[...] += part

    return pl.pallas_call(
        body,
        grid=(grid,),
        in_specs=[pl.BlockSpec((EBLK, F), lambda i: (i, 0))],
        out_specs=pl.BlockSpec((2, F), lambda i: (0, 0)),
        out_shape=jax.ShapeDtypeStruct((2, F), jnp.float32),
    )(hat)


def _edge_update(hat, e_prev, e_i, stats, g, b, w3, b3, n_valid):
    Ep = hat.shape[0]
    grid = Ep // EBLK

    def body(hat_r, ep_r, ei_r, st_r, g_r, b_r, w_r, b3_r, en_r, o_r):
        s = st_r[0, :]
        sq = st_r[1, :]
        m = s / n_valid
        var = sq / n_valid - m * m
        scale = g_r[0, :] / jnp.sqrt(var + BN_EPS)
        shift = b_r[0, :] - m * scale
        en = ep_r[...] + jnp.maximum(hat_r[...] * scale + shift, 0.0) + ei_r[...]
        en_r[...] = en
        o_r[...] = jnp.dot(en, w_r[...], preferred_element_type=jnp.float32) + b3_r[...]

    blk = pl.BlockSpec((EBLK, F), lambda i: (i, 0))
    return pl.pallas_call(
        body,
        grid=(grid,),
        in_specs=[blk, blk, blk, _full((2, F)), _full((1, F)), _full((1, F)),
                  _full((F, F)), _full((1, F))],
        out_specs=[blk, blk],
        out_shape=[jax.ShapeDtypeStruct((Ep, F), jnp.float32),
                   jax.ShapeDtypeStruct((Ep, F), jnp.float32)],
    )(hat, e_prev, e_i, stats, g, b, w3, b3)


def _node_stats(a1, hagg):
    n = a1.shape[0]
    grid = n // NBLK

    def body(a_r, h_r, o_r):
        i = pl.program_id(0)
        x = a_r[...] + h_r[...]
        s = jnp.sum(x, axis=0, keepdims=True)
        sq = jnp.sum(x * x, axis=0, keepdims=True)
        part = jnp.concatenate([s, sq], axis=0)

        @pl.when(i == 0)
        def _():
            o_r[...] = jnp.zeros_like(o_r)

        o_r[...] += part

    blk = pl.BlockSpec((NBLK, F), lambda i: (i, 0))
    return pl.pallas_call(
        body,
        grid=(grid,),
        in_specs=[blk, blk],
        out_specs=pl.BlockSpec((2, F), lambda i: (0, 0)),
        out_shape=jax.ShapeDtypeStruct((2, F), jnp.float32),
    )(a1, hagg)


def _node_update(a1, hagg, stats, g, b, h_prev, h_i,
                 c1, pagg, p_prev, p_i, n_valid):
    n = a1.shape[0]
    grid = n // NBLK

    def body(a_r, ha_r, st_r, g_r, b_r, hp_r, hi_r, c_r, pa_r, pp_r, pi_r,
             hn_r, pn_r):
        s = st_r[0, :]
        sq = st_r[1, :]
        m = s / n_valid
        var = sq / n_valid - m * m
        scale = g_r[0, :] / jnp.sqrt(var + BN_EPS)
        shift = b_r[0, :] - m * scale
        x = a_r[...] + ha_r[...]
        hn_r[...] = hp_r[...] + jnp.maximum(x * scale + shift, 0.0) + hi_r[...]
        pn_r[...] = pp_r[...] + jnp.tanh(c_r[...] + pa_r[...]) + pi_r[...]

    blk = pl.BlockSpec((NBLK, F), lambda i: (i, 0))
    return pl.pallas_call(
        body,
        grid=(grid,),
        in_specs=[blk, blk, _full((2, F)), _full((1, F)), _full((1, F)),
                  blk, blk, blk, blk, blk, blk],
        out_specs=[blk, blk],
        out_shape=[jax.ShapeDtypeStruct((n, F), jnp.float32),
                   jax.ShapeDtypeStruct((n, F), jnp.float32)],
    )(a1, hagg, stats, g, b, h_prev, h_i, c1, pagg, p_prev, p_i)


def _pool_mlp(h, gid_row, params, n_graphs):
    n = h.shape[0]

    def body(h_r, gid_r, w1, b1, g1, bb1, w2, b2, g2, bb2, w3, b3, o_r):
        gids = gid_r[...]
        onehot = (lax.broadcasted_iota(jnp.int32, (n_graphs, n), 0)
                  == gids).astype(jnp.float32)
        hg = jnp.dot(onehot, h_r[...], preferred_element_type=jnp.float32)

        def bn_elu(x, gg, bb):
            m = jnp.mean(x, axis=0, keepdims=True)
            var = jnp.mean((x - m) ** 2, axis=0, keepdims=True)
            y = (x - m) / jnp.sqrt(var + BN_EPS) * gg + bb
            return jnp.where(y > 0, y, jnp.exp(jnp.minimum(y, 0.0)) - 1.0)

        x = bn_elu(jnp.dot(hg, w1[...], preferred_element_type=jnp.float32) + b1[...],
                   g1[...], bb1[...])
        x = bn_elu(jnp.dot(x, w2[...], preferred_element_type=jnp.float32) + b2[...],
                   g2[...], bb2[...])
        o_r[...] = jnp.dot(x, w3[...], preferred_element_type=jnp.float32) + b3[...]

    mp = params
    return pl.pallas_call(
        body,
        in_specs=[
            pl.BlockSpec((n, F), lambda: (0, 0)),
            pl.BlockSpec((1, n), lambda: (0, 0)),
            _full2((F, F)), _full2((1, F)), _full2((1, F)), _full2((1, F)),
            _full2((F, 32)), _full2((1, 32)), _full2((1, 32)), _full2((1, 32)),
            _full2((32, 1)), _full2((1, 1)),
        ],
        out_specs=pl.BlockSpec((n_graphs, 1), lambda: (0, 0)),
        out_shape=jax.ShapeDtypeStruct((n_graphs, 1), jnp.float32),
    )(h, gid_row,
      mp['mlp_l1']['W'], mp['mlp_l1']['b'].reshape(1, F),
      mp['mlp_bn1']['g'].reshape(1, F), mp['mlp_bn1']['b'].reshape(1, F),
      mp['mlp_l2']['W'], mp['mlp_l2']['b'].reshape(1, 32),
      mp['mlp_bn2']['g'].reshape(1, 32), mp['mlp_bn2']['b'].reshape(1, 32),
      mp['mlp_l3']['W'], mp['mlp_l3']['b'].reshape(1, 1))


def _full2(shape):
    return pl.BlockSpec(shape, lambda: (0, 0))


def kernel(node_feats, edge_feats, pos_enc, fp, edge_index, graph_ids, params):
    n = node_feats.shape[0]
    e_cnt = edge_index.shape[1]
    n_graphs = fp.shape[0]

    src, dst = edge_index[0], edge_index[1]
    perm = jnp.argsort(dst)
    dst_s = dst[perm]
    src_s = src[perm]
    ef_s = edge_feats[perm]

    n_ranges = -(-n // NPT)
    Ep = e_cnt + K
    Ep = -(-Ep // EBLK) * EBLK
    esz = (-(-(n_ranges + 17) // 16)) * 16

    npad = n_ranges * NPT
    src_p = jnp.zeros((Ep,), jnp.int32).at[:e_cnt].set(src_s)
    dst_p = jnp.full((Ep,), n - 1, jnp.int32).at[:e_cnt].set(dst_s)
    counts = jnp.zeros((npad,), jnp.int32).at[dst].add(1, mode='drop')
    nst = jnp.concatenate([jnp.zeros((1,), jnp.int32),
                           jnp.cumsum(counts, dtype=jnp.int32)])
    estart = nst[jnp.arange(n_ranges + 1, dtype=jnp.int32) * NPT]
    estart_p = jnp.full((esz,), e_cnt, jnp.int32).at[:n_ranges + 1].set(estart)
    nst_p = jnp.full((npad + 32,), e_cnt, jnp.int32).at[:npad + 1].set(nst)
    ef_p = jnp.zeros((Ep, 16), jnp.float32).at[:e_cnt].set(ef_s)

    p = params
    h, pe = _enc_node(
        node_feats, pos_enc,
        p['enc_node']['W'], p['enc_node']['b'].reshape(1, F),
        p['ln_node']['g'].reshape(1, F), p['ln_node']['b'].reshape(1, F),
        p['enc_pose']['W'], p['enc_pose']['b'].reshape(1, F))
    e, b3e = _enc_edge(
        ef_p,
        p['enc_edge']['W'], p['enc_edge']['b'].reshape(1, F),
        p['ln_edge']['g'].reshape(1, F), p['ln_edge']['b'].reshape(1, F),
        p['layers'][0]['B3']['W'], p['layers'][0]['B3']['b'].reshape(1, F))

    h_i, e_i, p_i = h, e, pe
    n_layers = len(p['layers'])
    e_prev = e

    mm = _node_matmuls(h, pe, p['layers'][0])
    for li, lp in enumerate(p['layers']):
        bh1, bh2, a1, vv, c1, c2p = mm
        hat, hagg, pagg = _sc_layer(bh1, bh2, vv, c2p, b3e,
                                    src_p, dst_p, estart_p, nst_p, n_ranges)
        if li + 1 < n_layers:
            stats_e = _edge_stats(hat, e_cnt)
            nlp = p['layers'][li + 1]
            e_prev, b3e = _edge_update(
                hat, e_prev, e_i, stats_e,
                lp['bn_e']['g'].reshape(1, F), lp['bn_e']['b'].reshape(1, F),
                nlp['B3']['W'], nlp['B3']['b'].reshape(1, F), e_cnt)
        stats_h = _node_stats(a1, hagg)
        if li + 1 < n_layers:
            h, pe, *mm = _node_update_mm(
                a1, hagg, stats_h,
                lp['bn_h']['g'].reshape(1, F), lp['bn_h']['b'].reshape(1, F),
                h, h_i, c1, pagg, pe, p_i, n, p['layers'][li + 1])
        else:
            h, pe = _node_update(
                a1, hagg, stats_h,
                lp['bn_h']['g'].reshape(1, F), lp['bn_h']['b'].reshape(1, F),
                h, h_i, c1, pagg, pe, p_i, n)

    gid_row = graph_ids.reshape(1, n).astype(jnp.int32)
    return _pool_mlp(h, gid_row, p, n_graphs)


# speedup vs baseline: 4.8513x; 1.0001x over previous
import functools

import jax
import jax.numpy as jnp
from jax import lax
from jax.experimental import pallas as pl
from jax.experimental.pallas import tpu as pltpu
from jax.experimental.pallas import tpu_sc as plsc

F = 128
NPT = 64
K = 64
NW = 32
EBLK = 1152
NBLK = 2000
BN_EPS = 1e-5
ETA_EPS = 1e-6


def _sigmoid16(x):
    return 1.0 / (1.0 + jnp.exp(-x))


def _sc_layer(bh1, bh2, vv, c2p, b3e, src_p, dst_p, estart_p, nst_p,
                 n_ranges):
    Ep = b3e.shape[0]
    Npad = n_ranges * NPT
    ESZ = estart_p.shape[0]
    mesh = plsc.VectorSubcoreMesh(core_axis_name="c", subcore_axis_name="s",
                                  num_cores=2, num_subcores=16)

    @functools.partial(
        pl.kernel,
        out_type=(
            jax.ShapeDtypeStruct((Ep, F), jnp.float32),
            jax.ShapeDtypeStruct((Npad, F), jnp.float32),
            jax.ShapeDtypeStruct((Npad, F), jnp.float32),
        ),
        mesh=mesh,
        scratch_types=[
            pltpu.VMEM((K, F), jnp.float32),
            pltpu.VMEM((K, F), jnp.float32),
            pltpu.VMEM((K, F), jnp.float32),
            pltpu.VMEM((K, F), jnp.float32),
            pltpu.VMEM((K, F), jnp.float32),
            pltpu.VMEM((K, F), jnp.float32),
            pltpu.VMEM((K, F), jnp.float32),
            pltpu.VMEM((K, F), jnp.float32),
            pltpu.VMEM((K, F), jnp.float32),
            pltpu.VMEM((K, F), jnp.float32),
            pltpu.VMEM((NPT, F), jnp.float32),
            pltpu.VMEM((NPT, F), jnp.float32),
            pltpu.VMEM((NPT, F), jnp.float32),
            pltpu.VMEM((K,), jnp.int32),
            pltpu.VMEM((K,), jnp.int32),
            pltpu.VMEM((K,), jnp.int32),
            pltpu.VMEM((K,), jnp.int32),
            pltpu.VMEM((K + 16,), jnp.int32),
            pltpu.VMEM((K + 16,), jnp.int32),
            pltpu.VMEM((ESZ,), jnp.int32),
            pltpu.VMEM((NPT + 32,), jnp.int32),
            pltpu.SemaphoreType.DMA,
            pltpu.SemaphoreType.DMA,
            pltpu.SemaphoreType.DMA,
            pltpu.SemaphoreType.DMA,
            pltpu.SemaphoreType.DMA,
        ],
    )
    def k(bh1_h, bh2_h, vv_h, c2p_h, b3e_h, src_h, dst_h, est_h, nst_h,
          hat_h, hagg_h, pagg_h,
          A0, A1, B0, B1, C0, C1, D0, D1, E0, E1, ss, hacc, pacc,
          sidx0, sidx1, didx0, didx1, dsm0, dsm1, estv, nstv,
          isem0, isem1, dsem0, dsem1, wsem):
        w = lax.axis_index("s") * 2 + lax.axis_index("c")
        pltpu.sync_copy(est_h, estv)

        A_ = (A0, A1)
        B_ = (B0, B1)
        C_ = (C0, C1)
        D_ = (D0, D1)
        E_ = (E0, E1)
        SI = (sidx0, sidx1)
        DI = (didx0, didx1)
        DS = (dsm0, dsm1)
        IS = (isem0, isem1)
        DSEM = (dsem0, dsem1)

        n_mine = (n_ranges - 1 - w) // NW + 1

        def sval(ref, i):
            return ref[pl.ds(i, 16)][0]

        def per_range(ri, _):
            r = w + ri * NW
            base = r * NPT
            e0 = sval(estv, r)
            e1 = sval(estv, r + 1)
            c0 = (e0 // 8) * 8
            nch = jnp.maximum(e1 - c0 + K - 1, 0) // K
            pltpu.sync_copy(nst_h.at[pl.ds(base, NPT + 32)], nstv)

            def zero_rows(nl, _):
                zz = jnp.zeros((16,), jnp.float32)
                for j in range(F // 16):
                    sl = pl.ds(j * 16, 16)
                    ss[nl, sl] = zz
                    hacc[nl, sl] = zz
                    pacc[nl, sl] = zz
                return 0

            lax.fori_loop(0, NPT, zero_rows, 0)

            def issue_idx(ci, b):
                c = c0 + ci * K
                pltpu.async_copy(src_h.at[pl.ds(c, K)], SI[b], IS[b])
                pltpu.async_copy(dst_h.at[pl.ds(c, K)], DI[b], IS[b])
                pltpu.async_copy(dst_h.at[pl.ds(c, K)],
                                 DS[b].at[pl.ds(0, K)], IS[b])

            def wait_idx(b):
                pltpu.make_async_copy(src_h.at[pl.ds(0, K)], SI[b],
                                      IS[b]).wait()
                pltpu.make_async_copy(dst_h.at[pl.ds(0, K)], DI[b],
                                      IS[b]).wait()
                pltpu.make_async_copy(dst_h.at[pl.ds(0, K)],
                                      DS[b].at[pl.ds(0, K)], IS[b]).wait()

            def issue_data(ci, b):
                c = c0 + ci * K
                pltpu.async_copy(b3e_h.at[pl.ds(c, K)], A_[b], DSEM[b])
                pltpu.async_copy(bh1_h.at[DI[b]], B_[b], DSEM[b])
                pltpu.async_copy(bh2_h.at[SI[b]], C_[b], DSEM[b])
                pltpu.async_copy(vv_h.at[SI[b]], D_[b], DSEM[b])
                pltpu.async_copy(c2p_h.at[SI[b]], E_[b], DSEM[b])

            def wait_data(b):
                for buf in (A_[b], B_[b], C_[b], D_[b], E_[b]):
                    pltpu.make_async_copy(b3e_h.at[pl.ds(0, K)], buf,
                                          DSEM[b]).wait()

            def prefetch(cur, nb):
                nxt = cur + 1

                @pl.when(nxt < nch)
                def _():
                    issue_idx(nxt, nb)
                    wait_idx(nb)
                    issue_data(nxt, nb)

            def compute(cur, b):
                c = c0 + cur * K

                def bulk(el, _):
                    for j in range(F // 16):
                        sl = pl.ds(j * 16, 16)
                        A_[b][el, sl] = (A_[b][el, sl] + B_[b][el, sl]
                                         + C_[b][el, sl])
                    return 0

                lax.fori_loop(0, K, bulk, 0)
                pltpu.async_copy(A_[b], hat_h.at[pl.ds(c, K)], wsem)

                nlo = jnp.clip(sval(DS[b], 0) - base, 0, NPT - 1)
                nhi = jnp.clip(DS[b][pl.ds(K - 16, 16)][15] - base,
                               0, NPT - 1) + 1

                def per_node(nl, _):
                    es = jnp.maximum(sval(nstv, nl), c)
                    ee = jnp.minimum(sval(nstv, nl + 1), c + K)

                    def edge_body(e, accs):
                        el = e - c
                        out = []
                        for j in range(F // 16):
                            sl = pl.ds(j * 16, 16)
                            sig = _sigmoid16(A_[b][el, sl])
                            out.append(accs[j] + sig)
                            out.append(accs[j + 8] + sig * D_[b][el, sl])
                            out.append(accs[j + 16] + sig * E_[b][el, sl])
                        return (tuple(out[0::3]) + tuple(out[1::3])
                                + tuple(out[2::3]))

                    accs = lax.fori_loop(
                        es, ee, edge_body,
                        tuple(jnp.zeros((16,), jnp.float32)
                              for _ in range(3 * (F // 16))))
                    for j in range(F // 16):
                        sl = pl.ds(j * 16, 16)
                        ss[nl, sl] = ss[nl, sl] + accs[j]
                        hacc[nl, sl] = hacc[nl, sl] + accs[j + 8]
                        pacc[nl, sl] = pacc[nl, sl] + accs[j + 16]
                    return 0

                lax.fori_loop(nlo, nhi, per_node, 0)
                pltpu.make_async_copy(A_[b], hat_h.at[pl.ds(c, K)],
                                      wsem).wait()

            @pl.when(nch > 0)
            def _():
                issue_idx(0, 0)
                wait_idx(0)
                issue_data(0, 0)

                def step(ci, _):
                    for b in range(2):
                        def mk(cur_b):
                            def inner():
                                cur = 2 * ci + cur_b
                                prefetch(cur, (cur_b + 1) % 2)
                                wait_data(cur_b)
                                compute(cur, cur_b)
                            return inner

                        pl.when(2 * ci + b < nch)(mk(b))
                    return 0

                lax.fori_loop(0, (nch + 1) // 2, step, 0)

            def fin(nl, _):
                for j in range(F // 16):
                    sl = pl.ds(j * 16, 16)
                    inv = 1.0 / (ss[nl, sl] + ETA_EPS)
                    hacc[nl, sl] = hacc[nl, sl] * inv
                    pacc[nl, sl] = pacc[nl, sl] * inv
                return 0

            lax.fori_loop(0, NPT, fin, 0)

            pltpu.sync_copy(hacc, hagg_h.at[pl.ds(base, NPT)])
            pltpu.sync_copy(pacc, pagg_h.at[pl.ds(base, NPT)])
            return 0

        lax.fori_loop(0, n_mine, per_range, 0)

    return k(bh1, bh2, vv, c2p, b3e, src_p, dst_p, estart_p, nst_p)


def _full(shape):
    return pl.BlockSpec(shape, lambda i: (0, 0))


def _enc_node(nf, pos, wn, bn_, gn, bbn, wp, bp):
    n = nf.shape[0]
    grid = n // NBLK

    def body(nf_r, pos_r, wn_r, bn_r, gn_r, bbn_r, wp_r, bp_r, h_r, pe_r):
        x = jnp.dot(nf_r[...], wn_r[...], preferred_element_type=jnp.float32) + bn_r[...]
        m = jnp.mean(x, axis=-1, keepdims=True)
        v = jnp.mean((x - m) ** 2, axis=-1, keepdims=True)
        h_r[...] = (x - m) / jnp.sqrt(v + BN_EPS) * gn_r[...] + bbn_r[...]
        pe_r[...] = jnp.dot(pos_r[...], wp_r[...], preferred_element_type=jnp.float32) + bp_r[...]

    return pl.pallas_call(
        body,
        grid=(grid,),
        in_specs=[
            pl.BlockSpec((NBLK, F), lambda i: (i, 0)),
            pl.BlockSpec((NBLK, 16), lambda i: (i, 0)),
            _full((F, F)), _full((1, F)), _full((1, F)), _full((1, F)),
            _full((16, F)), _full((1, F)),
        ],
        out_specs=[
            pl.BlockSpec((NBLK, F), lambda i: (i, 0)),
            pl.BlockSpec((NBLK, F), lambda i: (i, 0)),
        ],
        out_shape=[
            jax.ShapeDtypeStruct((n, F), jnp.float32),
            jax.ShapeDtypeStruct((n, F), jnp.float32),
        ],
    )(nf, pos, wn, bn_, gn, bbn, wp, bp)


def _enc_edge(ef, we, be, ge, bbe, w3, b3):
    Ep = ef.shape[0]
    grid = Ep // EBLK

    def body(ef_r, we_r, be_r, ge_r, bbe_r, w3_r, b3_r, e_r, b3e_r):
        x = jnp.dot(ef_r[...], we_r[...], preferred_element_type=jnp.float32) + be_r[...]
        m = jnp.mean(x, axis=-1, keepdims=True)
        v = jnp.mean((x - m) ** 2, axis=-1, keepdims=True)
        e0 = (x - m) / jnp.sqrt(v + BN_EPS) * ge_r[...] + bbe_r[...]
        e_r[...] = e0
        b3e_r[...] = jnp.dot(e0, w3_r[...],
                             preferred_element_type=jnp.float32) + b3_r[...]

    blk = pl.BlockSpec((EBLK, F), lambda i: (i, 0))
    return pl.pallas_call(
        body,
        grid=(grid,),
        in_specs=[
            pl.BlockSpec((EBLK, 16), lambda i: (i, 0)),
            _full((16, F)), _full((1, F)), _full((1, F)), _full((1, F)),
            _full((F, F)), _full((1, F)),
        ],
        out_specs=[blk, blk],
        out_shape=[jax.ShapeDtypeStruct((Ep, F), jnp.float32),
                   jax.ShapeDtypeStruct((Ep, F), jnp.float32)],
    )(ef, we, be, ge, bbe, w3, b3)


def _node_matmuls(h, pe, lp):
    n = h.shape[0]
    grid = n // NBLK
    ws = [lp['B1']['W'], lp['B2']['W'],
          lp['A1']['W'][:F], lp['A1']['W'][F:],
          lp['A2']['W'][:F], lp['A2']['W'][F:],
          lp['C1']['W'], lp['C2']['W']]
    bs = [lp['B1']['b'].reshape(1, F), lp['B2']['b'].reshape(1, F),
          lp['A1']['b'].reshape(1, F), lp['A2']['b'].reshape(1, F),
          lp['C1']['b'].reshape(1, F), lp['C2']['b'].reshape(1, F)]

    def body(h_r, pe_r, b1w, b2w, a1h, a1p, a2h, a2p, c1w, c2w,
             b1b, b2b, a1b, a2b, c1b, c2b,
             bh1_r, bh2_r, a1_r, vv_r, c1_r, c2p_r):
        hb = h_r[...]
        pb = pe_r[...]
        dot = lambda a, b: jnp.dot(a, b, preferred_element_type=jnp.float32)
        bh1_r[...] = dot(hb, b1w[...]) + b1b[...]
        bh2_r[...] = dot(hb, b2w[...]) + b2b[...]
        a1_r[...] = dot(hb, a1h[...]) + dot(pb, a1p[...]) + a1b[...]
        vv_r[...] = dot(hb, a2h[...]) + dot(pb, a2p[...]) + a2b[...]
        c1_r[...] = dot(pb, c1w[...]) + c1b[...]
        c2p_r[...] = dot(pb, c2w[...]) + c2b[...]

    blk = pl.BlockSpec((NBLK, F), lambda i: (i, 0))
    return pl.pallas_call(
        body,
        grid=(grid,),
        in_specs=[blk, blk] + [_full((F, F))] * 8 + [_full((1, F))] * 6,
        out_specs=[blk] * 6,
        out_shape=[jax.ShapeDtypeStruct((n, F), jnp.float32)] * 6,
    )(h, pe, *ws, *bs)


def _node_update_mm(a1, hagg, stats, g, b, h_prev, h_i,
                    c1, pagg, p_prev, p_i, n_valid, nlp):
    n = a1.shape[0]
    grid = n // NBLK
    ws = [nlp['B1']['W'], nlp['B2']['W'],
          nlp['A1']['W'][:F], nlp['A1']['W'][F:],
          nlp['A2']['W'][:F], nlp['A2']['W'][F:],
          nlp['C1']['W'], nlp['C2']['W']]
    bs = [nlp['B1']['b'].reshape(1, F), nlp['B2']['b'].reshape(1, F),
          nlp['A1']['b'].reshape(1, F), nlp['A2']['b'].reshape(1, F),
          nlp['C1']['b'].reshape(1, F), nlp['C2']['b'].reshape(1, F)]

    def body(a_r, ha_r, st_r, g_r, b_r, hp_r, hi_r, c_r, pa_r, pp_r, pi_r,
             b1w, b2w, a1h, a1p, a2h, a2p, c1w, c2w,
             b1b, b2b, a1b, a2b, c1b, c2b,
             hn_r, pn_r, bh1_r, bh2_r, na1_r, vv_r, nc1_r, c2p_r):
        s = st_r[0, :]
        sq = st_r[1, :]
        m = s / n_valid
        var = sq / n_valid - m * m
        scale = g_r[0, :] / jnp.sqrt(var + BN_EPS)
        shift = b_r[0, :] - m * scale
        x = a_r[...] + ha_r[...]
        hb = hp_r[...] + jnp.maximum(x * scale + shift, 0.0) + hi_r[...]
        pb = pp_r[...] + jnp.tanh(c_r[...] + pa_r[...]) + pi_r[...]
        hn_r[...] = hb
        pn_r[...] = pb
        dot = lambda u, v_: jnp.dot(u, v_, preferred_element_type=jnp.float32)
        bh1_r[...] = dot(hb, b1w[...]) + b1b[...]
        bh2_r[...] = dot(hb, b2w[...]) + b2b[...]
        na1_r[...] = dot(hb, a1h[...]) + dot(pb, a1p[...]) + a1b[...]
        vv_r[...] = dot(hb, a2h[...]) + dot(pb, a2p[...]) + a2b[...]
        nc1_r[...] = dot(pb, c1w[...]) + c1b[...]
        c2p_r[...] = dot(pb, c2w[...]) + c2b[...]

    blk = pl.BlockSpec((NBLK, F), lambda i: (i, 0))
    return pl.pallas_call(
        body,
        grid=(grid,),
        in_specs=([blk, blk, _full((2, F)), _full((1, F)), _full((1, F)),
                   blk, blk, blk, blk, blk, blk]
                  + [_full((F, F))] * 8 + [_full((1, F))] * 6),
        out_specs=[blk] * 8,
        out_shape=[jax.ShapeDtypeStruct((n, F), jnp.float32)] * 8,
    )(a1, hagg, stats, g, b, h_prev, h_i, c1, pagg, p_prev, p_i, *ws, *bs)


def _edge_b3(e, w3, b3):
    Ep = e.shape[0]
    grid = Ep // EBLK

    def body(e_r, w_r, b_r, o_r):
        o_r[...] = jnp.dot(e_r[...], w_r[...], preferred_element_type=jnp.float32) + b_r[...]

    blk = pl.BlockSpec((EBLK, F), lambda i: (i, 0))
    return pl.pallas_call(
        body,
        grid=(grid,),
        in_specs=[blk, _full((F, F)), _full((1, F))],
        out_specs=blk,
        out_shape=jax.ShapeDtypeStruct((Ep, F), jnp.float32),
    )(e, w3, b3)


def _edge_stats(hat, n_valid):
    Ep = hat.shape[0]
    grid = Ep // EBLK

    def body(hat_r, o_r):
        i = pl.program_id(0)
        rows = i * EBLK + lax.broadcasted_iota(jnp.int32, (EBLK, F), 0)
        x = jnp.where(rows < n_valid, hat_r[...], 0.0)
        s = jnp.sum(x, axis=0, keepdims=True)
        sq = jnp.sum(x * x, axis=0, keepdims=True)
        part = jnp.concatenate([s, sq], axis=0)

        @pl.when(i == 0)
        def _():
            o_r[...] = jnp.zeros_like(o_r)

        o_r[...] += part

    return pl.pallas_call(
        body,
        grid=(grid,),
        in_specs=[pl.BlockSpec((EBLK, F), lambda i: (i, 0))],
        out_specs=pl.BlockSpec((2, F), lambda i: (0, 0)),
        out_shape=jax.ShapeDtypeStruct((2, F), jnp.float32),
    )(hat)


def _edge_update(hat, e_prev, e_i, stats, g, b, w3, b3, n_valid):
    Ep = hat.shape[0]
    grid = Ep // EBLK

    def body(hat_r, ep_r, ei_r, st_r, g_r, b_r, w_r, b3_r, en_r, o_r):
        s = st_r[0, :]
        sq = st_r[1, :]
        m = s / n_valid
        var = sq / n_valid - m * m
        scale = g_r[0, :] / jnp.sqrt(var + BN_EPS)
        shift = b_r[0, :] - m * scale
        en = ep_r[...] + jnp.maximum(hat_r[...] * scale + shift, 0.0) + ei_r[...]
        en_r[...] = en
        o_r[...] = jnp.dot(en, w_r[...], preferred_element_type=jnp.float32) + b3_r[...]

    blk = pl.BlockSpec((EBLK, F), lambda i: (i, 0))
    return pl.pallas_call(
        body,
        grid=(grid,),
        in_specs=[blk, blk, blk, _full((2, F)), _full((1, F)), _full((1, F)),
                  _full((F, F)), _full((1, F))],
        out_specs=[blk, blk],
        out_shape=[jax.ShapeDtypeStruct((Ep, F), jnp.float32),
                   jax.ShapeDtypeStruct((Ep, F), jnp.float32)],
    )(hat, e_prev, e_i, stats, g, b, w3, b3)


def _node_stats(a1, hagg):
    n = a1.shape[0]
    grid = n // NBLK

    def body(a_r, h_r, o_r):
        i = pl.program_id(0)
        x = a_r[...] + h_r[...]
        s = jnp.sum(x, axis=0, keepdims=True)
        sq = jnp.sum(x * x, axis=0, keepdims=True)
        part = jnp.concatenate([s, sq], axis=0)

        @pl.when(i == 0)
        def _():
            o_r[...] = jnp.zeros_like(o_r)

        o_r[...] += part

    blk = pl.BlockSpec((NBLK, F), lambda i: (i, 0))
    return pl.pallas_call(
        body,
        grid=(grid,),
        in_specs=[blk, blk],
        out_specs=pl.BlockSpec((2, F), lambda i: (0, 0)),
        out_shape=jax.ShapeDtypeStruct((2, F), jnp.float32),
    )(a1, hagg)


def _node_update(a1, hagg, stats, g, b, h_prev, h_i,
                 c1, pagg, p_prev, p_i, n_valid):
    n = a1.shape[0]
    grid = n // NBLK

    def body(a_r, ha_r, st_r, g_r, b_r, hp_r, hi_r, c_r, pa_r, pp_r, pi_r,
             hn_r, pn_r):
        s = st_r[0, :]
        sq = st_r[1, :]
        m = s / n_valid
        var = sq / n_valid - m * m
        scale = g_r[0, :] / jnp.sqrt(var + BN_EPS)
        shift = b_r[0, :] - m * scale
        x = a_r[...] + ha_r[...]
        hn_r[...] = hp_r[...] + jnp.maximum(x * scale + shift, 0.0) + hi_r[...]
        pn_r[...] = pp_r[...] + jnp.tanh(c_r[...] + pa_r[...]) + pi_r[...]

    blk = pl.BlockSpec((NBLK, F), lambda i: (i, 0))
    return pl.pallas_call(
        body,
        grid=(grid,),
        in_specs=[blk, blk, _full((2, F)), _full((1, F)), _full((1, F)),
                  blk, blk, blk, blk, blk, blk],
        out_specs=[blk, blk],
        out_shape=[jax.ShapeDtypeStruct((n, F), jnp.float32),
                   jax.ShapeDtypeStruct((n, F), jnp.float32)],
    )(a1, hagg, stats, g, b, h_prev, h_i, c1, pagg, p_prev, p_i)


def _pool_mlp(h, gid_row, params, n_graphs):
    n = h.shape[0]

    def body(h_r, gid_r, w1, b1, g1, bb1, w2, b2, g2, bb2, w3, b3, o_r):
        gids = gid_r[...]
        onehot = (lax.broadcasted_iota(jnp.int32, (n_graphs, n), 0)
                  == gids).astype(jnp.float32)
        hg = jnp.dot(onehot, h_r[...], preferred_element_type=jnp.float32)

        def bn_elu(x, gg, bb):
            m = jnp.mean(x, axis=0, keepdims=True)
            var = jnp.mean((x - m) ** 2, axis=0, keepdims=True)
            y = (x - m) / jnp.sqrt(var + BN_EPS) * gg + bb
            return jnp.where(y > 0, y, jnp.exp(jnp.minimum(y, 0.0)) - 1.0)

        x = bn_elu(jnp.dot(hg, w1[...], preferred_element_type=jnp.float32) + b1[...],
                   g1[...], bb1[...])
        x = bn_elu(jnp.dot(x, w2[...], preferred_element_type=jnp.float32) + b2[...],
                   g2[...], bb2[...])
        o_r[...] = jnp.dot(x, w3[...], preferred_element_type=jnp.float32) + b3[...]

    mp = params
    return pl.pallas_call(
        body,
        in_specs=[
            pl.BlockSpec((n, F), lambda: (0, 0)),
            pl.BlockSpec((1, n), lambda: (0, 0)),
            _full2((F, F)), _full2((1, F)), _full2((1, F)), _full2((1, F)),
            _full2((F, 32)), _full2((1, 32)), _full2((1, 32)), _full2((1, 32)),
            _full2((32, 1)), _full2((1, 1)),
        ],
        out_specs=pl.BlockSpec((n_graphs, 1), lambda: (0, 0)),
        out_shape=jax.ShapeDtypeStruct((n_graphs, 1), jnp.float32),
    )(h, gid_row,
      mp['mlp_l1']['W'], mp['mlp_l1']['b'].reshape(1, F),
      mp['mlp_bn1']['g'].reshape(1, F), mp['mlp_bn1']['b'].reshape(1, F),
      mp['mlp_l2']['W'], mp['mlp_l2']['b'].reshape(1, 32),
      mp['mlp_bn2']['g'].reshape(1, 32), mp['mlp_bn2']['b'].reshape(1, 32),
      mp['mlp_l3']['W'], mp['mlp_l3']['b'].reshape(1, 1))


def _full2(shape):
    return pl.BlockSpec(shape, lambda: (0, 0))


def kernel(node_feats, edge_feats, pos_enc, fp, edge_index, graph_ids, params):
    n = node_feats.shape[0]
    e_cnt = edge_index.shape[1]
    n_graphs = fp.shape[0]

    src, dst = edge_index[0], edge_index[1]
    perm = jnp.argsort(dst)
    dst_s = dst[perm]
    src_s = src[perm]
    ef_s = edge_feats[perm]

    n_ranges = -(-n // NPT)
    Ep = e_cnt + K
    Ep = -(-Ep // EBLK) * EBLK
    esz = (-(-(n_ranges + 17) // 16)) * 16

    npad = n_ranges * NPT
    src_p = jnp.zeros((Ep,), jnp.int32).at[:e_cnt].set(src_s)
    dst_p = jnp.full((Ep,), n - 1, jnp.int32).at[:e_cnt].set(dst_s)
    counts = jnp.zeros((npad,), jnp.int32).at[dst].add(1, mode='drop')
    nst = jnp.concatenate([jnp.zeros((1,), jnp.int32),
                           jnp.cumsum(counts, dtype=jnp.int32)])
    estart = nst[jnp.arange(n_ranges + 1, dtype=jnp.int32) * NPT]
    estart_p = jnp.full((esz,), e_cnt, jnp.int32).at[:n_ranges + 1].set(estart)
    nst_p = jnp.full((npad + 32,), e_cnt, jnp.int32).at[:npad + 1].set(nst)
    ef_p = jnp.zeros((Ep, 16), jnp.float32).at[:e_cnt].set(ef_s)

    p = params
    h, pe = _enc_node(
        node_feats, pos_enc,
        p['enc_node']['W'], p['enc_node']['b'].reshape(1, F),
        p['ln_node']['g'].reshape(1, F), p['ln_node']['b'].reshape(1, F),
        p['enc_pose']['W'], p['enc_pose']['b'].reshape(1, F))
    e, b3e = _enc_edge(
        ef_p,
        p['enc_edge']['W'], p['enc_edge']['b'].reshape(1, F),
        p['ln_edge']['g'].reshape(1, F), p['ln_edge']['b'].reshape(1, F),
        p['layers'][0]['B3']['W'], p['layers'][0]['B3']['b'].reshape(1, F))

    h_i, e_i, p_i = h, e, pe
    n_layers = len(p['layers'])
    e_prev = e

    mm = _node_matmuls(h, pe, p['layers'][0])
    for li, lp in enumerate(p['layers']):
        bh1, bh2, a1, vv, c1, c2p = mm
        hat, hagg, pagg = _sc_layer(bh1, bh2, vv, c2p, b3e,
                                    src_p, dst_p, estart_p, nst_p, n_ranges)
        if li + 1 < n_layers:
            stats_e = _edge_stats(hat, e_cnt)
            nlp = p['layers'][li + 1]
            e_prev, b3e = _edge_update(
                hat, e_prev, e_i, stats_e,
                lp['bn_e']['g'].reshape(1, F), lp['bn_e']['b'].reshape(1, F),
                nlp['B3']['W'], nlp['B3']['b'].reshape(1, F), e_cnt)
        stats_h = _node_stats(a1, hagg)
        if li + 1 < n_layers:
            h, pe, *mm = _node_update_mm(
                a1, hagg, stats_h,
                lp['bn_h']['g'].reshape(1, F), lp['bn_h']['b'].reshape(1, F),
                h, h_i, c1, pagg, pe, p_i, n, p['layers'][li + 1])
        else:
            h, pe = _node_update(
                a1, hagg, stats_h,
                lp['bn_h']['g'].reshape(1, F), lp['bn_h']['b'].reshape(1, F),
                h, h_i, c1, pagg, pe, p_i, n)

    gid_row = graph_ids.reshape(1, n).astype(jnp.int32)
    return _pool_mlp(h, gid_row, p, n_graphs)
